# baseline (speedup 1.0000x reference)
import functools

import jax
import jax.numpy as jnp
import numpy as np
from jax import lax
from jax.experimental import pallas as pl
from jax.experimental.pallas import tpu as pltpu
from jax.experimental.pallas import tpu_sc as plsc

N = 10000
E = 160000
H = 64
K = 3
EP = 163840
NPAD = 10112
NC = 2
NS = 16
NW = NC * NS
C = 128

_f32 = jnp.float32


def _tcmap(name, nrows, block, ins, consts, out_dims, body):
    grid = nrows // block
    in_specs = []
    for a in ins:
        if a.ndim == 3:
            in_specs.append(pl.BlockSpec((a.shape[0], block, a.shape[2]),
                                         lambda i: (0, i, 0)))
        else:
            in_specs.append(pl.BlockSpec((block, a.shape[1]), lambda i: (i, 0)))
    for c in consts:
        in_specs.append(pl.BlockSpec(c.shape, lambda i: (0,) * c.ndim))
    out_specs = [pl.BlockSpec((block, d), lambda i: (i, 0)) for d in out_dims]
    nin, ncon = len(ins), len(consts)

    def kern(*refs):
        ib = [refs[i][...] for i in range(nin)]
        cb = [refs[nin + i][...] for i in range(ncon)]
        outs = body(ib, cb)
        for k, ob in enumerate(outs):
            refs[nin + ncon + k][...] = ob

    return pl.pallas_call(
        kern,
        grid=(grid,),
        in_specs=in_specs,
        out_specs=out_specs,
        out_shape=[jax.ShapeDtypeStruct((nrows, d), _f32) for d in out_dims],
    )(*ins, *consts)


def _silu(x):
    return x * jax.nn.sigmoid(x)


def _lnorm(x, g, b):
    mu = jnp.mean(x, axis=-1, keepdims=True)
    var = jnp.mean((x - mu) ** 2, axis=-1, keepdims=True)
    return g * (x - mu) / jnp.sqrt(var + 1e-5) + b


def _mlpblk(x, w, b, g, bb):
    return _silu(_lnorm(jnp.dot(x, w, preferred_element_type=_f32) + b, g, bb))


def _linblk(x, w, b):
    y = jnp.dot(x, w, preferred_element_type=_f32)
    return y if b is None else y + b


_MESH = plsc.VectorSubcoreMesh(core_axis_name="c", subcore_axis_name="s")


def _wid_base(rows_pw):
    c = lax.axis_index("c")
    s = lax.axis_index("s")
    return (s * NC + c) * rows_pw


def _sc_gather_pairs(table, idx_a, idx_b, nrows, ow):
    rows_pw = nrows // NW
    nchunk = rows_pw // C
    np16 = ow // 16

    @functools.partial(
        pl.kernel,
        out_type=[jax.ShapeDtypeStruct((nrows, ow), _f32),
                  jax.ShapeDtypeStruct((nrows, ow), _f32)],
        mesh=_MESH,
        scratch_types=[pltpu.VMEM((C,), jnp.int32),
                       pltpu.VMEM((C,), jnp.int32),
                       pltpu.VMEM((C, 128), _f32),
                       pltpu.VMEM((C, 128), _f32),
                       pltpu.VMEM((C, ow), _f32),
                       pltpu.VMEM((C, ow), _f32),
                       pltpu.SemaphoreType.DMA,
                       pltpu.SemaphoreType.DMA],
    )
    def k(tab, ia, ib, oa, ob, iva, ivb, ga, gb, pa, pb, sema, semb):
        base0 = _wid_base(rows_pw)

        def step(j, _):
            base = base0 + j * C
            pltpu.sync_copy(ia.at[pl.ds(base, C)], iva)
            pltpu.sync_copy(ib.at[pl.ds(base, C)], ivb)
            cpa = pltpu.async_copy(tab.at[iva], ga, sema)
            cpb = pltpu.async_copy(tab.at[ivb], gb, semb)
            cpa.wait()
            cpb.wait()

            def row(i, _):
                for kk in range(np16):
                    sl = pl.ds(kk * 16, 16)
                    pa[i, sl] = ga[i, sl]
                    pb[i, sl] = gb[i, sl]
                return _
            lax.fori_loop(0, C, row, None)
            pltpu.sync_copy(pa, oa.at[pl.ds(base, C)])
            pltpu.sync_copy(pb, ob.at[pl.ds(base, C)])
            return _

        lax.fori_loop(0, nchunk, step, None)

    return k(table, idx_a, idx_b)


def _sc_gather_one(table, idx, nrows, ow):
    rows_pw = nrows // NW
    nchunk = rows_pw // C
    np16 = ow // 16

    @functools.partial(
        pl.kernel,
        out_type=[jax.ShapeDtypeStruct((nrows, ow), _f32)],
        mesh=_MESH,
        scratch_types=[pltpu.VMEM((C,), jnp.int32),
                       pltpu.VMEM((C, 128), _f32),
                       pltpu.VMEM((C, ow), _f32),
                       pltpu.SemaphoreType.DMA],
    )
    def k(tab, ia, oa, iva, ga, pa, sema):
        base0 = _wid_base(rows_pw)

        def step(j, _):
            base = base0 + j * C
            pltpu.sync_copy(ia.at[pl.ds(base, C)], iva)
            pltpu.async_copy(tab.at[iva], ga, sema).wait()

            def row(i, _):
                for kk in range(np16):
                    sl = pl.ds(kk * 16, 16)
                    pa[i, sl] = ga[i, sl]
                return _
            lax.fori_loop(0, C, row, None)
            pltpu.sync_copy(pa, oa.at[pl.ds(base, C)])
            return _

        lax.fori_loop(0, nchunk, step, None)

    res = k(table, idx)
    return res[0] if isinstance(res, (list, tuple)) else res


def _sc_edge_egg_dense(ts, td64, g, lgs, write_m, tp3):
    CB = 64
    CT = 3 * CB
    rows_pw = tp3 // NW
    bonds_pw = rows_pw // 3
    nchunk = rows_pw // CT
    outs = [jax.ShapeDtypeStruct((EP, 128), _f32)]
    if write_m:
        outs.append(jax.ShapeDtypeStruct((tp3, 64), _f32))

    @functools.partial(
        pl.kernel,
        out_type=outs,
        mesh=_MESH,
        scratch_types=[pltpu.VMEM((CT,), jnp.int32),
                       pltpu.VMEM((CT, 128), _f32),
                       pltpu.VMEM((CB, 64), _f32),
                       pltpu.VMEM((CT, 64), _f32),
                       pltpu.VMEM((CB, 128), _f32),
                       pltpu.SemaphoreType.DMA,
                       pltpu.SemaphoreType.DMA],
    )
    def k(tsr, tdr, gr, sr, so, *rest):
        if write_m:
            mo = rest[0]
            ivs, gs, tdv, gv, qs, sema, semb = rest[1:]
        else:
            mo = None
            ivs, gs, tdv, gv, qs, sema, semb = rest
        cid = lax.axis_index("c")
        sid = lax.axis_index("s")
        w = sid * NC + cid
        tbase0 = w * rows_pw
        bbase0 = w * bonds_pw

        def step(j, _):
            tbase = tbase0 + j * CT
            bbase = bbase0 + j * CB
            pltpu.sync_copy(sr.at[pl.ds(tbase, CT)], ivs)
            cpa = pltpu.async_copy(tsr.at[ivs.at[pl.ds(0, C)]],
                                   gs.at[pl.ds(0, C)], sema)
            cpb = pltpu.async_copy(tsr.at[ivs.at[pl.ds(C, CT - C)]],
                                   gs.at[pl.ds(C, CT - C)], semb)
            pltpu.sync_copy(tdr.at[pl.ds(bbase, CB)], tdv)
            pltpu.sync_copy(gr.at[pl.ds(tbase, CT)], gv)
            cpa.wait()
            cpb.wait()

            def bond(b, _):
                for kk in range(4):
                    sl = pl.ds(kk * 16, 16)
                    sl2 = pl.ds(64 + kk * 16, 16)
                    d = tdv[b, sl]
                    ssum = jnp.zeros((16,), _f32)
                    shsum = jnp.zeros((16,), _f32)
                    for q in range(3):
                        i = b * 3 + q
                        m = gs[i, sl] + d + gv[i, sl]
                        if write_m:
                            gv[i, sl] = m
                        sig = 1.0 / (1.0 + jnp.exp(-m))
                        ssum = ssum + sig
                        shsum = shsum + sig * gs[i, sl2]
                    qs[b, sl] = ssum
                    qs[b, sl2] = shsum
                return _
            lax.fori_loop(0, CB, bond, None)
            if write_m:
                pltpu.sync_copy(gv, mo.at[pl.ds(tbase, CT)])
            pltpu.sync_copy(qs, so.at[pl.ds(bbase, CB)])
            return _

        lax.fori_loop(0, nchunk, step, None)

    res = k(ts, td64, g, lgs)
    if write_m:
        return res[0], res[1]
    return res[0] if isinstance(res, (list, tuple)) else res


def _sc_node_egg(ts, td, g, src, dst):
    rows_pw = EP // NW
    nchunk = rows_pw // C
    zrows = NPAD // NS

    @functools.partial(
        pl.kernel,
        out_type=[jax.ShapeDtypeStruct((EP, 64), _f32),
                  jax.ShapeDtypeStruct((NC, NPAD, 128), _f32)],
        mesh=_MESH,
        scratch_types=[pltpu.VMEM((C,), jnp.int32),
                       pltpu.VMEM((C,), jnp.int32),
                       pltpu.VMEM((C, 128), _f32),
                       pltpu.VMEM((C, 128), _f32),
                       pltpu.VMEM((C, 64), _f32),
                       pltpu.VMEM_SHARED((NPAD, 128), _f32),
                       pltpu.SemaphoreType.DMA,
                       pltpu.SemaphoreType.DMA],
    )
    def k(tsr, tdr, gr, sr, dr, mo, so, ivs, ivd, gs, gd, gv, acc, sema, semb):
        cid = lax.axis_index("c")
        sid = lax.axis_index("s")
        base0 = (sid * NC + cid) * rows_pw

        def zrow(i, _):
            for kk in range(8):
                gs[i, pl.ds(kk * 16, 16)] = jnp.zeros((16,), _f32)
            return _
        lax.fori_loop(0, C, zrow, None)
        for t in range(4):
            pltpu.sync_copy(gs, acc.at[pl.ds(sid * zrows + t * C, C)])
        pltpu.sync_copy(gs.at[pl.ds(0, zrows - 4 * C)],
                        acc.at[pl.ds(sid * zrows + 4 * C, zrows - 4 * C)])
        plsc.subcore_barrier()

        def step(j, _):
            base = base0 + j * C
            pltpu.sync_copy(sr.at[pl.ds(base, C)], ivs)
            pltpu.sync_copy(dr.at[pl.ds(base, C)], ivd)
            cpa = pltpu.async_copy(tsr.at[ivs], gs, sema)
            cpb = pltpu.async_copy(tdr.at[ivd], gd, semb)
            pltpu.sync_copy(gr.at[pl.ds(base, C)], gv)
            cpa.wait()
            cpb.wait()

            def row(i, _):
                for kk in range(4):
                    a = gs[i, pl.ds(kk * 16, 16)]
                    d = gd[i, pl.ds(kk * 16, 16)]
                    gg = gv[i, pl.ds(kk * 16, 16)]
                    m = a + d + gg
                    gv[i, pl.ds(kk * 16, 16)] = m
                    sig = 1.0 / (1.0 + jnp.exp(-m))
                    gd[i, pl.ds(kk * 16, 16)] = sig
                    bh = gs[i, pl.ds(64 + kk * 16, 16)]
                    gd[i, pl.ds(64 + kk * 16, 16)] = sig * bh
                return _
            lax.fori_loop(0, C, row, None)
            pltpu.sync_copy(gv, mo.at[pl.ds(base, C)])
            pltpu.sync_copy(gd, acc.at[ivd], add=True)
            return _

        lax.fori_loop(0, nchunk, step, None)
        plsc.subcore_barrier()
        pltpu.sync_copy(acc.at[pl.ds(sid * zrows, zrows)],
                        so.at[cid, pl.ds(sid * zrows, zrows)])

    return k(ts, td, g, src, dst)


def _sc_edge_egg(ts, td, g, lgs, lgd, tp, write_m, qrows):
    rows_pw = tp // NW
    nchunk = rows_pw // C
    outs = [jax.ShapeDtypeStruct((qrows, 128), _f32)]
    if write_m:
        outs.append(jax.ShapeDtypeStruct((tp, 64), _f32))

    @functools.partial(
        pl.kernel,
        out_type=outs,
        mesh=_MESH,
        scratch_types=[pltpu.VMEM((C,), jnp.int32),
                       pltpu.VMEM((C,), jnp.int32),
                       pltpu.VMEM((C, 128), _f32),
                       pltpu.VMEM((C, 128), _f32),
                       pltpu.VMEM((C, 64), _f32),
                       pltpu.VMEM((C, 64), _f32),
                       pltpu.VMEM((C, 128), _f32),
                       pltpu.SemaphoreType.DMA,
                       pltpu.SemaphoreType.DMA],
    )
    def k(tsr, tdr, gr, sr, dr, qo, *rest):
        if write_m:
            mo = rest[0]
            ivs, ivd, gs, gd, gv, mv, qv, sema, semb = rest[1:]
        else:
            mo = None
            ivs, ivd, gs, gd, gv, mv, qv, sema, semb = rest
        cid = lax.axis_index("c")
        sid = lax.axis_index("s")
        base0 = (sid * NC + cid) * rows_pw

        @pl.when(jnp.logical_and(cid == 0, sid == 0))
        def _():
            def zrow(i, _):
                for kk in range(8):
                    qv[i, pl.ds(kk * 16, 16)] = jnp.zeros((16,), _f32)
                return _
            lax.fori_loop(0, C, zrow, None)
            pltpu.sync_copy(qv, qo.at[pl.ds(tp, C)])

        def step(j, _):
            base = base0 + j * C
            pltpu.sync_copy(sr.at[pl.ds(base, C)], ivs)
            pltpu.sync_copy(dr.at[pl.ds(base, C)], ivd)
            cpa = pltpu.async_copy(tsr.at[ivs], gs, sema)
            cpb = pltpu.async_copy(tdr.at[ivd], gd, semb)
            pltpu.sync_copy(gr.at[pl.ds(base, C)], gv)
            cpa.wait()
            cpb.wait()

            def row(i, _):
                for kk in range(4):
                    a = gs[i, pl.ds(kk * 16, 16)]
                    d = gd[i, pl.ds(kk * 16, 16)]
                    gg = gv[i, pl.ds(kk * 16, 16)]
                    m = a + d + gg
                    if write_m:
                        mv[i, pl.ds(kk * 16, 16)] = m
                    sig = 1.0 / (1.0 + jnp.exp(-m))
                    qv[i, pl.ds(kk * 16, 16)] = sig
                    bh = gs[i, pl.ds(64 + kk * 16, 16)]
                    qv[i, pl.ds(64 + kk * 16, 16)] = sig * bh
                return _
            lax.fori_loop(0, C, row, None)
            if write_m:
                pltpu.sync_copy(mv, mo.at[pl.ds(base, C)])
            pltpu.sync_copy(qv, qo.at[pl.ds(base, C)])
            return _

        lax.fori_loop(0, nchunk, step, None)

    res = k(ts, td, g, lgs, lgd)
    if write_m:
        return res[0], res[1]
    return res[0] if isinstance(res, (list, tuple)) else res


def _sc_gather3(q, i0, i1, i2):
    rows_pw = EP // NW
    nchunk = rows_pw // C

    @functools.partial(
        pl.kernel,
        out_type=[jax.ShapeDtypeStruct((EP, 128), _f32)],
        mesh=_MESH,
        scratch_types=[pltpu.VMEM((C,), jnp.int32),
                       pltpu.VMEM((C,), jnp.int32),
                       pltpu.VMEM((C,), jnp.int32),
                       pltpu.VMEM((C, 128), _f32),
                       pltpu.VMEM((C, 128), _f32),
                       pltpu.VMEM((C, 128), _f32),
                       pltpu.VMEM((C, 128), _f32),
                       pltpu.SemaphoreType.DMA,
                       pltpu.SemaphoreType.DMA,
                       pltpu.SemaphoreType.DMA],
    )
    def k(qr, r0, r1, r2, so, v0, v1, v2, g0, g1, g2, ov, s0, s1, s2):
        base0 = _wid_base(rows_pw)

        def step(j, _):
            base = base0 + j * C
            pltpu.sync_copy(r0.at[pl.ds(base, C)], v0)
            pltpu.sync_copy(r1.at[pl.ds(base, C)], v1)
            pltpu.sync_copy(r2.at[pl.ds(base, C)], v2)
            c0 = pltpu.async_copy(qr.at[v0], g0, s0)
            c1 = pltpu.async_copy(qr.at[v1], g1, s1)
            c2 = pltpu.async_copy(qr.at[v2], g2, s2)
            c0.wait()
            c1.wait()
            c2.wait()

            def row(i, _):
                for kk in range(8):
                    sl = pl.ds(kk * 16, 16)
                    ov[i, sl] = g0[i, sl] + g1[i, sl] + g2[i, sl]
                return _
            lax.fori_loop(0, C, row, None)
            pltpu.sync_copy(ov, so.at[pl.ds(base, C)])
            return _

        lax.fori_loop(0, nchunk, step, None)

    res = k(q, i0, i1, i2)
    return res[0] if isinstance(res, (list, tuple)) else res


def _sc_gather_add80(ta, tdx, g, src, dst):
    rows_pw = EP // NW
    nchunk = rows_pw // C

    @functools.partial(
        pl.kernel,
        out_type=[jax.ShapeDtypeStruct((EP, 80), _f32)],
        mesh=_MESH,
        scratch_types=[pltpu.VMEM((C,), jnp.int32),
                       pltpu.VMEM((C,), jnp.int32),
                       pltpu.VMEM((C, 128), _f32),
                       pltpu.VMEM((C, 128), _f32),
                       pltpu.VMEM((C, 80), _f32),
                       pltpu.VMEM((C, 80), _f32),
                       pltpu.SemaphoreType.DMA,
                       pltpu.SemaphoreType.DMA],
    )
    def k(tar, tdr, gr, sr, dr, mo, ivs, ivd, ga, gd, gv, mv, sema, semb):
        base0 = _wid_base(rows_pw)

        def step(j, _):
            base = base0 + j * C
            pltpu.sync_copy(sr.at[pl.ds(base, C)], ivs)
            pltpu.sync_copy(dr.at[pl.ds(base, C)], ivd)
            cpa = pltpu.async_copy(tar.at[ivs], ga, sema)
            cpb = pltpu.async_copy(tdr.at[ivd], gd, semb)
            pltpu.sync_copy(gr.at[pl.ds(base, C)], gv)
            cpa.wait()
            cpb.wait()

            def row(i, _):
                for kk in range(5):
                    sl = pl.ds(kk * 16, 16)
                    mv[i, sl] = ga[i, sl] + gd[i, sl] + gv[i, sl]
                return _
            lax.fori_loop(0, C, row, None)
            pltpu.sync_copy(mv, mo.at[pl.ds(base, C)])
            return _

        lax.fori_loop(0, nchunk, step, None)

    res = k(ta, tdx, g, src, dst)
    return res[0] if isinstance(res, (list, tuple)) else res


def _w(p):
    return p["w"]


def _b2(p):
    return p["b"].reshape(1, -1)


def _g2(p):
    return p["g"].reshape(1, -1)


def _bb2(p):
    return p["b"].reshape(1, -1)


def kernel(r, params, atom_numbers, edge_index, lg_src, lg_dst):
    T = lg_dst.shape[0]
    dense3 = (T == 3 * E)
    TPad = 3 * EP if dense3 else ((T + 4095) // 4096) * 4096

    src = edge_index[0].astype(jnp.int32)
    dst = edge_index[1].astype(jnp.int32)
    lgs = lg_src.astype(jnp.int32)
    lgd = lg_dst.astype(jnp.int32)

    src_p = jnp.concatenate([src, jnp.full((EP - E,), N, jnp.int32)])
    dst_p = jnp.concatenate([dst, jnp.full((EP - E,), N, jnp.int32)])
    lgs_p = jnp.concatenate([lgs, jnp.full((TPad - T,), E, jnp.int32)])
    lgd_p = jnp.concatenate([lgd, jnp.full((TPad - T,), E, jnp.int32)])
    if dense3:
        qrows = None
        idx3 = None
    else:
        qrows = TPad + C
        se = jnp.searchsorted(lgd, jnp.arange(E + 1, dtype=jnp.int32)).astype(jnp.int32)
        s_p = jnp.concatenate([se[:E], jnp.zeros((EP - E,), jnp.int32)])
        e_p = jnp.concatenate([se[1:], jnp.zeros((EP - E,), jnp.int32)])
        idx3 = [jnp.where(s_p + i < e_p, s_p + i, TPad).astype(jnp.int32)
                for i in range(K)]

    r16 = jnp.zeros((EP, 16), _f32).at[:E, 0:3].set(r.astype(_f32))
    an2 = jnp.concatenate([atom_numbers.astype(jnp.int32),
                           jnp.zeros((NPAD - N,), jnp.int32)]).reshape(NPAD, 1)

    p = params
    al0n, al0e = p["alignn"][0]["node"], p["alignn"][0]["edge"]
    al1n, al1e = p["alignn"][1]["node"], p["alignn"][1]["edge"]
    gc0, gc1 = p["gcn"][0], p["gcn"][1]

    cent_e = jnp.linspace(0.0, 8.0, 80).astype(_f32).reshape(1, 80)
    cent_a = jnp.linspace(-1.0, 1.0, 40).astype(_f32).reshape(1, 40)
    gam_e = float(79.0 / 8.0)
    gam_a = 19.5

    def t1_body(ib, cb):
        (X,) = ib
        (ce, w1, b1, g1_, bb1, w2, b2_, g2_, bb2, weg, beg) = cb
        bl2 = jnp.sum(X * X, axis=-1, keepdims=True)
        bl = jnp.sqrt(bl2)
        inv = 1.0 / jnp.maximum(bl, 1e-9)
        u = X * inv
        cutv = jnp.where(bl < 3.8, 1.0, 0.5 - 0.5 * jnp.sin(np.pi * (bl - 3.9) / 0.2))
        cutv = jnp.where(bl > 4.0, 0.0, cutv)
        rb = jnp.exp(-gam_e * (bl - ce) ** 2)
        y0 = _mlpblk(rb, w1, b1, g1_, bb1)
        y0 = _mlpblk(y0, w2, b2_, g2_, bb2)
        G1 = _linblk(y0, weg, beg)
        zpad = jnp.zeros((X.shape[0], 123), _f32)
        u_ext = jnp.concatenate([u[:, 0:3], bl, cutv, zpad], axis=1)
        ue16 = u_ext[:, 0:16]
        return y0, G1, u_ext, ue16

    y0, G1, u_ext, ue16 = _tcmap(
        "t1", EP, 2048, [r16],
        [cent_e,
         _w(p["edge_mlp1"]["lin"]), _b2(p["edge_mlp1"]["lin"]),
         _g2(p["edge_mlp1"]["ln"]), _bb2(p["edge_mlp1"]["ln"]),
         _w(p["edge_mlp2"]["lin"]), _b2(p["edge_mlp2"]["lin"]),
         _g2(p["edge_mlp2"]["ln"]), _bb2(p["edge_mlp2"]["ln"]),
         _w(al0n["edge_gate"]), _b2(al0n["edge_gate"])],
        [64, 64, 128, 16], t1_body)

    if dense3:
        uu1 = _sc_gather_one(u_ext, lgs_p, TPad, 16)
        uu2 = jnp.repeat(ue16, 3, axis=0)
    else:
        uu1, uu2 = _sc_gather_pairs(u_ext, lgs_p, lgd_p, TPad, 16)

    def t2_body(ib, cb):
        (U1, U2) = ib
        (ca, w1, b1, g1_, bb1, w2, b2_, g2_, bb2, weg, beg) = cb
        cos = -jnp.sum(U1[:, 0:3] * U2[:, 0:3], axis=-1, keepdims=True)
        cos = jnp.clip(cos, -1.0, 1.0)
        rb = jnp.exp(-gam_a * (cos - ca) ** 2)
        z0 = _mlpblk(rb, w1, b1, g1_, bb1)
        z0 = _mlpblk(z0, w2, b2_, g2_, bb2)
        gp = _linblk(z0, weg, beg)
        return z0, gp

    z0, gp1 = _tcmap(
        "t2", TPad, 4096, [uu1, uu2],
        [cent_a,
         _w(p["angle_mlp1"]["lin"]), _b2(p["angle_mlp1"]["lin"]),
         _g2(p["angle_mlp1"]["ln"]), _bb2(p["angle_mlp1"]["ln"]),
         _w(p["angle_mlp2"]["lin"]), _b2(p["angle_mlp2"]["lin"]),
         _g2(p["angle_mlp2"]["ln"]), _bb2(p["angle_mlp2"]["ln"]),
         _w(al0e["edge_gate"]), _b2(al0e["edge_gate"])],
        [64, 64], t2_body)

    def t3_body(ib, cb):
        (an,) = ib
        (emb, wsg, bsg, wdu, bdu, wdg, bdg, wsu, bsu, wes, bes, wed) = cb
        onehot = (lax.broadcasted_iota(jnp.int32, (an.shape[0], 128), 1)
                  == an).astype(_f32)
        x0 = jnp.dot(onehot, emb, preferred_element_type=_f32)
        ts = jnp.concatenate([_linblk(x0, wsg, bsg), _linblk(x0, wdu, bdu)], axis=1)
        td = jnp.concatenate([_linblk(x0, wdg, bdg),
                              jnp.zeros((x0.shape[0], 64), _f32)], axis=1)
        su = _linblk(x0, wsu, bsu)
        es = _linblk(x0, wes, bes)
        ed = jnp.dot(x0, wed, preferred_element_type=_f32)
        zpad = jnp.zeros_like(es) * 0.0
        esed = jnp.concatenate([es, ed, zpad, zpad], axis=1)
        return x0, ts, td, su, esed

    x0, TS1, TD1, SU1, ES = _tcmap(
        "t3", NPAD, 1264, [an2],
        [p["atom_embedding"],
         _w(al0n["src_gate"]), _b2(al0n["src_gate"]),
         _w(al0n["dst_update"]), _b2(al0n["dst_update"]),
         _w(al0n["dst_gate"]), _b2(al0n["dst_gate"]),
         _w(al0n["src_update"]), _b2(al0n["src_update"]),
         _w(p["int_src"]), _b2(p["int_src"]),
         _w(p["int_dst"])],
        [64, 128, 128, 64, 16], t3_body)

    def node_update_body(ib, cb):
        (x, su, S) = ib
        lng, lnb = cb[0], cb[1]
        s = S[0, :, 0:64] + S[1, :, 0:64]
        sh = S[0, :, 64:128] + S[1, :, 64:128]
        h = sh / (s + 1e-6)
        xn = x + _silu(_lnorm(su + h, lng, lnb))
        outs = [xn]
        ws = cb[2:]
        res = []
        for t in range(0, len(ws), 2):
            res.append(_linblk(xn, ws[t], ws[t + 1]))
        if len(res) == 4:
            outs.append(jnp.concatenate([res[0], res[1]], axis=1))
            outs.append(jnp.concatenate([res[2], jnp.zeros_like(res[2])], axis=1))
            outs.append(res[3])
        else:
            outs.extend(res)
        return outs

    def edge_update_body(ib, cb):
        (m, y) = ib
        lng, lnb = cb[0], cb[1]
        yn = y + _silu(_lnorm(m, lng, lnb))
        outs = [yn]
        ws = cb[2:]
        res = []
        for t in range(0, len(ws), 2):
            res.append(_linblk(yn, ws[t], ws[t + 1]))
        if len(res) == 4:
            outs.append(jnp.concatenate([res[0], res[1]], axis=1))
            outs.append(res[2])
            outs.append(res[3])
        else:
            outs.extend(res)
        return outs

    def eggw(q):
        return [_w(q["src_gate"]), _b2(q["src_gate"]),
                _w(q["dst_update"]), _b2(q["dst_update"]),
                _w(q["dst_gate"]), _b2(q["dst_gate"]),
                _w(q["src_update"]), _b2(q["src_update"])]

    def tri_update_body(ib, cb):
        (ylg, su, Sp) = ib
        lng, lnb = cb[0], cb[1]
        s = Sp[:, 0:64]
        sh = Sp[:, 64:128]
        h = sh / (s + 1e-6)
        yn = ylg + _silu(_lnorm(su + h, lng, lnb))
        outs = [yn]
        ws = cb[2:]
        for t in range(0, len(ws), 2):
            outs.append(_linblk(yn, ws[t], ws[t + 1]))
        return outs

    M1, S1 = _sc_node_egg(TS1, TD1, G1, src_p, dst_p)
    x1, TS2, TD2, SU2 = _tcmap(
        "t4a", NPAD, 1264, [x0, SU1, S1],
        [_g2(al0n["ln_nodes"]), _bb2(al0n["ln_nodes"])] + eggw(al1n),
        [64, 128, 128, 64], node_update_body)
    ylg1, TSp1, TDp1, SUp1 = _tcmap(
        "t5a", EP, 2048, [M1, y0],
        [_g2(al0n["ln_edges"]), _bb2(al0n["ln_edges"])] + eggw(al0e),
        [64, 128, 64, 64], edge_update_body)
    if dense3:
        Sp1, Mp1 = _sc_edge_egg_dense(TSp1, TDp1, gp1, lgs_p, True, TPad)
    else:
        TDp1g = jnp.pad(TDp1, ((0, 0), (0, 64)))
        Q1, Mp1 = _sc_edge_egg(TSp1, TDp1g, gp1, lgs_p, lgd_p, TPad, True, qrows)
        Sp1 = _sc_gather3(Q1, *idx3)
    y1, G2 = _tcmap(
        "t6a", EP, 2048, [ylg1, SUp1, Sp1],
        [_g2(al0e["ln_nodes"]), _bb2(al0e["ln_nodes"]),
         _w(al1n["edge_gate"]), _b2(al1n["edge_gate"])],
        [64, 64], tri_update_body)

    def zup_body(ib, cb):
        (m, z) = ib
        lng, lnb, weg, beg = cb
        zn = z + _silu(_lnorm(m, lng, lnb))
        return (_linblk(zn, weg, beg),)

    (gp2,) = _tcmap(
        "t5z", TPad, 4096, [Mp1, z0],
        [_g2(al0e["ln_edges"]), _bb2(al0e["ln_edges"]),
         _w(al1e["edge_gate"]), _b2(al1e["edge_gate"])],
        [64], zup_body)

    M2, S2 = _sc_node_egg(TS2, TD2, G2, src_p, dst_p)
    x2, TS3, TD3, SU3 = _tcmap(
        "t4b", NPAD, 1264, [x1, SU2, S2],
        [_g2(al1n["ln_nodes"]), _bb2(al1n["ln_nodes"])] + eggw(gc0),
        [64, 128, 128, 64], node_update_body)
    ylg2, TSp2, TDp2, SUp2 = _tcmap(
        "t5b", EP, 2048, [M2, y1],
        [_g2(al1n["ln_edges"]), _bb2(al1n["ln_edges"])] + eggw(al1e),
        [64, 128, 64, 64], edge_update_body)
    if dense3:
        Sp2 = _sc_edge_egg_dense(TSp2, TDp2, gp2, lgs_p, False, TPad)
    else:
        TDp2g = jnp.pad(TDp2, ((0, 0), (0, 64)))
        Q2 = _sc_edge_egg(TSp2, TDp2g, gp2, lgs_p, lgd_p, TPad, False, qrows)
        Sp2 = _sc_gather3(Q2, *idx3)
    y2, G3 = _tcmap(
        "t6b", EP, 2048, [ylg2, SUp2, Sp2],
        [_g2(al1e["ln_nodes"]), _bb2(al1e["ln_nodes"]),
         _w(gc0["edge_gate"]), _b2(gc0["edge_gate"])],
        [64, 64], tri_update_body)

    M3, S3 = _sc_node_egg(TS3, TD3, G3, src_p, dst_p)

    def t4c_body(ib, cb):
        (x, su, S, esed) = ib
        lng, lnb, wsg, bsg, wdg, bdg = cb
        s = S[0, :, 0:64] + S[1, :, 0:64]
        sh = S[0, :, 64:128] + S[1, :, 64:128]
        h = sh / (s + 1e-6)
        xn = x + _silu(_lnorm(su + h, lng, lnb))
        a4 = _linblk(xn, wsg, bsg)
        d4 = _linblk(xn, wdg, bdg)
        zp = jnp.zeros((xn.shape[0], 60), _f32)
        ta = jnp.concatenate([a4, esed[:, 0:4], zp], axis=1)
        tdx = jnp.concatenate([d4, esed[:, 4:8], zp], axis=1)
        return ta, tdx

    TA, TDX = _tcmap(
        "t4c", NPAD, 1264, [x2, SU3, S3, ES],
        [_g2(gc0["ln_nodes"]), _bb2(gc0["ln_nodes"]),
         _w(gc1["src_gate"]), _b2(gc1["src_gate"]),
         _w(gc1["dst_gate"]), _b2(gc1["dst_gate"])],
        [128, 128], t4c_body)

    def t5c_body(ib, cb):
        (m, y) = ib
        lng, lnb, weg, beg = cb
        yn = y + _silu(_lnorm(m, lng, lnb))
        g4 = _linblk(yn, weg, beg)
        g4p = jnp.concatenate([g4, jnp.zeros_like(yn[:, 0:16])], axis=1)
        return yn, g4p

    y3, G4P = _tcmap(
        "t5c", EP, 2048, [M3, y2],
        [_g2(gc0["ln_edges"]), _bb2(gc0["ln_edges"]),
         _w(gc1["edge_gate"]), _b2(gc1["edge_gate"])],
        [64, 80], t5c_body)

    M4X = _sc_gather_add80(TA, TDX, G4P, src_p, dst_p)

    def t7(m4x_ref, y3_ref, ue_ref, lng_ref, lnb_ref, fw_ref, fb_ref, out_ref):
        i = pl.program_id(0)
        m4 = m4x_ref[:, 0:64]
        esd = m4x_ref[:, 64:68]
        y = y3_ref[...]
        yn = y + _silu(_lnorm(m4, lng_ref[...], lnb_ref[...]))
        bond = jax.nn.sigmoid(
            jnp.dot(yn, fw_ref[...], preferred_element_type=_f32)
            + fb_ref[...])
        pe = jnp.exp(esd)
        bl = ue_ref[:, 3:4]
        cutv = ue_ref[:, 4:5]
        f_rep = pe[:, 0:1] * jnp.exp(-pe[:, 1:2] * bl)
        f_att = pe[:, 2:3] * jnp.exp(-pe[:, 3:4] * bl)
        V = cutv * (f_rep - bond * f_att)
        rowid = i * 2048 + lax.broadcasted_iota(jnp.int32, (2048, 1), 0)
        V = jnp.where(rowid < E, V, 0.0)
        bs = jnp.sum(V)

        @pl.when(i == 0)
        def _():
            out_ref[...] = jnp.zeros((1, 1), _f32)
        out_ref[...] += jnp.reshape(bs, (1, 1))

    tot = pl.pallas_call(
        t7,
        grid=(EP // 2048,),
        in_specs=[pl.BlockSpec((2048, 80), lambda i: (i, 0)),
                  pl.BlockSpec((2048, 64), lambda i: (i, 0)),
                  pl.BlockSpec((2048, 16), lambda i: (i, 0)),
                  pl.BlockSpec((1, 64), lambda i: (0, 0)),
                  pl.BlockSpec((1, 64), lambda i: (0, 0)),
                  pl.BlockSpec((64, 1), lambda i: (0, 0)),
                  pl.BlockSpec((1, 1), lambda i: (0, 0))],
        out_specs=pl.BlockSpec((1, 1), lambda i: (0, 0)),
        out_shape=jax.ShapeDtypeStruct((1, 1), _f32),
    )(M4X, y3, ue16,
      _g2(gc1["ln_edges"]), _bb2(gc1["ln_edges"]),
      _w(p["fc"]), p["fc"]["b"].reshape(1, 1))

    return tot[0, 0] / np.float32(N)


# speedup vs baseline: 1.1169x; 1.1169x over previous
import functools

import jax
import jax.numpy as jnp
import numpy as np
from jax import lax
from jax.experimental import pallas as pl
from jax.experimental.pallas import tpu as pltpu
from jax.experimental.pallas import tpu_sc as plsc

N = 10000
E = 160000
H = 64
K = 3
EP = 163840
NPAD = 10112
NC = 2
NS = 16
NW = NC * NS
C = 128

_f32 = jnp.float32


def _tcmap(name, nrows, block, ins, consts, out_dims, body):
    grid = nrows // block
    in_specs = []
    for a in ins:
        if a.ndim == 3:
            in_specs.append(pl.BlockSpec((a.shape[0], block, a.shape[2]),
                                         lambda i: (0, i, 0)))
        else:
            in_specs.append(pl.BlockSpec((block, a.shape[1]), lambda i: (i, 0)))
    for c in consts:
        in_specs.append(pl.BlockSpec(c.shape, lambda i: (0,) * c.ndim))
    out_specs = [pl.BlockSpec((block, d), lambda i: (i, 0)) for d in out_dims]
    nin, ncon = len(ins), len(consts)

    def kern(*refs):
        ib = [refs[i][...] for i in range(nin)]
        cb = [refs[nin + i][...] for i in range(ncon)]
        outs = body(ib, cb)
        for k, ob in enumerate(outs):
            refs[nin + ncon + k][...] = ob

    return pl.pallas_call(
        kern,
        grid=(grid,),
        in_specs=in_specs,
        out_specs=out_specs,
        out_shape=[jax.ShapeDtypeStruct((nrows, d), _f32) for d in out_dims],
    )(*ins, *consts)


def _silu(x):
    return x * jax.nn.sigmoid(x)


def _lnorm(x, g, b):
    mu = jnp.mean(x, axis=-1, keepdims=True)
    var = jnp.mean((x - mu) ** 2, axis=-1, keepdims=True)
    return g * (x - mu) / jnp.sqrt(var + 1e-5) + b


def _mlpblk(x, w, b, g, bb):
    return _silu(_lnorm(jnp.dot(x, w, preferred_element_type=_f32) + b, g, bb))


def _linblk(x, w, b):
    y = jnp.dot(x, w, preferred_element_type=_f32)
    return y if b is None else y + b


_MESH = plsc.VectorSubcoreMesh(core_axis_name="c", subcore_axis_name="s")


def _wid_base(rows_pw):
    c = lax.axis_index("c")
    s = lax.axis_index("s")
    return (s * NC + c) * rows_pw


def _sc_gather_pairs(table, idx_a, idx_b, nrows, ow):
    rows_pw = nrows // NW
    nchunk = rows_pw // C
    np16 = ow // 16

    @functools.partial(
        pl.kernel,
        out_type=[jax.ShapeDtypeStruct((nrows, ow), _f32),
                  jax.ShapeDtypeStruct((nrows, ow), _f32)],
        mesh=_MESH,
        scratch_types=[pltpu.VMEM((C,), jnp.int32),
                       pltpu.VMEM((C,), jnp.int32),
                       pltpu.VMEM((C, 128), _f32),
                       pltpu.VMEM((C, 128), _f32),
                       pltpu.VMEM((C, ow), _f32),
                       pltpu.VMEM((C, ow), _f32),
                       pltpu.SemaphoreType.DMA,
                       pltpu.SemaphoreType.DMA],
    )
    def k(tab, ia, ib, oa, ob, iva, ivb, ga, gb, pa, pb, sema, semb):
        base0 = _wid_base(rows_pw)

        def step(j, _):
            base = base0 + j * C
            pltpu.sync_copy(ia.at[pl.ds(base, C)], iva)
            pltpu.sync_copy(ib.at[pl.ds(base, C)], ivb)
            cpa = pltpu.async_copy(tab.at[iva], ga, sema)
            cpb = pltpu.async_copy(tab.at[ivb], gb, semb)
            cpa.wait()
            cpb.wait()

            @plsc.parallel_loop(0, C, 1, unroll=4)
            def row(i):
                for kk in range(np16):
                    sl = pl.ds(kk * 16, 16)
                    pa[i, sl] = ga[i, sl]
                    pb[i, sl] = gb[i, sl]
            pltpu.sync_copy(pa, oa.at[pl.ds(base, C)])
            pltpu.sync_copy(pb, ob.at[pl.ds(base, C)])
            return _

        lax.fori_loop(0, nchunk, step, None)

    return k(table, idx_a, idx_b)


def _sc_gather_one(table, idx, nrows, ow):
    rows_pw = nrows // NW
    nchunk = rows_pw // C
    np16 = ow // 16

    @functools.partial(
        pl.kernel,
        out_type=[jax.ShapeDtypeStruct((nrows, ow), _f32)],
        mesh=_MESH,
        scratch_types=[pltpu.VMEM((C,), jnp.int32),
                       pltpu.VMEM((C, 128), _f32),
                       pltpu.VMEM((C, ow), _f32),
                       pltpu.SemaphoreType.DMA],
    )
    def k(tab, ia, oa, iva, ga, pa, sema):
        base0 = _wid_base(rows_pw)

        def step(j, _):
            base = base0 + j * C
            pltpu.sync_copy(ia.at[pl.ds(base, C)], iva)
            pltpu.async_copy(tab.at[iva], ga, sema).wait()

            @plsc.parallel_loop(0, C, 1, unroll=4)
            def row(i):
                for kk in range(np16):
                    sl = pl.ds(kk * 16, 16)
                    pa[i, sl] = ga[i, sl]
            pltpu.sync_copy(pa, oa.at[pl.ds(base, C)])
            return _

        lax.fori_loop(0, nchunk, step, None)

    res = k(table, idx)
    return res[0] if isinstance(res, (list, tuple)) else res


def _sc_edge_egg_dense(ts, td64, g, lgs, write_m, tp3):
    CB = 64
    CT = 3 * CB
    rows_pw = tp3 // NW
    bonds_pw = rows_pw // 3
    nchunk = rows_pw // CT
    outs = [jax.ShapeDtypeStruct((EP, 128), _f32)]
    if write_m:
        outs.append(jax.ShapeDtypeStruct((tp3, 64), _f32))

    @functools.partial(
        pl.kernel,
        out_type=outs,
        mesh=_MESH,
        scratch_types=[pltpu.VMEM((CT,), jnp.int32),
                       pltpu.VMEM((CT, 128), _f32),
                       pltpu.VMEM((CB, 64), _f32),
                       pltpu.VMEM((CT, 64), _f32),
                       pltpu.VMEM((CB, 128), _f32),
                       pltpu.SemaphoreType.DMA,
                       pltpu.SemaphoreType.DMA],
    )
    def k(tsr, tdr, gr, sr, so, *rest):
        if write_m:
            mo = rest[0]
            ivs, gs, tdv, gv, qs, sema, semb = rest[1:]
        else:
            mo = None
            ivs, gs, tdv, gv, qs, sema, semb = rest
        cid = lax.axis_index("c")
        sid = lax.axis_index("s")
        w = sid * NC + cid
        tbase0 = w * rows_pw
        bbase0 = w * bonds_pw

        def step(j, _):
            tbase = tbase0 + j * CT
            bbase = bbase0 + j * CB
            pltpu.sync_copy(sr.at[pl.ds(tbase, CT)], ivs)
            cpa = pltpu.async_copy(tsr.at[ivs.at[pl.ds(0, C)]],
                                   gs.at[pl.ds(0, C)], sema)
            cpb = pltpu.async_copy(tsr.at[ivs.at[pl.ds(C, CT - C)]],
                                   gs.at[pl.ds(C, CT - C)], semb)
            pltpu.sync_copy(tdr.at[pl.ds(bbase, CB)], tdv)
            pltpu.sync_copy(gr.at[pl.ds(tbase, CT)], gv)
            cpa.wait()
            cpb.wait()

            @plsc.parallel_loop(0, CB, 1, unroll=2)
            def bond(b):
                for kk in range(4):
                    sl = pl.ds(kk * 16, 16)
                    sl2 = pl.ds(64 + kk * 16, 16)
                    d = tdv[b, sl]
                    ssum = jnp.zeros((16,), _f32)
                    shsum = jnp.zeros((16,), _f32)
                    for q in range(3):
                        i = b * 3 + q
                        m = gs[i, sl] + d + gv[i, sl]
                        if write_m:
                            gv[i, sl] = m
                        sig = 1.0 / (1.0 + jnp.exp(-m))
                        ssum = ssum + sig
                        shsum = shsum + sig * gs[i, sl2]
                    qs[b, sl] = ssum
                    qs[b, sl2] = shsum
            if write_m:
                pltpu.sync_copy(gv, mo.at[pl.ds(tbase, CT)])
            pltpu.sync_copy(qs, so.at[pl.ds(bbase, CB)])
            return _

        lax.fori_loop(0, nchunk, step, None)

    res = k(ts, td64, g, lgs)
    if write_m:
        return res[0], res[1]
    return res[0] if isinstance(res, (list, tuple)) else res


def _sc_node_egg(ts, td, g, src, dst):
    rows_pw = EP // NW
    nchunk = rows_pw // C
    zrows = NPAD // NS

    @functools.partial(
        pl.kernel,
        out_type=[jax.ShapeDtypeStruct((EP, 64), _f32),
                  jax.ShapeDtypeStruct((NC, NPAD, 128), _f32)],
        mesh=_MESH,
        scratch_types=[pltpu.VMEM((C,), jnp.int32),
                       pltpu.VMEM((C,), jnp.int32),
                       pltpu.VMEM((C, 128), _f32),
                       pltpu.VMEM((C, 128), _f32),
                       pltpu.VMEM((C, 64), _f32),
                       pltpu.VMEM_SHARED((NPAD, 128), _f32),
                       pltpu.SemaphoreType.DMA,
                       pltpu.SemaphoreType.DMA],
    )
    def k(tsr, tdr, gr, sr, dr, mo, so, ivs, ivd, gs, gd, gv, acc, sema, semb):
        cid = lax.axis_index("c")
        sid = lax.axis_index("s")
        base0 = (sid * NC + cid) * rows_pw

        @plsc.parallel_loop(0, C, 1, unroll=4)
        def zrow(i):
            for kk in range(8):
                gs[i, pl.ds(kk * 16, 16)] = jnp.zeros((16,), _f32)
        for t in range(4):
            pltpu.sync_copy(gs, acc.at[pl.ds(sid * zrows + t * C, C)])
        pltpu.sync_copy(gs.at[pl.ds(0, zrows - 4 * C)],
                        acc.at[pl.ds(sid * zrows + 4 * C, zrows - 4 * C)])
        plsc.subcore_barrier()

        def step(j, _):
            base = base0 + j * C
            pltpu.sync_copy(sr.at[pl.ds(base, C)], ivs)
            pltpu.sync_copy(dr.at[pl.ds(base, C)], ivd)
            cpa = pltpu.async_copy(tsr.at[ivs], gs, sema)
            cpb = pltpu.async_copy(tdr.at[ivd], gd, semb)
            pltpu.sync_copy(gr.at[pl.ds(base, C)], gv)
            cpa.wait()
            cpb.wait()

            @plsc.parallel_loop(0, C, 1, unroll=2)
            def row(i):
                for kk in range(4):
                    a = gs[i, pl.ds(kk * 16, 16)]
                    d = gd[i, pl.ds(kk * 16, 16)]
                    gg = gv[i, pl.ds(kk * 16, 16)]
                    m = a + d + gg
                    gv[i, pl.ds(kk * 16, 16)] = m
                    sig = 1.0 / (1.0 + jnp.exp(-m))
                    gd[i, pl.ds(kk * 16, 16)] = sig
                    bh = gs[i, pl.ds(64 + kk * 16, 16)]
                    gd[i, pl.ds(64 + kk * 16, 16)] = sig * bh
            pltpu.sync_copy(gv, mo.at[pl.ds(base, C)])
            pltpu.sync_copy(gd, acc.at[ivd], add=True)
            return _

        lax.fori_loop(0, nchunk, step, None)
        plsc.subcore_barrier()
        pltpu.sync_copy(acc.at[pl.ds(sid * zrows, zrows)],
                        so.at[cid, pl.ds(sid * zrows, zrows)])

    return k(ts, td, g, src, dst)


def _sc_edge_egg(ts, td, g, lgs, lgd, tp, write_m, qrows):
    rows_pw = tp // NW
    nchunk = rows_pw // C
    outs = [jax.ShapeDtypeStruct((qrows, 128), _f32)]
    if write_m:
        outs.append(jax.ShapeDtypeStruct((tp, 64), _f32))

    @functools.partial(
        pl.kernel,
        out_type=outs,
        mesh=_MESH,
        scratch_types=[pltpu.VMEM((C,), jnp.int32),
                       pltpu.VMEM((C,), jnp.int32),
                       pltpu.VMEM((C, 128), _f32),
                       pltpu.VMEM((C, 128), _f32),
                       pltpu.VMEM((C, 64), _f32),
                       pltpu.VMEM((C, 64), _f32),
                       pltpu.VMEM((C, 128), _f32),
                       pltpu.SemaphoreType.DMA,
                       pltpu.SemaphoreType.DMA],
    )
    def k(tsr, tdr, gr, sr, dr, qo, *rest):
        if write_m:
            mo = rest[0]
            ivs, ivd, gs, gd, gv, mv, qv, sema, semb = rest[1:]
        else:
            mo = None
            ivs, ivd, gs, gd, gv, mv, qv, sema, semb = rest
        cid = lax.axis_index("c")
        sid = lax.axis_index("s")
        base0 = (sid * NC + cid) * rows_pw

        @pl.when(jnp.logical_and(cid == 0, sid == 0))
        def _():
            def zrow(i, _):
                for kk in range(8):
                    qv[i, pl.ds(kk * 16, 16)] = jnp.zeros((16,), _f32)
                return _
            lax.fori_loop(0, C, zrow, None)
            pltpu.sync_copy(qv, qo.at[pl.ds(tp, C)])

        def step(j, _):
            base = base0 + j * C
            pltpu.sync_copy(sr.at[pl.ds(base, C)], ivs)
            pltpu.sync_copy(dr.at[pl.ds(base, C)], ivd)
            cpa = pltpu.async_copy(tsr.at[ivs], gs, sema)
            cpb = pltpu.async_copy(tdr.at[ivd], gd, semb)
            pltpu.sync_copy(gr.at[pl.ds(base, C)], gv)
            cpa.wait()
            cpb.wait()

            def row(i, _):
                for kk in range(4):
                    a = gs[i, pl.ds(kk * 16, 16)]
                    d = gd[i, pl.ds(kk * 16, 16)]
                    gg = gv[i, pl.ds(kk * 16, 16)]
                    m = a + d + gg
                    if write_m:
                        mv[i, pl.ds(kk * 16, 16)] = m
                    sig = 1.0 / (1.0 + jnp.exp(-m))
                    qv[i, pl.ds(kk * 16, 16)] = sig
                    bh = gs[i, pl.ds(64 + kk * 16, 16)]
                    qv[i, pl.ds(64 + kk * 16, 16)] = sig * bh
                return _
            lax.fori_loop(0, C, row, None)
            if write_m:
                pltpu.sync_copy(mv, mo.at[pl.ds(base, C)])
            pltpu.sync_copy(qv, qo.at[pl.ds(base, C)])
            return _

        lax.fori_loop(0, nchunk, step, None)

    res = k(ts, td, g, lgs, lgd)
    if write_m:
        return res[0], res[1]
    return res[0] if isinstance(res, (list, tuple)) else res


def _sc_gather3(q, i0, i1, i2):
    rows_pw = EP // NW
    nchunk = rows_pw // C

    @functools.partial(
        pl.kernel,
        out_type=[jax.ShapeDtypeStruct((EP, 128), _f32)],
        mesh=_MESH,
        scratch_types=[pltpu.VMEM((C,), jnp.int32),
                       pltpu.VMEM((C,), jnp.int32),
                       pltpu.VMEM((C,), jnp.int32),
                       pltpu.VMEM((C, 128), _f32),
                       pltpu.VMEM((C, 128), _f32),
                       pltpu.VMEM((C, 128), _f32),
                       pltpu.VMEM((C, 128), _f32),
                       pltpu.SemaphoreType.DMA,
                       pltpu.SemaphoreType.DMA,
                       pltpu.SemaphoreType.DMA],
    )
    def k(qr, r0, r1, r2, so, v0, v1, v2, g0, g1, g2, ov, s0, s1, s2):
        base0 = _wid_base(rows_pw)

        def step(j, _):
            base = base0 + j * C
            pltpu.sync_copy(r0.at[pl.ds(base, C)], v0)
            pltpu.sync_copy(r1.at[pl.ds(base, C)], v1)
            pltpu.sync_copy(r2.at[pl.ds(base, C)], v2)
            c0 = pltpu.async_copy(qr.at[v0], g0, s0)
            c1 = pltpu.async_copy(qr.at[v1], g1, s1)
            c2 = pltpu.async_copy(qr.at[v2], g2, s2)
            c0.wait()
            c1.wait()
            c2.wait()

            def row(i, _):
                for kk in range(8):
                    sl = pl.ds(kk * 16, 16)
                    ov[i, sl] = g0[i, sl] + g1[i, sl] + g2[i, sl]
                return _
            lax.fori_loop(0, C, row, None)
            pltpu.sync_copy(ov, so.at[pl.ds(base, C)])
            return _

        lax.fori_loop(0, nchunk, step, None)

    res = k(q, i0, i1, i2)
    return res[0] if isinstance(res, (list, tuple)) else res


def _sc_gather_add80(ta, tdx, g, src, dst):
    rows_pw = EP // NW
    nchunk = rows_pw // C

    @functools.partial(
        pl.kernel,
        out_type=[jax.ShapeDtypeStruct((EP, 80), _f32)],
        mesh=_MESH,
        scratch_types=[pltpu.VMEM((C,), jnp.int32),
                       pltpu.VMEM((C,), jnp.int32),
                       pltpu.VMEM((C, 128), _f32),
                       pltpu.VMEM((C, 128), _f32),
                       pltpu.VMEM((C, 80), _f32),
                       pltpu.VMEM((C, 80), _f32),
                       pltpu.SemaphoreType.DMA,
                       pltpu.SemaphoreType.DMA],
    )
    def k(tar, tdr, gr, sr, dr, mo, ivs, ivd, ga, gd, gv, mv, sema, semb):
        base0 = _wid_base(rows_pw)

        def step(j, _):
            base = base0 + j * C
            pltpu.sync_copy(sr.at[pl.ds(base, C)], ivs)
            pltpu.sync_copy(dr.at[pl.ds(base, C)], ivd)
            cpa = pltpu.async_copy(tar.at[ivs], ga, sema)
            cpb = pltpu.async_copy(tdr.at[ivd], gd, semb)
            pltpu.sync_copy(gr.at[pl.ds(base, C)], gv)
            cpa.wait()
            cpb.wait()

            @plsc.parallel_loop(0, C, 1, unroll=4)
            def row(i):
                for kk in range(5):
                    sl = pl.ds(kk * 16, 16)
                    mv[i, sl] = ga[i, sl] + gd[i, sl] + gv[i, sl]
            pltpu.sync_copy(mv, mo.at[pl.ds(base, C)])
            return _

        lax.fori_loop(0, nchunk, step, None)

    res = k(ta, tdx, g, src, dst)
    return res[0] if isinstance(res, (list, tuple)) else res


def _w(p):
    return p["w"]


def _b2(p):
    return p["b"].reshape(1, -1)


def _g2(p):
    return p["g"].reshape(1, -1)


def _bb2(p):
    return p["b"].reshape(1, -1)


def kernel(r, params, atom_numbers, edge_index, lg_src, lg_dst):
    T = lg_dst.shape[0]
    dense3 = (T == 3 * E)
    TPad = 3 * EP if dense3 else ((T + 4095) // 4096) * 4096

    src = edge_index[0].astype(jnp.int32)
    dst = edge_index[1].astype(jnp.int32)
    lgs = lg_src.astype(jnp.int32)
    lgd = lg_dst.astype(jnp.int32)

    src_p = jnp.concatenate([src, jnp.full((EP - E,), N, jnp.int32)])
    dst_p = jnp.concatenate([dst, jnp.full((EP - E,), N, jnp.int32)])
    lgs_p = jnp.concatenate([lgs, jnp.full((TPad - T,), E, jnp.int32)])
    lgd_p = jnp.concatenate([lgd, jnp.full((TPad - T,), E, jnp.int32)])
    if dense3:
        qrows = None
        idx3 = None
    else:
        qrows = TPad + C
        se = jnp.searchsorted(lgd, jnp.arange(E + 1, dtype=jnp.int32)).astype(jnp.int32)
        s_p = jnp.concatenate([se[:E], jnp.zeros((EP - E,), jnp.int32)])
        e_p = jnp.concatenate([se[1:], jnp.zeros((EP - E,), jnp.int32)])
        idx3 = [jnp.where(s_p + i < e_p, s_p + i, TPad).astype(jnp.int32)
                for i in range(K)]

    r16 = jnp.zeros((EP, 16), _f32).at[:E, 0:3].set(r.astype(_f32))
    an2 = jnp.concatenate([atom_numbers.astype(jnp.int32),
                           jnp.zeros((NPAD - N,), jnp.int32)]).reshape(NPAD, 1)

    p = params
    al0n, al0e = p["alignn"][0]["node"], p["alignn"][0]["edge"]
    al1n, al1e = p["alignn"][1]["node"], p["alignn"][1]["edge"]
    gc0, gc1 = p["gcn"][0], p["gcn"][1]

    cent_e = jnp.linspace(0.0, 8.0, 80).astype(_f32).reshape(1, 80)
    cent_a = jnp.linspace(-1.0, 1.0, 40).astype(_f32).reshape(1, 40)
    gam_e = float(79.0 / 8.0)
    gam_a = 19.5

    def t1_body(ib, cb):
        (X,) = ib
        (ce, w1, b1, g1_, bb1, w2, b2_, g2_, bb2, weg, beg) = cb
        bl2 = jnp.sum(X * X, axis=-1, keepdims=True)
        bl = jnp.sqrt(bl2)
        inv = 1.0 / jnp.maximum(bl, 1e-9)
        u = X * inv
        cutv = jnp.where(bl < 3.8, 1.0, 0.5 - 0.5 * jnp.sin(np.pi * (bl - 3.9) / 0.2))
        cutv = jnp.where(bl > 4.0, 0.0, cutv)
        rb = jnp.exp(-gam_e * (bl - ce) ** 2)
        y0 = _mlpblk(rb, w1, b1, g1_, bb1)
        y0 = _mlpblk(y0, w2, b2_, g2_, bb2)
        G1 = _linblk(y0, weg, beg)
        zpad = jnp.zeros((X.shape[0], 123), _f32)
        u_ext = jnp.concatenate([u[:, 0:3], bl, cutv, zpad], axis=1)
        ue16 = u_ext[:, 0:16]
        return y0, G1, u_ext, ue16

    y0, G1, u_ext, ue16 = _tcmap(
        "t1", EP, 2048, [r16],
        [cent_e,
         _w(p["edge_mlp1"]["lin"]), _b2(p["edge_mlp1"]["lin"]),
         _g2(p["edge_mlp1"]["ln"]), _bb2(p["edge_mlp1"]["ln"]),
         _w(p["edge_mlp2"]["lin"]), _b2(p["edge_mlp2"]["lin"]),
         _g2(p["edge_mlp2"]["ln"]), _bb2(p["edge_mlp2"]["ln"]),
         _w(al0n["edge_gate"]), _b2(al0n["edge_gate"])],
        [64, 64, 128, 16], t1_body)

    if dense3:
        uu1 = _sc_gather_one(u_ext, lgs_p, TPad, 16)
        uu2 = jnp.repeat(ue16, 3, axis=0)
    else:
        uu1, uu2 = _sc_gather_pairs(u_ext, lgs_p, lgd_p, TPad, 16)

    def t2_body(ib, cb):
        (U1, U2) = ib
        (ca, w1, b1, g1_, bb1, w2, b2_, g2_, bb2, weg, beg) = cb
        cos = -jnp.sum(U1[:, 0:3] * U2[:, 0:3], axis=-1, keepdims=True)
        cos = jnp.clip(cos, -1.0, 1.0)
        rb = jnp.exp(-gam_a * (cos - ca) ** 2)
        z0 = _mlpblk(rb, w1, b1, g1_, bb1)
        z0 = _mlpblk(z0, w2, b2_, g2_, bb2)
        gp = _linblk(z0, weg, beg)
        return z0, gp

    z0, gp1 = _tcmap(
        "t2", TPad, 4096, [uu1, uu2],
        [cent_a,
         _w(p["angle_mlp1"]["lin"]), _b2(p["angle_mlp1"]["lin"]),
         _g2(p["angle_mlp1"]["ln"]), _bb2(p["angle_mlp1"]["ln"]),
         _w(p["angle_mlp2"]["lin"]), _b2(p["angle_mlp2"]["lin"]),
         _g2(p["angle_mlp2"]["ln"]), _bb2(p["angle_mlp2"]["ln"]),
         _w(al0e["edge_gate"]), _b2(al0e["edge_gate"])],
        [64, 64], t2_body)

    def t3_body(ib, cb):
        (an,) = ib
        (emb, wsg, bsg, wdu, bdu, wdg, bdg, wsu, bsu, wes, bes, wed) = cb
        onehot = (lax.broadcasted_iota(jnp.int32, (an.shape[0], 128), 1)
                  == an).astype(_f32)
        x0 = jnp.dot(onehot, emb, preferred_element_type=_f32)
        ts = jnp.concatenate([_linblk(x0, wsg, bsg), _linblk(x0, wdu, bdu)], axis=1)
        td = jnp.concatenate([_linblk(x0, wdg, bdg),
                              jnp.zeros((x0.shape[0], 64), _f32)], axis=1)
        su = _linblk(x0, wsu, bsu)
        es = _linblk(x0, wes, bes)
        ed = jnp.dot(x0, wed, preferred_element_type=_f32)
        zpad = jnp.zeros_like(es) * 0.0
        esed = jnp.concatenate([es, ed, zpad, zpad], axis=1)
        return x0, ts, td, su, esed

    x0, TS1, TD1, SU1, ES = _tcmap(
        "t3", NPAD, 1264, [an2],
        [p["atom_embedding"],
         _w(al0n["src_gate"]), _b2(al0n["src_gate"]),
         _w(al0n["dst_update"]), _b2(al0n["dst_update"]),
         _w(al0n["dst_gate"]), _b2(al0n["dst_gate"]),
         _w(al0n["src_update"]), _b2(al0n["src_update"]),
         _w(p["int_src"]), _b2(p["int_src"]),
         _w(p["int_dst"])],
        [64, 128, 128, 64, 16], t3_body)

    def node_update_body(ib, cb):
        (x, su, S) = ib
        lng, lnb = cb[0], cb[1]
        s = S[0, :, 0:64] + S[1, :, 0:64]
        sh = S[0, :, 64:128] + S[1, :, 64:128]
        h = sh / (s + 1e-6)
        xn = x + _silu(_lnorm(su + h, lng, lnb))
        outs = [xn]
        ws = cb[2:]
        res = []
        for t in range(0, len(ws), 2):
            res.append(_linblk(xn, ws[t], ws[t + 1]))
        if len(res) == 4:
            outs.append(jnp.concatenate([res[0], res[1]], axis=1))
            outs.append(jnp.concatenate([res[2], jnp.zeros_like(res[2])], axis=1))
            outs.append(res[3])
        else:
            outs.extend(res)
        return outs

    def edge_update_body(ib, cb):
        (m, y) = ib
        lng, lnb = cb[0], cb[1]
        yn = y + _silu(_lnorm(m, lng, lnb))
        outs = [yn]
        ws = cb[2:]
        res = []
        for t in range(0, len(ws), 2):
            res.append(_linblk(yn, ws[t], ws[t + 1]))
        if len(res) == 4:
            outs.append(jnp.concatenate([res[0], res[1]], axis=1))
            outs.append(res[2])
            outs.append(res[3])
        else:
            outs.extend(res)
        return outs

    def eggw(q):
        return [_w(q["src_gate"]), _b2(q["src_gate"]),
                _w(q["dst_update"]), _b2(q["dst_update"]),
                _w(q["dst_gate"]), _b2(q["dst_gate"]),
                _w(q["src_update"]), _b2(q["src_update"])]

    def tri_update_body(ib, cb):
        (ylg, su, Sp) = ib
        lng, lnb = cb[0], cb[1]
        s = Sp[:, 0:64]
        sh = Sp[:, 64:128]
        h = sh / (s + 1e-6)
        yn = ylg + _silu(_lnorm(su + h, lng, lnb))
        outs = [yn]
        ws = cb[2:]
        for t in range(0, len(ws), 2):
            outs.append(_linblk(yn, ws[t], ws[t + 1]))
        return outs

    M1, S1 = _sc_node_egg(TS1, TD1, G1, src_p, dst_p)
    x1, TS2, TD2, SU2 = _tcmap(
        "t4a", NPAD, 1264, [x0, SU1, S1],
        [_g2(al0n["ln_nodes"]), _bb2(al0n["ln_nodes"])] + eggw(al1n),
        [64, 128, 128, 64], node_update_body)
    ylg1, TSp1, TDp1, SUp1 = _tcmap(
        "t5a", EP, 2048, [M1, y0],
        [_g2(al0n["ln_edges"]), _bb2(al0n["ln_edges"])] + eggw(al0e),
        [64, 128, 64, 64], edge_update_body)
    if dense3:
        Sp1, Mp1 = _sc_edge_egg_dense(TSp1, TDp1, gp1, lgs_p, True, TPad)
    else:
        TDp1g = jnp.pad(TDp1, ((0, 0), (0, 64)))
        Q1, Mp1 = _sc_edge_egg(TSp1, TDp1g, gp1, lgs_p, lgd_p, TPad, True, qrows)
        Sp1 = _sc_gather3(Q1, *idx3)
    y1, G2 = _tcmap(
        "t6a", EP, 2048, [ylg1, SUp1, Sp1],
        [_g2(al0e["ln_nodes"]), _bb2(al0e["ln_nodes"]),
         _w(al1n["edge_gate"]), _b2(al1n["edge_gate"])],
        [64, 64], tri_update_body)

    def zup_body(ib, cb):
        (m, z) = ib
        lng, lnb, weg, beg = cb
        zn = z + _silu(_lnorm(m, lng, lnb))
        return (_linblk(zn, weg, beg),)

    (gp2,) = _tcmap(
        "t5z", TPad, 4096, [Mp1, z0],
        [_g2(al0e["ln_edges"]), _bb2(al0e["ln_edges"]),
         _w(al1e["edge_gate"]), _b2(al1e["edge_gate"])],
        [64], zup_body)

    M2, S2 = _sc_node_egg(TS2, TD2, G2, src_p, dst_p)
    x2, TS3, TD3, SU3 = _tcmap(
        "t4b", NPAD, 1264, [x1, SU2, S2],
        [_g2(al1n["ln_nodes"]), _bb2(al1n["ln_nodes"])] + eggw(gc0),
        [64, 128, 128, 64], node_update_body)
    ylg2, TSp2, TDp2, SUp2 = _tcmap(
        "t5b", EP, 2048, [M2, y1],
        [_g2(al1n["ln_edges"]), _bb2(al1n["ln_edges"])] + eggw(al1e),
        [64, 128, 64, 64], edge_update_body)
    if dense3:
        Sp2 = _sc_edge_egg_dense(TSp2, TDp2, gp2, lgs_p, False, TPad)
    else:
        TDp2g = jnp.pad(TDp2, ((0, 0), (0, 64)))
        Q2 = _sc_edge_egg(TSp2, TDp2g, gp2, lgs_p, lgd_p, TPad, False, qrows)
        Sp2 = _sc_gather3(Q2, *idx3)
    y2, G3 = _tcmap(
        "t6b", EP, 2048, [ylg2, SUp2, Sp2],
        [_g2(al1e["ln_nodes"]), _bb2(al1e["ln_nodes"]),
         _w(gc0["edge_gate"]), _b2(gc0["edge_gate"])],
        [64, 64], tri_update_body)

    M3, S3 = _sc_node_egg(TS3, TD3, G3, src_p, dst_p)

    def t4c_body(ib, cb):
        (x, su, S, esed) = ib
        lng, lnb, wsg, bsg, wdg, bdg = cb
        s = S[0, :, 0:64] + S[1, :, 0:64]
        sh = S[0, :, 64:128] + S[1, :, 64:128]
        h = sh / (s + 1e-6)
        xn = x + _silu(_lnorm(su + h, lng, lnb))
        a4 = _linblk(xn, wsg, bsg)
        d4 = _linblk(xn, wdg, bdg)
        zp = jnp.zeros((xn.shape[0], 60), _f32)
        ta = jnp.concatenate([a4, esed[:, 0:4], zp], axis=1)
        tdx = jnp.concatenate([d4, esed[:, 4:8], zp], axis=1)
        return ta, tdx

    TA, TDX = _tcmap(
        "t4c", NPAD, 1264, [x2, SU3, S3, ES],
        [_g2(gc0["ln_nodes"]), _bb2(gc0["ln_nodes"]),
         _w(gc1["src_gate"]), _b2(gc1["src_gate"]),
         _w(gc1["dst_gate"]), _b2(gc1["dst_gate"])],
        [128, 128], t4c_body)

    def t5c_body(ib, cb):
        (m, y) = ib
        lng, lnb, weg, beg = cb
        yn = y + _silu(_lnorm(m, lng, lnb))
        g4 = _linblk(yn, weg, beg)
        g4p = jnp.concatenate([g4, jnp.zeros_like(yn[:, 0:16])], axis=1)
        return yn, g4p

    y3, G4P = _tcmap(
        "t5c", EP, 2048, [M3, y2],
        [_g2(gc0["ln_edges"]), _bb2(gc0["ln_edges"]),
         _w(gc1["edge_gate"]), _b2(gc1["edge_gate"])],
        [64, 80], t5c_body)

    M4X = _sc_gather_add80(TA, TDX, G4P, src_p, dst_p)

    def t7(m4x_ref, y3_ref, ue_ref, lng_ref, lnb_ref, fw_ref, fb_ref, out_ref):
        i = pl.program_id(0)
        m4 = m4x_ref[:, 0:64]
        esd = m4x_ref[:, 64:68]
        y = y3_ref[...]
        yn = y + _silu(_lnorm(m4, lng_ref[...], lnb_ref[...]))
        bond = jax.nn.sigmoid(
            jnp.dot(yn, fw_ref[...], preferred_element_type=_f32)
            + fb_ref[...])
        pe = jnp.exp(esd)
        bl = ue_ref[:, 3:4]
        cutv = ue_ref[:, 4:5]
        f_rep = pe[:, 0:1] * jnp.exp(-pe[:, 1:2] * bl)
        f_att = pe[:, 2:3] * jnp.exp(-pe[:, 3:4] * bl)
        V = cutv * (f_rep - bond * f_att)
        rowid = i * 2048 + lax.broadcasted_iota(jnp.int32, (2048, 1), 0)
        V = jnp.where(rowid < E, V, 0.0)
        bs = jnp.sum(V)

        @pl.when(i == 0)
        def _():
            out_ref[...] = jnp.zeros((1, 1), _f32)
        out_ref[...] += jnp.reshape(bs, (1, 1))

    tot = pl.pallas_call(
        t7,
        grid=(EP // 2048,),
        in_specs=[pl.BlockSpec((2048, 80), lambda i: (i, 0)),
                  pl.BlockSpec((2048, 64), lambda i: (i, 0)),
                  pl.BlockSpec((2048, 16), lambda i: (i, 0)),
                  pl.BlockSpec((1, 64), lambda i: (0, 0)),
                  pl.BlockSpec((1, 64), lambda i: (0, 0)),
                  pl.BlockSpec((64, 1), lambda i: (0, 0)),
                  pl.BlockSpec((1, 1), lambda i: (0, 0))],
        out_specs=pl.BlockSpec((1, 1), lambda i: (0, 0)),
        out_shape=jax.ShapeDtypeStruct((1, 1), _f32),
    )(M4X, y3, ue16,
      _g2(gc1["ln_edges"]), _bb2(gc1["ln_edges"]),
      _w(p["fc"]), p["fc"]["b"].reshape(1, 1))

    return tot[0, 0] / np.float32(N)


# speedup vs baseline: 1.1220x; 1.0046x over previous
import functools

import jax
import jax.numpy as jnp
import numpy as np
from jax import lax
from jax.experimental import pallas as pl
from jax.experimental.pallas import tpu as pltpu
from jax.experimental.pallas import tpu_sc as plsc

N = 10000
E = 160000
H = 64
K = 3
EP = 163840
NPAD = 10112
NC = 2
NS = 16
NW = NC * NS
C = 128

_f32 = jnp.float32


def _tcmap(name, nrows, block, ins, consts, out_dims, body):
    grid = nrows // block
    in_specs = []
    for a in ins:
        if a.ndim == 3:
            in_specs.append(pl.BlockSpec((a.shape[0], block, a.shape[2]),
                                         lambda i: (0, i, 0)))
        else:
            rb = block * a.shape[0] // nrows
            in_specs.append(pl.BlockSpec((rb, a.shape[1]), lambda i: (i, 0)))
    for c in consts:
        in_specs.append(pl.BlockSpec(c.shape, lambda i: (0,) * c.ndim))
    out_specs = [pl.BlockSpec((block, d), lambda i: (i, 0)) for d in out_dims]
    nin, ncon = len(ins), len(consts)

    def kern(*refs):
        ib = [refs[i][...] for i in range(nin)]
        cb = [refs[nin + i][...] for i in range(ncon)]
        outs = body(ib, cb)
        for k, ob in enumerate(outs):
            refs[nin + ncon + k][...] = ob

    return pl.pallas_call(
        kern,
        grid=(grid,),
        in_specs=in_specs,
        out_specs=out_specs,
        out_shape=[jax.ShapeDtypeStruct((nrows, d), _f32) for d in out_dims],
    )(*ins, *consts)


def _silu(x):
    return x * jax.nn.sigmoid(x)


def _lnorm(x, g, b):
    mu = jnp.mean(x, axis=-1, keepdims=True)
    var = jnp.mean((x - mu) ** 2, axis=-1, keepdims=True)
    return g * (x - mu) / jnp.sqrt(var + 1e-5) + b


def _mlpblk(x, w, b, g, bb):
    return _silu(_lnorm(jnp.dot(x, w, preferred_element_type=_f32) + b, g, bb))


def _linblk(x, w, b):
    y = jnp.dot(x, w, preferred_element_type=_f32)
    return y if b is None else y + b


_MESH = plsc.VectorSubcoreMesh(core_axis_name="c", subcore_axis_name="s")


def _wid_base(rows_pw):
    c = lax.axis_index("c")
    s = lax.axis_index("s")
    return (s * NC + c) * rows_pw


def _sc_gather_pairs(table, idx_a, idx_b, nrows, ow):
    rows_pw = nrows // NW
    nchunk = rows_pw // C
    np16 = ow // 16

    @functools.partial(
        pl.kernel,
        out_type=[jax.ShapeDtypeStruct((nrows, ow), _f32),
                  jax.ShapeDtypeStruct((nrows, ow), _f32)],
        mesh=_MESH,
        scratch_types=[pltpu.VMEM((C,), jnp.int32),
                       pltpu.VMEM((C,), jnp.int32),
                       pltpu.VMEM((C, 128), _f32),
                       pltpu.VMEM((C, 128), _f32),
                       pltpu.VMEM((C, ow), _f32),
                       pltpu.VMEM((C, ow), _f32),
                       pltpu.SemaphoreType.DMA,
                       pltpu.SemaphoreType.DMA],
    )
    def k(tab, ia, ib, oa, ob, iva, ivb, ga, gb, pa, pb, sema, semb):
        base0 = _wid_base(rows_pw)

        def step(j, _):
            base = base0 + j * C
            pltpu.sync_copy(ia.at[pl.ds(base, C)], iva)
            pltpu.sync_copy(ib.at[pl.ds(base, C)], ivb)
            cpa = pltpu.async_copy(tab.at[iva], ga, sema)
            cpb = pltpu.async_copy(tab.at[ivb], gb, semb)
            cpa.wait()
            cpb.wait()

            @plsc.parallel_loop(0, C, 1, unroll=4)
            def row(i):
                for kk in range(np16):
                    sl = pl.ds(kk * 16, 16)
                    pa[i, sl] = ga[i, sl]
                    pb[i, sl] = gb[i, sl]
            pltpu.sync_copy(pa, oa.at[pl.ds(base, C)])
            pltpu.sync_copy(pb, ob.at[pl.ds(base, C)])
            return _

        lax.fori_loop(0, nchunk, step, None)

    return k(table, idx_a, idx_b)


def _sc_gather_one(table, idx, nrows, ow):
    rows_pw = nrows // NW
    nchunk = rows_pw // C
    np16 = ow // 16

    @functools.partial(
        pl.kernel,
        out_type=[jax.ShapeDtypeStruct((nrows, ow), _f32)],
        mesh=_MESH,
        scratch_types=[pltpu.VMEM((C,), jnp.int32),
                       pltpu.VMEM((C, 128), _f32),
                       pltpu.VMEM((C, ow), _f32),
                       pltpu.SemaphoreType.DMA],
    )
    def k(tab, ia, oa, iva, ga, pa, sema):
        base0 = _wid_base(rows_pw)

        def step(j, _):
            base = base0 + j * C
            pltpu.sync_copy(ia.at[pl.ds(base, C)], iva)
            pltpu.async_copy(tab.at[iva], ga, sema).wait()

            @plsc.parallel_loop(0, C, 1, unroll=4)
            def row(i):
                for kk in range(np16):
                    sl = pl.ds(kk * 16, 16)
                    pa[i, sl] = ga[i, sl]
            pltpu.sync_copy(pa, oa.at[pl.ds(base, C)])
            return _

        lax.fori_loop(0, nchunk, step, None)

    res = k(table, idx)
    return res[0] if isinstance(res, (list, tuple)) else res


def _sc_edge_egg_dense(ts, td64, g, lgs, write_m, tp3):
    CB = 64
    CT = 3 * CB
    rows_pw = tp3 // NW
    bonds_pw = rows_pw // 3
    nchunk = rows_pw // CT
    outs = [jax.ShapeDtypeStruct((EP, 128), _f32)]
    if write_m:
        outs.append(jax.ShapeDtypeStruct((tp3, 64), _f32))

    @functools.partial(
        pl.kernel,
        out_type=outs,
        mesh=_MESH,
        scratch_types=[pltpu.VMEM((CT,), jnp.int32),
                       pltpu.VMEM((CT, 128), _f32),
                       pltpu.VMEM((CB, 64), _f32),
                       pltpu.VMEM((CT, 64), _f32),
                       pltpu.VMEM((CB, 128), _f32),
                       pltpu.SemaphoreType.DMA,
                       pltpu.SemaphoreType.DMA],
    )
    def k(tsr, tdr, gr, sr, so, *rest):
        if write_m:
            mo = rest[0]
            ivs, gs, tdv, gv, qs, sema, semb = rest[1:]
        else:
            mo = None
            ivs, gs, tdv, gv, qs, sema, semb = rest
        cid = lax.axis_index("c")
        sid = lax.axis_index("s")
        w = sid * NC + cid
        tbase0 = w * rows_pw
        bbase0 = w * bonds_pw

        def step(j, _):
            tbase = tbase0 + j * CT
            bbase = bbase0 + j * CB
            pltpu.sync_copy(sr.at[pl.ds(tbase, CT)], ivs)
            cpa = pltpu.async_copy(tsr.at[ivs.at[pl.ds(0, C)]],
                                   gs.at[pl.ds(0, C)], sema)
            cpb = pltpu.async_copy(tsr.at[ivs.at[pl.ds(C, CT - C)]],
                                   gs.at[pl.ds(C, CT - C)], semb)
            pltpu.sync_copy(tdr.at[pl.ds(bbase, CB)], tdv)
            pltpu.sync_copy(gr.at[pl.ds(tbase, CT)], gv)
            cpa.wait()
            cpb.wait()

            @plsc.parallel_loop(0, CB, 1, unroll=2)
            def bond(b):
                for kk in range(4):
                    sl = pl.ds(kk * 16, 16)
                    sl2 = pl.ds(64 + kk * 16, 16)
                    d = tdv[b, sl]
                    ssum = jnp.zeros((16,), _f32)
                    shsum = jnp.zeros((16,), _f32)
                    for q in range(3):
                        i = b * 3 + q
                        m = gs[i, sl] + d + gv[i, sl]
                        if write_m:
                            gv[i, sl] = m
                        sig = 1.0 / (1.0 + jnp.exp(-m))
                        ssum = ssum + sig
                        shsum = shsum + sig * gs[i, sl2]
                    qs[b, sl] = ssum
                    qs[b, sl2] = shsum
            if write_m:
                pltpu.sync_copy(gv, mo.at[pl.ds(tbase, CT)])
            pltpu.sync_copy(qs, so.at[pl.ds(bbase, CB)])
            return _

        lax.fori_loop(0, nchunk, step, None)

    res = k(ts, td64, g, lgs)
    if write_m:
        return res[0], res[1]
    return res[0] if isinstance(res, (list, tuple)) else res


def _sc_node_egg(ts, td, g, src, dst):
    rows_pw = EP // NW
    nchunk = rows_pw // C
    zrows = NPAD // NS

    @functools.partial(
        pl.kernel,
        out_type=[jax.ShapeDtypeStruct((EP, 64), _f32),
                  jax.ShapeDtypeStruct((NC, NPAD, 128), _f32)],
        mesh=_MESH,
        scratch_types=[pltpu.VMEM((C,), jnp.int32),
                       pltpu.VMEM((C,), jnp.int32),
                       pltpu.VMEM((C, 128), _f32),
                       pltpu.VMEM((C, 128), _f32),
                       pltpu.VMEM((C, 64), _f32),
                       pltpu.VMEM_SHARED((NPAD, 128), _f32),
                       pltpu.SemaphoreType.DMA,
                       pltpu.SemaphoreType.DMA],
    )
    def k(tsr, tdr, gr, sr, dr, mo, so, ivs, ivd, gs, gd, gv, acc, sema, semb):
        cid = lax.axis_index("c")
        sid = lax.axis_index("s")
        base0 = (sid * NC + cid) * rows_pw

        @plsc.parallel_loop(0, C, 1, unroll=4)
        def zrow(i):
            for kk in range(8):
                gs[i, pl.ds(kk * 16, 16)] = jnp.zeros((16,), _f32)
        for t in range(4):
            pltpu.sync_copy(gs, acc.at[pl.ds(sid * zrows + t * C, C)])
        pltpu.sync_copy(gs.at[pl.ds(0, zrows - 4 * C)],
                        acc.at[pl.ds(sid * zrows + 4 * C, zrows - 4 * C)])
        plsc.subcore_barrier()

        def step(j, _):
            base = base0 + j * C
            pltpu.sync_copy(sr.at[pl.ds(base, C)], ivs)
            pltpu.sync_copy(dr.at[pl.ds(base, C)], ivd)
            cpa = pltpu.async_copy(tsr.at[ivs], gs, sema)
            cpb = pltpu.async_copy(tdr.at[ivd], gd, semb)
            pltpu.sync_copy(gr.at[pl.ds(base, C)], gv)
            cpa.wait()
            cpb.wait()

            @plsc.parallel_loop(0, C, 1, unroll=2)
            def row(i):
                for kk in range(4):
                    a = gs[i, pl.ds(kk * 16, 16)]
                    d = gd[i, pl.ds(kk * 16, 16)]
                    gg = gv[i, pl.ds(kk * 16, 16)]
                    m = a + d + gg
                    gv[i, pl.ds(kk * 16, 16)] = m
                    sig = 1.0 / (1.0 + jnp.exp(-m))
                    gd[i, pl.ds(kk * 16, 16)] = sig
                    bh = gs[i, pl.ds(64 + kk * 16, 16)]
                    gd[i, pl.ds(64 + kk * 16, 16)] = sig * bh
            pltpu.sync_copy(gv, mo.at[pl.ds(base, C)])
            pltpu.sync_copy(gd, acc.at[ivd], add=True)
            return _

        lax.fori_loop(0, nchunk, step, None)
        plsc.subcore_barrier()
        pltpu.sync_copy(acc.at[pl.ds(sid * zrows, zrows)],
                        so.at[cid, pl.ds(sid * zrows, zrows)])

    return k(ts, td, g, src, dst)


def _sc_edge_egg(ts, td, g, lgs, lgd, tp, write_m, qrows):
    rows_pw = tp // NW
    nchunk = rows_pw // C
    outs = [jax.ShapeDtypeStruct((qrows, 128), _f32)]
    if write_m:
        outs.append(jax.ShapeDtypeStruct((tp, 64), _f32))

    @functools.partial(
        pl.kernel,
        out_type=outs,
        mesh=_MESH,
        scratch_types=[pltpu.VMEM((C,), jnp.int32),
                       pltpu.VMEM((C,), jnp.int32),
                       pltpu.VMEM((C, 128), _f32),
                       pltpu.VMEM((C, 128), _f32),
                       pltpu.VMEM((C, 64), _f32),
                       pltpu.VMEM((C, 64), _f32),
                       pltpu.VMEM((C, 128), _f32),
                       pltpu.SemaphoreType.DMA,
                       pltpu.SemaphoreType.DMA],
    )
    def k(tsr, tdr, gr, sr, dr, qo, *rest):
        if write_m:
            mo = rest[0]
            ivs, ivd, gs, gd, gv, mv, qv, sema, semb = rest[1:]
        else:
            mo = None
            ivs, ivd, gs, gd, gv, mv, qv, sema, semb = rest
        cid = lax.axis_index("c")
        sid = lax.axis_index("s")
        base0 = (sid * NC + cid) * rows_pw

        @pl.when(jnp.logical_and(cid == 0, sid == 0))
        def _():
            def zrow(i, _):
                for kk in range(8):
                    qv[i, pl.ds(kk * 16, 16)] = jnp.zeros((16,), _f32)
                return _
            lax.fori_loop(0, C, zrow, None)
            pltpu.sync_copy(qv, qo.at[pl.ds(tp, C)])

        def step(j, _):
            base = base0 + j * C
            pltpu.sync_copy(sr.at[pl.ds(base, C)], ivs)
            pltpu.sync_copy(dr.at[pl.ds(base, C)], ivd)
            cpa = pltpu.async_copy(tsr.at[ivs], gs, sema)
            cpb = pltpu.async_copy(tdr.at[ivd], gd, semb)
            pltpu.sync_copy(gr.at[pl.ds(base, C)], gv)
            cpa.wait()
            cpb.wait()

            def row(i, _):
                for kk in range(4):
                    a = gs[i, pl.ds(kk * 16, 16)]
                    d = gd[i, pl.ds(kk * 16, 16)]
                    gg = gv[i, pl.ds(kk * 16, 16)]
                    m = a + d + gg
                    if write_m:
                        mv[i, pl.ds(kk * 16, 16)] = m
                    sig = 1.0 / (1.0 + jnp.exp(-m))
                    qv[i, pl.ds(kk * 16, 16)] = sig
                    bh = gs[i, pl.ds(64 + kk * 16, 16)]
                    qv[i, pl.ds(64 + kk * 16, 16)] = sig * bh
                return _
            lax.fori_loop(0, C, row, None)
            if write_m:
                pltpu.sync_copy(mv, mo.at[pl.ds(base, C)])
            pltpu.sync_copy(qv, qo.at[pl.ds(base, C)])
            return _

        lax.fori_loop(0, nchunk, step, None)

    res = k(ts, td, g, lgs, lgd)
    if write_m:
        return res[0], res[1]
    return res[0] if isinstance(res, (list, tuple)) else res


def _sc_gather3(q, i0, i1, i2):
    rows_pw = EP // NW
    nchunk = rows_pw // C

    @functools.partial(
        pl.kernel,
        out_type=[jax.ShapeDtypeStruct((EP, 128), _f32)],
        mesh=_MESH,
        scratch_types=[pltpu.VMEM((C,), jnp.int32),
                       pltpu.VMEM((C,), jnp.int32),
                       pltpu.VMEM((C,), jnp.int32),
                       pltpu.VMEM((C, 128), _f32),
                       pltpu.VMEM((C, 128), _f32),
                       pltpu.VMEM((C, 128), _f32),
                       pltpu.VMEM((C, 128), _f32),
                       pltpu.SemaphoreType.DMA,
                       pltpu.SemaphoreType.DMA,
                       pltpu.SemaphoreType.DMA],
    )
    def k(qr, r0, r1, r2, so, v0, v1, v2, g0, g1, g2, ov, s0, s1, s2):
        base0 = _wid_base(rows_pw)

        def step(j, _):
            base = base0 + j * C
            pltpu.sync_copy(r0.at[pl.ds(base, C)], v0)
            pltpu.sync_copy(r1.at[pl.ds(base, C)], v1)
            pltpu.sync_copy(r2.at[pl.ds(base, C)], v2)
            c0 = pltpu.async_copy(qr.at[v0], g0, s0)
            c1 = pltpu.async_copy(qr.at[v1], g1, s1)
            c2 = pltpu.async_copy(qr.at[v2], g2, s2)
            c0.wait()
            c1.wait()
            c2.wait()

            def row(i, _):
                for kk in range(8):
                    sl = pl.ds(kk * 16, 16)
                    ov[i, sl] = g0[i, sl] + g1[i, sl] + g2[i, sl]
                return _
            lax.fori_loop(0, C, row, None)
            pltpu.sync_copy(ov, so.at[pl.ds(base, C)])
            return _

        lax.fori_loop(0, nchunk, step, None)

    res = k(q, i0, i1, i2)
    return res[0] if isinstance(res, (list, tuple)) else res


def _sc_gather_add80(ta, tdx, g, src, dst):
    rows_pw = EP // NW
    nchunk = rows_pw // C

    @functools.partial(
        pl.kernel,
        out_type=[jax.ShapeDtypeStruct((EP, 80), _f32)],
        mesh=_MESH,
        scratch_types=[pltpu.VMEM((C,), jnp.int32),
                       pltpu.VMEM((C,), jnp.int32),
                       pltpu.VMEM((C, 128), _f32),
                       pltpu.VMEM((C, 128), _f32),
                       pltpu.VMEM((C, 80), _f32),
                       pltpu.VMEM((C, 80), _f32),
                       pltpu.SemaphoreType.DMA,
                       pltpu.SemaphoreType.DMA],
    )
    def k(tar, tdr, gr, sr, dr, mo, ivs, ivd, ga, gd, gv, mv, sema, semb):
        base0 = _wid_base(rows_pw)

        def step(j, _):
            base = base0 + j * C
            pltpu.sync_copy(sr.at[pl.ds(base, C)], ivs)
            pltpu.sync_copy(dr.at[pl.ds(base, C)], ivd)
            cpa = pltpu.async_copy(tar.at[ivs], ga, sema)
            cpb = pltpu.async_copy(tdr.at[ivd], gd, semb)
            pltpu.sync_copy(gr.at[pl.ds(base, C)], gv)
            cpa.wait()
            cpb.wait()

            @plsc.parallel_loop(0, C, 1, unroll=4)
            def row(i):
                for kk in range(5):
                    sl = pl.ds(kk * 16, 16)
                    mv[i, sl] = ga[i, sl] + gd[i, sl] + gv[i, sl]
            pltpu.sync_copy(mv, mo.at[pl.ds(base, C)])
            return _

        lax.fori_loop(0, nchunk, step, None)

    res = k(ta, tdx, g, src, dst)
    return res[0] if isinstance(res, (list, tuple)) else res


def _w(p):
    return p["w"]


def _b2(p):
    return p["b"].reshape(1, -1)


def _g2(p):
    return p["g"].reshape(1, -1)


def _bb2(p):
    return p["b"].reshape(1, -1)


def kernel(r, params, atom_numbers, edge_index, lg_src, lg_dst):
    T = lg_dst.shape[0]
    dense3 = (T == 3 * E)
    TPad = 3 * EP if dense3 else ((T + 4095) // 4096) * 4096

    src = edge_index[0].astype(jnp.int32)
    dst = edge_index[1].astype(jnp.int32)
    lgs = lg_src.astype(jnp.int32)
    lgd = lg_dst.astype(jnp.int32)

    src_p = jnp.concatenate([src, jnp.full((EP - E,), N, jnp.int32)])
    dst_p = jnp.concatenate([dst, jnp.full((EP - E,), N, jnp.int32)])
    lgs_p = jnp.concatenate([lgs, jnp.full((TPad - T,), E, jnp.int32)])
    lgd_p = jnp.concatenate([lgd, jnp.full((TPad - T,), E, jnp.int32)])
    if dense3:
        qrows = None
        idx3 = None
    else:
        qrows = TPad + C
        se = jnp.searchsorted(lgd, jnp.arange(E + 1, dtype=jnp.int32)).astype(jnp.int32)
        s_p = jnp.concatenate([se[:E], jnp.zeros((EP - E,), jnp.int32)])
        e_p = jnp.concatenate([se[1:], jnp.zeros((EP - E,), jnp.int32)])
        idx3 = [jnp.where(s_p + i < e_p, s_p + i, TPad).astype(jnp.int32)
                for i in range(K)]

    r16 = jnp.zeros((EP, 16), _f32).at[:E, 0:3].set(r.astype(_f32))
    an2 = jnp.concatenate([atom_numbers.astype(jnp.int32),
                           jnp.zeros((NPAD - N,), jnp.int32)]).reshape(NPAD, 1)

    p = params
    al0n, al0e = p["alignn"][0]["node"], p["alignn"][0]["edge"]
    al1n, al1e = p["alignn"][1]["node"], p["alignn"][1]["edge"]
    gc0, gc1 = p["gcn"][0], p["gcn"][1]

    cent_e = jnp.linspace(0.0, 8.0, 80).astype(_f32).reshape(1, 80)
    cent_a = jnp.linspace(-1.0, 1.0, 40).astype(_f32).reshape(1, 40)
    gam_e = float(79.0 / 8.0)
    gam_a = 19.5

    def t1_body(ib, cb):
        (X,) = ib
        (ce, w1, b1, g1_, bb1, w2, b2_, g2_, bb2, weg, beg) = cb
        bl2 = jnp.sum(X * X, axis=-1, keepdims=True)
        bl = jnp.sqrt(bl2)
        inv = 1.0 / jnp.maximum(bl, 1e-9)
        u = X * inv
        cutv = jnp.where(bl < 3.8, 1.0, 0.5 - 0.5 * jnp.sin(np.pi * (bl - 3.9) / 0.2))
        cutv = jnp.where(bl > 4.0, 0.0, cutv)
        rb = jnp.exp(-gam_e * (bl - ce) ** 2)
        y0 = _mlpblk(rb, w1, b1, g1_, bb1)
        y0 = _mlpblk(y0, w2, b2_, g2_, bb2)
        G1 = _linblk(y0, weg, beg)
        zpad = jnp.zeros((X.shape[0], 123), _f32)
        u_ext = jnp.concatenate([u[:, 0:3], bl, cutv, zpad], axis=1)
        ue16 = u_ext[:, 0:16]
        return y0, G1, u_ext, ue16

    y0, G1, u_ext, ue16 = _tcmap(
        "t1", EP, 4096, [r16],
        [cent_e,
         _w(p["edge_mlp1"]["lin"]), _b2(p["edge_mlp1"]["lin"]),
         _g2(p["edge_mlp1"]["ln"]), _bb2(p["edge_mlp1"]["ln"]),
         _w(p["edge_mlp2"]["lin"]), _b2(p["edge_mlp2"]["lin"]),
         _g2(p["edge_mlp2"]["ln"]), _bb2(p["edge_mlp2"]["ln"]),
         _w(al0n["edge_gate"]), _b2(al0n["edge_gate"])],
        [64, 64, 128, 16], t1_body)

    if dense3:
        uu1 = _sc_gather_one(u_ext, lgs_p, TPad, 16)
        uu2 = None
    else:
        uu1, uu2 = _sc_gather_pairs(u_ext, lgs_p, lgd_p, TPad, 16)

    def t2_body(ib, cb):
        (U1, U2) = ib
        (ca, w1, b1, g1_, bb1, w2, b2_, g2_, bb2, weg, beg) = cb
        cos = -jnp.sum(U1[:, 0:3] * U2[:, 0:3], axis=-1, keepdims=True)
        cos = jnp.clip(cos, -1.0, 1.0)
        rb = jnp.exp(-gam_a * (cos - ca) ** 2)
        z0 = _mlpblk(rb, w1, b1, g1_, bb1)
        z0 = _mlpblk(z0, w2, b2_, g2_, bb2)
        gp = _linblk(z0, weg, beg)
        return z0, gp

    def t2d_body(ib, cb):
        (U1, UE) = ib
        u2 = jnp.repeat(UE, 3, axis=0)
        return t2_body([U1, u2], cb)

    t2_ins = [uu1, ue16] if dense3 else [uu1, uu2]
    t2_block = 3072 if dense3 else 4096
    z0, gp1 = _tcmap(
        "t2", TPad, t2_block, t2_ins,
        [cent_a,
         _w(p["angle_mlp1"]["lin"]), _b2(p["angle_mlp1"]["lin"]),
         _g2(p["angle_mlp1"]["ln"]), _bb2(p["angle_mlp1"]["ln"]),
         _w(p["angle_mlp2"]["lin"]), _b2(p["angle_mlp2"]["lin"]),
         _g2(p["angle_mlp2"]["ln"]), _bb2(p["angle_mlp2"]["ln"]),
         _w(al0e["edge_gate"]), _b2(al0e["edge_gate"])],
        [64, 64], t2d_body if dense3 else t2_body)

    def t3_body(ib, cb):
        (an,) = ib
        (emb, wsg, bsg, wdu, bdu, wdg, bdg, wsu, bsu, wes, bes, wed) = cb
        onehot = (lax.broadcasted_iota(jnp.int32, (an.shape[0], 128), 1)
                  == an).astype(_f32)
        x0 = jnp.dot(onehot, emb, preferred_element_type=_f32)
        ts = jnp.concatenate([_linblk(x0, wsg, bsg), _linblk(x0, wdu, bdu)], axis=1)
        td = jnp.concatenate([_linblk(x0, wdg, bdg),
                              jnp.zeros((x0.shape[0], 64), _f32)], axis=1)
        su = _linblk(x0, wsu, bsu)
        es = _linblk(x0, wes, bes)
        ed = jnp.dot(x0, wed, preferred_element_type=_f32)
        zpad = jnp.zeros_like(es) * 0.0
        esed = jnp.concatenate([es, ed, zpad, zpad], axis=1)
        return x0, ts, td, su, esed

    x0, TS1, TD1, SU1, ES = _tcmap(
        "t3", NPAD, 1264, [an2],
        [p["atom_embedding"],
         _w(al0n["src_gate"]), _b2(al0n["src_gate"]),
         _w(al0n["dst_update"]), _b2(al0n["dst_update"]),
         _w(al0n["dst_gate"]), _b2(al0n["dst_gate"]),
         _w(al0n["src_update"]), _b2(al0n["src_update"]),
         _w(p["int_src"]), _b2(p["int_src"]),
         _w(p["int_dst"])],
        [64, 128, 128, 64, 16], t3_body)

    def node_update_body(ib, cb):
        (x, su, S) = ib
        lng, lnb = cb[0], cb[1]
        s = S[0, :, 0:64] + S[1, :, 0:64]
        sh = S[0, :, 64:128] + S[1, :, 64:128]
        h = sh / (s + 1e-6)
        xn = x + _silu(_lnorm(su + h, lng, lnb))
        outs = [xn]
        ws = cb[2:]
        res = []
        for t in range(0, len(ws), 2):
            res.append(_linblk(xn, ws[t], ws[t + 1]))
        if len(res) == 4:
            outs.append(jnp.concatenate([res[0], res[1]], axis=1))
            outs.append(jnp.concatenate([res[2], jnp.zeros_like(res[2])], axis=1))
            outs.append(res[3])
        else:
            outs.extend(res)
        return outs

    def edge_update_body(ib, cb):
        (m, y) = ib
        lng, lnb = cb[0], cb[1]
        yn = y + _silu(_lnorm(m, lng, lnb))
        outs = [yn]
        ws = cb[2:]
        res = []
        for t in range(0, len(ws), 2):
            res.append(_linblk(yn, ws[t], ws[t + 1]))
        if len(res) == 4:
            outs.append(jnp.concatenate([res[0], res[1]], axis=1))
            outs.append(res[2])
            outs.append(res[3])
        else:
            outs.extend(res)
        return outs

    def eggw(q):
        return [_w(q["src_gate"]), _b2(q["src_gate"]),
                _w(q["dst_update"]), _b2(q["dst_update"]),
                _w(q["dst_gate"]), _b2(q["dst_gate"]),
                _w(q["src_update"]), _b2(q["src_update"])]

    def tri_update_body(ib, cb):
        (ylg, su, Sp) = ib
        lng, lnb = cb[0], cb[1]
        s = Sp[:, 0:64]
        sh = Sp[:, 64:128]
        h = sh / (s + 1e-6)
        yn = ylg + _silu(_lnorm(su + h, lng, lnb))
        outs = [yn]
        ws = cb[2:]
        for t in range(0, len(ws), 2):
            outs.append(_linblk(yn, ws[t], ws[t + 1]))
        return outs

    M1, S1 = _sc_node_egg(TS1, TD1, G1, src_p, dst_p)
    x1, TS2, TD2, SU2 = _tcmap(
        "t4a", NPAD, 1264, [x0, SU1, S1],
        [_g2(al0n["ln_nodes"]), _bb2(al0n["ln_nodes"])] + eggw(al1n),
        [64, 128, 128, 64], node_update_body)
    ylg1, TSp1, TDp1, SUp1 = _tcmap(
        "t5a", EP, 4096, [M1, y0],
        [_g2(al0n["ln_edges"]), _bb2(al0n["ln_edges"])] + eggw(al0e),
        [64, 128, 64, 64], edge_update_body)
    if dense3:
        Sp1, Mp1 = _sc_edge_egg_dense(TSp1, TDp1, gp1, lgs_p, True, TPad)
    else:
        TDp1g = jnp.pad(TDp1, ((0, 0), (0, 64)))
        Q1, Mp1 = _sc_edge_egg(TSp1, TDp1g, gp1, lgs_p, lgd_p, TPad, True, qrows)
        Sp1 = _sc_gather3(Q1, *idx3)
    y1, G2 = _tcmap(
        "t6a", EP, 4096, [ylg1, SUp1, Sp1],
        [_g2(al0e["ln_nodes"]), _bb2(al0e["ln_nodes"]),
         _w(al1n["edge_gate"]), _b2(al1n["edge_gate"])],
        [64, 64], tri_update_body)

    def zup_body(ib, cb):
        (m, z) = ib
        lng, lnb, weg, beg = cb
        zn = z + _silu(_lnorm(m, lng, lnb))
        return (_linblk(zn, weg, beg),)

    (gp2,) = _tcmap(
        "t5z", TPad, 4096, [Mp1, z0],
        [_g2(al0e["ln_edges"]), _bb2(al0e["ln_edges"]),
         _w(al1e["edge_gate"]), _b2(al1e["edge_gate"])],
        [64], zup_body)

    M2, S2 = _sc_node_egg(TS2, TD2, G2, src_p, dst_p)
    x2, TS3, TD3, SU3 = _tcmap(
        "t4b", NPAD, 1264, [x1, SU2, S2],
        [_g2(al1n["ln_nodes"]), _bb2(al1n["ln_nodes"])] + eggw(gc0),
        [64, 128, 128, 64], node_update_body)
    ylg2, TSp2, TDp2, SUp2 = _tcmap(
        "t5b", EP, 4096, [M2, y1],
        [_g2(al1n["ln_edges"]), _bb2(al1n["ln_edges"])] + eggw(al1e),
        [64, 128, 64, 64], edge_update_body)
    if dense3:
        Sp2 = _sc_edge_egg_dense(TSp2, TDp2, gp2, lgs_p, False, TPad)
    else:
        TDp2g = jnp.pad(TDp2, ((0, 0), (0, 64)))
        Q2 = _sc_edge_egg(TSp2, TDp2g, gp2, lgs_p, lgd_p, TPad, False, qrows)
        Sp2 = _sc_gather3(Q2, *idx3)
    y2, G3 = _tcmap(
        "t6b", EP, 4096, [ylg2, SUp2, Sp2],
        [_g2(al1e["ln_nodes"]), _bb2(al1e["ln_nodes"]),
         _w(gc0["edge_gate"]), _b2(gc0["edge_gate"])],
        [64, 64], tri_update_body)

    M3, S3 = _sc_node_egg(TS3, TD3, G3, src_p, dst_p)

    def t4c_body(ib, cb):
        (x, su, S, esed) = ib
        lng, lnb, wsg, bsg, wdg, bdg = cb
        s = S[0, :, 0:64] + S[1, :, 0:64]
        sh = S[0, :, 64:128] + S[1, :, 64:128]
        h = sh / (s + 1e-6)
        xn = x + _silu(_lnorm(su + h, lng, lnb))
        a4 = _linblk(xn, wsg, bsg)
        d4 = _linblk(xn, wdg, bdg)
        zp = jnp.zeros((xn.shape[0], 60), _f32)
        ta = jnp.concatenate([a4, esed[:, 0:4], zp], axis=1)
        tdx = jnp.concatenate([d4, esed[:, 4:8], zp], axis=1)
        return ta, tdx

    TA, TDX = _tcmap(
        "t4c", NPAD, 1264, [x2, SU3, S3, ES],
        [_g2(gc0["ln_nodes"]), _bb2(gc0["ln_nodes"]),
         _w(gc1["src_gate"]), _b2(gc1["src_gate"]),
         _w(gc1["dst_gate"]), _b2(gc1["dst_gate"])],
        [128, 128], t4c_body)

    def t5c_body(ib, cb):
        (m, y) = ib
        lng, lnb, weg, beg = cb
        yn = y + _silu(_lnorm(m, lng, lnb))
        g4 = _linblk(yn, weg, beg)
        g4p = jnp.concatenate([g4, jnp.zeros_like(yn[:, 0:16])], axis=1)
        return yn, g4p

    y3, G4P = _tcmap(
        "t5c", EP, 4096, [M3, y2],
        [_g2(gc0["ln_edges"]), _bb2(gc0["ln_edges"]),
         _w(gc1["edge_gate"]), _b2(gc1["edge_gate"])],
        [64, 80], t5c_body)

    M4X = _sc_gather_add80(TA, TDX, G4P, src_p, dst_p)

    def t7(m4x_ref, y3_ref, ue_ref, lng_ref, lnb_ref, fw_ref, fb_ref, out_ref):
        i = pl.program_id(0)
        m4 = m4x_ref[:, 0:64]
        esd = m4x_ref[:, 64:68]
        y = y3_ref[...]
        yn = y + _silu(_lnorm(m4, lng_ref[...], lnb_ref[...]))
        bond = jax.nn.sigmoid(
            jnp.dot(yn, fw_ref[...], preferred_element_type=_f32)
            + fb_ref[...])
        pe = jnp.exp(esd)
        bl = ue_ref[:, 3:4]
        cutv = ue_ref[:, 4:5]
        f_rep = pe[:, 0:1] * jnp.exp(-pe[:, 1:2] * bl)
        f_att = pe[:, 2:3] * jnp.exp(-pe[:, 3:4] * bl)
        V = cutv * (f_rep - bond * f_att)
        rowid = i * 4096 + lax.broadcasted_iota(jnp.int32, (4096, 1), 0)
        V = jnp.where(rowid < E, V, 0.0)
        bs = jnp.sum(V)

        @pl.when(i == 0)
        def _():
            out_ref[...] = jnp.zeros((1, 1), _f32)
        out_ref[...] += jnp.reshape(bs, (1, 1))

    tot = pl.pallas_call(
        t7,
        grid=(EP // 4096,),
        in_specs=[pl.BlockSpec((4096, 80), lambda i: (i, 0)),
                  pl.BlockSpec((4096, 64), lambda i: (i, 0)),
                  pl.BlockSpec((4096, 16), lambda i: (i, 0)),
                  pl.BlockSpec((1, 64), lambda i: (0, 0)),
                  pl.BlockSpec((1, 64), lambda i: (0, 0)),
                  pl.BlockSpec((64, 1), lambda i: (0, 0)),
                  pl.BlockSpec((1, 1), lambda i: (0, 0))],
        out_specs=pl.BlockSpec((1, 1), lambda i: (0, 0)),
        out_shape=jax.ShapeDtypeStruct((1, 1), _f32),
    )(M4X, y3, ue16,
      _g2(gc1["ln_edges"]), _bb2(gc1["ln_edges"]),
      _w(p["fc"]), p["fc"]["b"].reshape(1, 1))

    return tot[0, 0] / np.float32(N)


# speedup vs baseline: 1.2240x; 1.0909x over previous
import functools

import jax
import jax.numpy as jnp
import numpy as np
from jax import lax
from jax.experimental import pallas as pl
from jax.experimental.pallas import tpu as pltpu
from jax.experimental.pallas import tpu_sc as plsc

N = 10000
E = 160000
H = 64
K = 3
EP = 163840
NPAD = 10112
NC = 2
NS = 16
NW = NC * NS
C = 128

_f32 = jnp.float32


def _tcmap(name, nrows, block, ins, consts, out_dims, body):
    grid = nrows // block
    in_specs = []
    for a in ins:
        if a.ndim == 3:
            in_specs.append(pl.BlockSpec((a.shape[0], block, a.shape[2]),
                                         lambda i: (0, i, 0)))
        else:
            rb = block * a.shape[0] // nrows
            in_specs.append(pl.BlockSpec((rb, a.shape[1]), lambda i: (i, 0)))
    for c in consts:
        in_specs.append(pl.BlockSpec(c.shape, lambda i: (0,) * c.ndim))
    out_specs = [pl.BlockSpec((block, d), lambda i: (i, 0)) for d in out_dims]
    nin, ncon = len(ins), len(consts)

    def kern(*refs):
        ib = [refs[i][...] for i in range(nin)]
        cb = [refs[nin + i][...] for i in range(ncon)]
        outs = body(ib, cb)
        for k, ob in enumerate(outs):
            refs[nin + ncon + k][...] = ob

    return pl.pallas_call(
        kern,
        grid=(grid,),
        in_specs=in_specs,
        out_specs=out_specs,
        out_shape=[jax.ShapeDtypeStruct((nrows, d), _f32) for d in out_dims],
    )(*ins, *consts)


def _silu(x):
    return x * jax.nn.sigmoid(x)


def _lnorm(x, g, b):
    mu = jnp.mean(x, axis=-1, keepdims=True)
    var = jnp.mean((x - mu) ** 2, axis=-1, keepdims=True)
    return g * (x - mu) / jnp.sqrt(var + 1e-5) + b


def _mlpblk(x, w, b, g, bb):
    return _silu(_lnorm(jnp.dot(x, w, preferred_element_type=_f32) + b, g, bb))


def _linblk(x, w, b):
    y = jnp.dot(x, w, preferred_element_type=_f32)
    return y if b is None else y + b


_MESH = plsc.VectorSubcoreMesh(core_axis_name="c", subcore_axis_name="s")


def _wid_base(rows_pw):
    c = lax.axis_index("c")
    s = lax.axis_index("s")
    return (s * NC + c) * rows_pw


def _sc_gather_pairs(table, idx_a, idx_b, nrows, ow):
    rows_pw = nrows // NW
    nchunk = rows_pw // C
    np16 = ow // 16

    @functools.partial(
        pl.kernel,
        out_type=[jax.ShapeDtypeStruct((nrows, ow), _f32),
                  jax.ShapeDtypeStruct((nrows, ow), _f32)],
        mesh=_MESH,
        scratch_types=[pltpu.VMEM((C,), jnp.int32),
                       pltpu.VMEM((C,), jnp.int32),
                       pltpu.VMEM((C, 128), _f32),
                       pltpu.VMEM((C, 128), _f32),
                       pltpu.VMEM((C, ow), _f32),
                       pltpu.VMEM((C, ow), _f32),
                       pltpu.SemaphoreType.DMA,
                       pltpu.SemaphoreType.DMA],
    )
    def k(tab, ia, ib, oa, ob, iva, ivb, ga, gb, pa, pb, sema, semb):
        base0 = _wid_base(rows_pw)

        def step(j, _):
            base = base0 + j * C
            pltpu.sync_copy(ia.at[pl.ds(base, C)], iva)
            pltpu.sync_copy(ib.at[pl.ds(base, C)], ivb)
            cpa = pltpu.async_copy(tab.at[iva], ga, sema)
            cpb = pltpu.async_copy(tab.at[ivb], gb, semb)
            cpa.wait()
            cpb.wait()

            @plsc.parallel_loop(0, C, 1, unroll=4)
            def row(i):
                for kk in range(np16):
                    sl = pl.ds(kk * 16, 16)
                    pa[i, sl] = ga[i, sl]
                    pb[i, sl] = gb[i, sl]
            pltpu.sync_copy(pa, oa.at[pl.ds(base, C)])
            pltpu.sync_copy(pb, ob.at[pl.ds(base, C)])
            return _

        lax.fori_loop(0, nchunk, step, None)

    return k(table, idx_a, idx_b)


def _sc_gather_one(table, idx, nrows, ow):
    rows_pw = nrows // NW
    nchunk = rows_pw // C
    np16 = ow // 16

    @functools.partial(
        pl.kernel,
        out_type=[jax.ShapeDtypeStruct((nrows, ow), _f32)],
        mesh=_MESH,
        scratch_types=[pltpu.VMEM((C,), jnp.int32),
                       pltpu.VMEM((C,), jnp.int32),
                       pltpu.VMEM((C, 128), _f32),
                       pltpu.VMEM((C, 128), _f32),
                       pltpu.VMEM((C, ow), _f32),
                       pltpu.SemaphoreType.DMA,
                       pltpu.SemaphoreType.DMA],
    )
    def k(tab, ia, oa, iva0, iva1, ga0, ga1, pa, sem0, sem1):
        base0 = _wid_base(rows_pw)
        iva = [iva0, iva1]
        ga = [ga0, ga1]
        sem = [sem0, sem1]

        def issue(j, b):
            base = base0 + j * C
            pltpu.sync_copy(ia.at[pl.ds(base, C)], iva[b])
            pltpu.async_copy(tab.at[iva[b]], ga[b], sem[b])

        def finish(j, b):
            pltpu.make_async_copy(tab.at[iva[b]], ga[b], sem[b]).wait()

            @plsc.parallel_loop(0, C, 1, unroll=4)
            def row(i):
                for kk in range(np16):
                    sl = pl.ds(kk * 16, 16)
                    pa[i, sl] = ga[b][i, sl]
            pltpu.sync_copy(pa, oa.at[pl.ds(base0 + j * C, C)])

        issue(0, 0)

        def step(p, _):
            j0 = 2 * p
            @pl.when(j0 + 1 < nchunk)
            def _():
                issue(j0 + 1, 1)
            finish(j0, 0)
            @pl.when(j0 + 2 < nchunk)
            def _():
                issue(j0 + 2, 0)
            @pl.when(j0 + 1 < nchunk)
            def _():
                finish(j0 + 1, 1)
            return _

        lax.fori_loop(0, (nchunk + 1) // 2, step, None)

    res = k(table, idx)
    return res[0] if isinstance(res, (list, tuple)) else res


def _sc_edge_egg_dense(ts, td64, g, lgs, write_m, tp3):
    CB = 64
    CT = 3 * CB
    rows_pw = tp3 // NW
    bonds_pw = rows_pw // 3
    nchunk = rows_pw // CT
    outs = [jax.ShapeDtypeStruct((EP, 128), _f32)]
    if write_m:
        outs.append(jax.ShapeDtypeStruct((tp3, 64), _f32))

    @functools.partial(
        pl.kernel,
        out_type=outs,
        mesh=_MESH,
        scratch_types=[pltpu.VMEM((CT,), jnp.int32),
                       pltpu.VMEM((CT,), jnp.int32),
                       pltpu.VMEM((CT, 128), _f32),
                       pltpu.VMEM((CT, 128), _f32),
                       pltpu.VMEM((CB, 64), _f32),
                       pltpu.VMEM((CB, 64), _f32),
                       pltpu.VMEM((CT, 64), _f32),
                       pltpu.VMEM((CT, 64), _f32),
                       pltpu.VMEM((CB, 128), _f32),
                       pltpu.SemaphoreType.DMA,
                       pltpu.SemaphoreType.DMA],
    )
    def k(tsr, tdr, gr, sr, so, *rest):
        if write_m:
            mo = rest[0]
            rest = rest[1:]
        else:
            mo = None
        (ivs0, ivs1, gs0, gs1, tdv0, tdv1, gv0, gv1, qs, sem0, sem1) = rest
        ivs = [ivs0, ivs1]
        gs = [gs0, gs1]
        tdv = [tdv0, tdv1]
        gv = [gv0, gv1]
        sem = [sem0, sem1]
        cid = lax.axis_index("c")
        sid = lax.axis_index("s")
        w = sid * NC + cid
        tbase0 = w * rows_pw
        bbase0 = w * bonds_pw

        def issue(j, b):
            tbase = tbase0 + j * CT
            bbase = bbase0 + j * CB
            pltpu.sync_copy(sr.at[pl.ds(tbase, CT)], ivs[b])
            pltpu.async_copy(tsr.at[ivs[b].at[pl.ds(0, C)]],
                             gs[b].at[pl.ds(0, C)], sem[b])
            pltpu.async_copy(tsr.at[ivs[b].at[pl.ds(C, CT - C)]],
                             gs[b].at[pl.ds(C, CT - C)], sem[b])
            pltpu.async_copy(tdr.at[pl.ds(bbase, CB)], tdv[b], sem[b])
            pltpu.async_copy(gr.at[pl.ds(tbase, CT)], gv[b], sem[b])

        def finish(j, b):
            tbase = tbase0 + j * CT
            bbase = bbase0 + j * CB
            pltpu.make_async_copy(tsr.at[ivs[b].at[pl.ds(0, C)]],
                                  gs[b].at[pl.ds(0, C)], sem[b]).wait()
            pltpu.make_async_copy(tsr.at[ivs[b].at[pl.ds(C, CT - C)]],
                                  gs[b].at[pl.ds(C, CT - C)], sem[b]).wait()
            pltpu.make_async_copy(tdr.at[pl.ds(bbase, CB)], tdv[b], sem[b]).wait()
            pltpu.make_async_copy(gr.at[pl.ds(tbase, CT)], gv[b], sem[b]).wait()
            gsb, tdb, gvb = gs[b], tdv[b], gv[b]

            @plsc.parallel_loop(0, CB, 1, unroll=2)
            def bond(bb):
                for kk in range(4):
                    sl = pl.ds(kk * 16, 16)
                    sl2 = pl.ds(64 + kk * 16, 16)
                    d = tdb[bb, sl]
                    ssum = jnp.zeros((16,), _f32)
                    shsum = jnp.zeros((16,), _f32)
                    for q in range(3):
                        i = bb * 3 + q
                        m = gsb[i, sl] + d + gvb[i, sl]
                        if write_m:
                            gvb[i, sl] = m
                        sig = 1.0 / (1.0 + jnp.exp(-m))
                        ssum = ssum + sig
                        shsum = shsum + sig * gsb[i, sl2]
                    qs[bb, sl] = ssum
                    qs[bb, sl2] = shsum
            if write_m:
                pltpu.sync_copy(gvb, mo.at[pl.ds(tbase, CT)])
            pltpu.sync_copy(qs, so.at[pl.ds(bbase, CB)])

        issue(0, 0)

        def step(p, _):
            j0 = 2 * p
            @pl.when(j0 + 1 < nchunk)
            def _():
                issue(j0 + 1, 1)
            finish(j0, 0)
            @pl.when(j0 + 2 < nchunk)
            def _():
                issue(j0 + 2, 0)
            @pl.when(j0 + 1 < nchunk)
            def _():
                finish(j0 + 1, 1)
            return _

        lax.fori_loop(0, (nchunk + 1) // 2, step, None)

    res = k(ts, td64, g, lgs)
    if write_m:
        return res[0], res[1]
    return res[0] if isinstance(res, (list, tuple)) else res


def _sc_node_egg(ts, td, g, src, dst):
    rows_pw = EP // NW
    nchunk = rows_pw // C
    zrows = NPAD // NS

    @functools.partial(
        pl.kernel,
        out_type=[jax.ShapeDtypeStruct((EP, 64), _f32),
                  jax.ShapeDtypeStruct((NC, NPAD, 128), _f32)],
        mesh=_MESH,
        scratch_types=[pltpu.VMEM((C,), jnp.int32),
                       pltpu.VMEM((C,), jnp.int32),
                       pltpu.VMEM((C, 128), _f32),
                       pltpu.VMEM((C, 128), _f32),
                       pltpu.VMEM((C, 64), _f32),
                       pltpu.VMEM_SHARED((NPAD, 128), _f32),
                       pltpu.SemaphoreType.DMA,
                       pltpu.SemaphoreType.DMA],
    )
    def k(tsr, tdr, gr, sr, dr, mo, so, ivs, ivd, gs, gd, gv, acc, sema, semb):
        cid = lax.axis_index("c")
        sid = lax.axis_index("s")
        base0 = (sid * NC + cid) * rows_pw

        @plsc.parallel_loop(0, C, 1, unroll=4)
        def zrow(i):
            for kk in range(8):
                gs[i, pl.ds(kk * 16, 16)] = jnp.zeros((16,), _f32)
        for t in range(4):
            pltpu.sync_copy(gs, acc.at[pl.ds(sid * zrows + t * C, C)])
        pltpu.sync_copy(gs.at[pl.ds(0, zrows - 4 * C)],
                        acc.at[pl.ds(sid * zrows + 4 * C, zrows - 4 * C)])
        plsc.subcore_barrier()

        def step(j, _):
            base = base0 + j * C
            pltpu.sync_copy(sr.at[pl.ds(base, C)], ivs)
            pltpu.sync_copy(dr.at[pl.ds(base, C)], ivd)
            cpa = pltpu.async_copy(tsr.at[ivs], gs, sema)
            cpb = pltpu.async_copy(tdr.at[ivd], gd, semb)
            pltpu.sync_copy(gr.at[pl.ds(base, C)], gv)
            cpa.wait()
            cpb.wait()

            @plsc.parallel_loop(0, C, 1, unroll=2)
            def row(i):
                for kk in range(4):
                    a = gs[i, pl.ds(kk * 16, 16)]
                    d = gd[i, pl.ds(kk * 16, 16)]
                    gg = gv[i, pl.ds(kk * 16, 16)]
                    m = a + d + gg
                    gv[i, pl.ds(kk * 16, 16)] = m
                    sig = 1.0 / (1.0 + jnp.exp(-m))
                    gd[i, pl.ds(kk * 16, 16)] = sig
                    bh = gs[i, pl.ds(64 + kk * 16, 16)]
                    gd[i, pl.ds(64 + kk * 16, 16)] = sig * bh
            pltpu.sync_copy(gv, mo.at[pl.ds(base, C)])
            pltpu.sync_copy(gd, acc.at[ivd], add=True)
            return _

        lax.fori_loop(0, nchunk, step, None)
        plsc.subcore_barrier()
        pltpu.sync_copy(acc.at[pl.ds(sid * zrows, zrows)],
                        so.at[cid, pl.ds(sid * zrows, zrows)])

    return k(ts, td, g, src, dst)


def _sc_edge_egg(ts, td, g, lgs, lgd, tp, write_m, qrows):
    rows_pw = tp // NW
    nchunk = rows_pw // C
    outs = [jax.ShapeDtypeStruct((qrows, 128), _f32)]
    if write_m:
        outs.append(jax.ShapeDtypeStruct((tp, 64), _f32))

    @functools.partial(
        pl.kernel,
        out_type=outs,
        mesh=_MESH,
        scratch_types=[pltpu.VMEM((C,), jnp.int32),
                       pltpu.VMEM((C,), jnp.int32),
                       pltpu.VMEM((C, 128), _f32),
                       pltpu.VMEM((C, 128), _f32),
                       pltpu.VMEM((C, 64), _f32),
                       pltpu.VMEM((C, 64), _f32),
                       pltpu.VMEM((C, 128), _f32),
                       pltpu.SemaphoreType.DMA,
                       pltpu.SemaphoreType.DMA],
    )
    def k(tsr, tdr, gr, sr, dr, qo, *rest):
        if write_m:
            mo = rest[0]
            ivs, ivd, gs, gd, gv, mv, qv, sema, semb = rest[1:]
        else:
            mo = None
            ivs, ivd, gs, gd, gv, mv, qv, sema, semb = rest
        cid = lax.axis_index("c")
        sid = lax.axis_index("s")
        base0 = (sid * NC + cid) * rows_pw

        @pl.when(jnp.logical_and(cid == 0, sid == 0))
        def _():
            def zrow(i, _):
                for kk in range(8):
                    qv[i, pl.ds(kk * 16, 16)] = jnp.zeros((16,), _f32)
                return _
            lax.fori_loop(0, C, zrow, None)
            pltpu.sync_copy(qv, qo.at[pl.ds(tp, C)])

        def step(j, _):
            base = base0 + j * C
            pltpu.sync_copy(sr.at[pl.ds(base, C)], ivs)
            pltpu.sync_copy(dr.at[pl.ds(base, C)], ivd)
            cpa = pltpu.async_copy(tsr.at[ivs], gs, sema)
            cpb = pltpu.async_copy(tdr.at[ivd], gd, semb)
            pltpu.sync_copy(gr.at[pl.ds(base, C)], gv)
            cpa.wait()
            cpb.wait()

            def row(i, _):
                for kk in range(4):
                    a = gs[i, pl.ds(kk * 16, 16)]
                    d = gd[i, pl.ds(kk * 16, 16)]
                    gg = gv[i, pl.ds(kk * 16, 16)]
                    m = a + d + gg
                    if write_m:
                        mv[i, pl.ds(kk * 16, 16)] = m
                    sig = 1.0 / (1.0 + jnp.exp(-m))
                    qv[i, pl.ds(kk * 16, 16)] = sig
                    bh = gs[i, pl.ds(64 + kk * 16, 16)]
                    qv[i, pl.ds(64 + kk * 16, 16)] = sig * bh
                return _
            lax.fori_loop(0, C, row, None)
            if write_m:
                pltpu.sync_copy(mv, mo.at[pl.ds(base, C)])
            pltpu.sync_copy(qv, qo.at[pl.ds(base, C)])
            return _

        lax.fori_loop(0, nchunk, step, None)

    res = k(ts, td, g, lgs, lgd)
    if write_m:
        return res[0], res[1]
    return res[0] if isinstance(res, (list, tuple)) else res


def _sc_gather3(q, i0, i1, i2):
    rows_pw = EP // NW
    nchunk = rows_pw // C

    @functools.partial(
        pl.kernel,
        out_type=[jax.ShapeDtypeStruct((EP, 128), _f32)],
        mesh=_MESH,
        scratch_types=[pltpu.VMEM((C,), jnp.int32),
                       pltpu.VMEM((C,), jnp.int32),
                       pltpu.VMEM((C,), jnp.int32),
                       pltpu.VMEM((C, 128), _f32),
                       pltpu.VMEM((C, 128), _f32),
                       pltpu.VMEM((C, 128), _f32),
                       pltpu.VMEM((C, 128), _f32),
                       pltpu.SemaphoreType.DMA,
                       pltpu.SemaphoreType.DMA,
                       pltpu.SemaphoreType.DMA],
    )
    def k(qr, r0, r1, r2, so, v0, v1, v2, g0, g1, g2, ov, s0, s1, s2):
        base0 = _wid_base(rows_pw)

        def step(j, _):
            base = base0 + j * C
            pltpu.sync_copy(r0.at[pl.ds(base, C)], v0)
            pltpu.sync_copy(r1.at[pl.ds(base, C)], v1)
            pltpu.sync_copy(r2.at[pl.ds(base, C)], v2)
            c0 = pltpu.async_copy(qr.at[v0], g0, s0)
            c1 = pltpu.async_copy(qr.at[v1], g1, s1)
            c2 = pltpu.async_copy(qr.at[v2], g2, s2)
            c0.wait()
            c1.wait()
            c2.wait()

            def row(i, _):
                for kk in range(8):
                    sl = pl.ds(kk * 16, 16)
                    ov[i, sl] = g0[i, sl] + g1[i, sl] + g2[i, sl]
                return _
            lax.fori_loop(0, C, row, None)
            pltpu.sync_copy(ov, so.at[pl.ds(base, C)])
            return _

        lax.fori_loop(0, nchunk, step, None)

    res = k(q, i0, i1, i2)
    return res[0] if isinstance(res, (list, tuple)) else res


def _sc_gather_add80(ta, tdx, g, src, dst):
    rows_pw = EP // NW
    nchunk = rows_pw // C

    @functools.partial(
        pl.kernel,
        out_type=[jax.ShapeDtypeStruct((EP, 80), _f32)],
        mesh=_MESH,
        scratch_types=[pltpu.VMEM((C,), jnp.int32),
                       pltpu.VMEM((C,), jnp.int32),
                       pltpu.VMEM((C, 128), _f32),
                       pltpu.VMEM((C, 128), _f32),
                       pltpu.VMEM((C, 80), _f32),
                       pltpu.VMEM((C, 80), _f32),
                       pltpu.SemaphoreType.DMA,
                       pltpu.SemaphoreType.DMA],
    )
    def k(tar, tdr, gr, sr, dr, mo, ivs, ivd, ga, gd, gv, mv, sema, semb):
        base0 = _wid_base(rows_pw)

        def step(j, _):
            base = base0 + j * C
            pltpu.sync_copy(sr.at[pl.ds(base, C)], ivs)
            pltpu.sync_copy(dr.at[pl.ds(base, C)], ivd)
            cpa = pltpu.async_copy(tar.at[ivs], ga, sema)
            cpb = pltpu.async_copy(tdr.at[ivd], gd, semb)
            pltpu.sync_copy(gr.at[pl.ds(base, C)], gv)
            cpa.wait()
            cpb.wait()

            @plsc.parallel_loop(0, C, 1, unroll=4)
            def row(i):
                for kk in range(5):
                    sl = pl.ds(kk * 16, 16)
                    mv[i, sl] = ga[i, sl] + gd[i, sl] + gv[i, sl]
            pltpu.sync_copy(mv, mo.at[pl.ds(base, C)])
            return _

        lax.fori_loop(0, nchunk, step, None)

    res = k(ta, tdx, g, src, dst)
    return res[0] if isinstance(res, (list, tuple)) else res


def _w(p):
    return p["w"]


def _b2(p):
    return p["b"].reshape(1, -1)


def _g2(p):
    return p["g"].reshape(1, -1)


def _bb2(p):
    return p["b"].reshape(1, -1)


def kernel(r, params, atom_numbers, edge_index, lg_src, lg_dst):
    T = lg_dst.shape[0]
    dense3 = (T == 3 * E)
    TPad = 3 * EP if dense3 else ((T + 4095) // 4096) * 4096

    src = edge_index[0].astype(jnp.int32)
    dst = edge_index[1].astype(jnp.int32)
    lgs = lg_src.astype(jnp.int32)
    lgd = lg_dst.astype(jnp.int32)

    src_p = jnp.concatenate([src, jnp.full((EP - E,), N, jnp.int32)])
    dst_p = jnp.concatenate([dst, jnp.full((EP - E,), N, jnp.int32)])
    lgs_p = jnp.concatenate([lgs, jnp.full((TPad - T,), E, jnp.int32)])
    lgd_p = jnp.concatenate([lgd, jnp.full((TPad - T,), E, jnp.int32)])
    if dense3:
        qrows = None
        idx3 = None
    else:
        qrows = TPad + C
        se = jnp.searchsorted(lgd, jnp.arange(E + 1, dtype=jnp.int32)).astype(jnp.int32)
        s_p = jnp.concatenate([se[:E], jnp.zeros((EP - E,), jnp.int32)])
        e_p = jnp.concatenate([se[1:], jnp.zeros((EP - E,), jnp.int32)])
        idx3 = [jnp.where(s_p + i < e_p, s_p + i, TPad).astype(jnp.int32)
                for i in range(K)]

    r16 = jnp.zeros((EP, 16), _f32).at[:E, 0:3].set(r.astype(_f32))
    an2 = jnp.concatenate([atom_numbers.astype(jnp.int32),
                           jnp.zeros((NPAD - N,), jnp.int32)]).reshape(NPAD, 1)

    p = params
    al0n, al0e = p["alignn"][0]["node"], p["alignn"][0]["edge"]
    al1n, al1e = p["alignn"][1]["node"], p["alignn"][1]["edge"]
    gc0, gc1 = p["gcn"][0], p["gcn"][1]

    cent_e = jnp.linspace(0.0, 8.0, 80).astype(_f32).reshape(1, 80)
    cent_a = jnp.linspace(-1.0, 1.0, 40).astype(_f32).reshape(1, 40)
    gam_e = float(79.0 / 8.0)
    gam_a = 19.5

    def t1_body(ib, cb):
        (X,) = ib
        (ce, w1, b1, g1_, bb1, w2, b2_, g2_, bb2, weg, beg) = cb
        bl2 = jnp.sum(X * X, axis=-1, keepdims=True)
        bl = jnp.sqrt(bl2)
        inv = 1.0 / jnp.maximum(bl, 1e-9)
        u = X * inv
        cutv = jnp.where(bl < 3.8, 1.0, 0.5 - 0.5 * jnp.sin(np.pi * (bl - 3.9) / 0.2))
        cutv = jnp.where(bl > 4.0, 0.0, cutv)
        rb = jnp.exp(-gam_e * (bl - ce) ** 2)
        y0 = _mlpblk(rb, w1, b1, g1_, bb1)
        y0 = _mlpblk(y0, w2, b2_, g2_, bb2)
        G1 = _linblk(y0, weg, beg)
        zpad = jnp.zeros((X.shape[0], 123), _f32)
        u_ext = jnp.concatenate([u[:, 0:3], bl, cutv, zpad], axis=1)
        ue16 = u_ext[:, 0:16]
        return y0, G1, u_ext, ue16

    y0, G1, u_ext, ue16 = _tcmap(
        "t1", EP, 4096, [r16],
        [cent_e,
         _w(p["edge_mlp1"]["lin"]), _b2(p["edge_mlp1"]["lin"]),
         _g2(p["edge_mlp1"]["ln"]), _bb2(p["edge_mlp1"]["ln"]),
         _w(p["edge_mlp2"]["lin"]), _b2(p["edge_mlp2"]["lin"]),
         _g2(p["edge_mlp2"]["ln"]), _bb2(p["edge_mlp2"]["ln"]),
         _w(al0n["edge_gate"]), _b2(al0n["edge_gate"])],
        [64, 64, 128, 16], t1_body)

    if dense3:
        uu1 = _sc_gather_one(u_ext, lgs_p, TPad, 16)
        uu2 = None
    else:
        uu1, uu2 = _sc_gather_pairs(u_ext, lgs_p, lgd_p, TPad, 16)

    def t2_body(ib, cb):
        (U1, U2) = ib
        (ca, w1, b1, g1_, bb1, w2, b2_, g2_, bb2, weg, beg) = cb
        cos = -jnp.sum(U1[:, 0:3] * U2[:, 0:3], axis=-1, keepdims=True)
        cos = jnp.clip(cos, -1.0, 1.0)
        rb = jnp.exp(-gam_a * (cos - ca) ** 2)
        z0 = _mlpblk(rb, w1, b1, g1_, bb1)
        z0 = _mlpblk(z0, w2, b2_, g2_, bb2)
        gp = _linblk(z0, weg, beg)
        return z0, gp

    def t2d_body(ib, cb):
        (U1, UE) = ib
        u2 = jnp.repeat(UE, 3, axis=0)
        return t2_body([U1, u2], cb)

    t2_ins = [uu1, ue16] if dense3 else [uu1, uu2]
    t2_block = 3072 if dense3 else 4096
    z0, gp1 = _tcmap(
        "t2", TPad, t2_block, t2_ins,
        [cent_a,
         _w(p["angle_mlp1"]["lin"]), _b2(p["angle_mlp1"]["lin"]),
         _g2(p["angle_mlp1"]["ln"]), _bb2(p["angle_mlp1"]["ln"]),
         _w(p["angle_mlp2"]["lin"]), _b2(p["angle_mlp2"]["lin"]),
         _g2(p["angle_mlp2"]["ln"]), _bb2(p["angle_mlp2"]["ln"]),
         _w(al0e["edge_gate"]), _b2(al0e["edge_gate"])],
        [64, 64], t2d_body if dense3 else t2_body)

    def t3_body(ib, cb):
        (an,) = ib
        (emb, wsg, bsg, wdu, bdu, wdg, bdg, wsu, bsu, wes, bes, wed) = cb
        onehot = (lax.broadcasted_iota(jnp.int32, (an.shape[0], 128), 1)
                  == an).astype(_f32)
        x0 = jnp.dot(onehot, emb, preferred_element_type=_f32)
        ts = jnp.concatenate([_linblk(x0, wsg, bsg), _linblk(x0, wdu, bdu)], axis=1)
        td = jnp.concatenate([_linblk(x0, wdg, bdg),
                              jnp.zeros((x0.shape[0], 64), _f32)], axis=1)
        su = _linblk(x0, wsu, bsu)
        es = _linblk(x0, wes, bes)
        ed = jnp.dot(x0, wed, preferred_element_type=_f32)
        zpad = jnp.zeros_like(es) * 0.0
        esed = jnp.concatenate([es, ed, zpad, zpad], axis=1)
        return x0, ts, td, su, esed

    x0, TS1, TD1, SU1, ES = _tcmap(
        "t3", NPAD, 1264, [an2],
        [p["atom_embedding"],
         _w(al0n["src_gate"]), _b2(al0n["src_gate"]),
         _w(al0n["dst_update"]), _b2(al0n["dst_update"]),
         _w(al0n["dst_gate"]), _b2(al0n["dst_gate"]),
         _w(al0n["src_update"]), _b2(al0n["src_update"]),
         _w(p["int_src"]), _b2(p["int_src"]),
         _w(p["int_dst"])],
        [64, 128, 128, 64, 16], t3_body)

    def node_update_body(ib, cb):
        (x, su, S) = ib
        lng, lnb = cb[0], cb[1]
        s = S[0, :, 0:64] + S[1, :, 0:64]
        sh = S[0, :, 64:128] + S[1, :, 64:128]
        h = sh / (s + 1e-6)
        xn = x + _silu(_lnorm(su + h, lng, lnb))
        outs = [xn]
        ws = cb[2:]
        res = []
        for t in range(0, len(ws), 2):
            res.append(_linblk(xn, ws[t], ws[t + 1]))
        if len(res) == 4:
            outs.append(jnp.concatenate([res[0], res[1]], axis=1))
            outs.append(jnp.concatenate([res[2], jnp.zeros_like(res[2])], axis=1))
            outs.append(res[3])
        else:
            outs.extend(res)
        return outs

    def edge_update_body(ib, cb):
        (m, y) = ib
        lng, lnb = cb[0], cb[1]
        yn = y + _silu(_lnorm(m, lng, lnb))
        outs = [yn]
        ws = cb[2:]
        res = []
        for t in range(0, len(ws), 2):
            res.append(_linblk(yn, ws[t], ws[t + 1]))
        if len(res) == 4:
            outs.append(jnp.concatenate([res[0], res[1]], axis=1))
            outs.append(res[2])
            outs.append(res[3])
        else:
            outs.extend(res)
        return outs

    def eggw(q):
        return [_w(q["src_gate"]), _b2(q["src_gate"]),
                _w(q["dst_update"]), _b2(q["dst_update"]),
                _w(q["dst_gate"]), _b2(q["dst_gate"]),
                _w(q["src_update"]), _b2(q["src_update"])]

    def tri_update_body(ib, cb):
        (ylg, su, Sp) = ib
        lng, lnb = cb[0], cb[1]
        s = Sp[:, 0:64]
        sh = Sp[:, 64:128]
        h = sh / (s + 1e-6)
        yn = ylg + _silu(_lnorm(su + h, lng, lnb))
        outs = [yn]
        ws = cb[2:]
        for t in range(0, len(ws), 2):
            outs.append(_linblk(yn, ws[t], ws[t + 1]))
        return outs

    M1, S1 = _sc_node_egg(TS1, TD1, G1, src_p, dst_p)
    x1, TS2, TD2, SU2 = _tcmap(
        "t4a", NPAD, 1264, [x0, SU1, S1],
        [_g2(al0n["ln_nodes"]), _bb2(al0n["ln_nodes"])] + eggw(al1n),
        [64, 128, 128, 64], node_update_body)
    ylg1, TSp1, TDp1, SUp1 = _tcmap(
        "t5a", EP, 4096, [M1, y0],
        [_g2(al0n["ln_edges"]), _bb2(al0n["ln_edges"])] + eggw(al0e),
        [64, 128, 64, 64], edge_update_body)
    if dense3:
        Sp1, Mp1 = _sc_edge_egg_dense(TSp1, TDp1, gp1, lgs_p, True, TPad)
    else:
        TDp1g = jnp.pad(TDp1, ((0, 0), (0, 64)))
        Q1, Mp1 = _sc_edge_egg(TSp1, TDp1g, gp1, lgs_p, lgd_p, TPad, True, qrows)
        Sp1 = _sc_gather3(Q1, *idx3)
    y1, G2 = _tcmap(
        "t6a", EP, 4096, [ylg1, SUp1, Sp1],
        [_g2(al0e["ln_nodes"]), _bb2(al0e["ln_nodes"]),
         _w(al1n["edge_gate"]), _b2(al1n["edge_gate"])],
        [64, 64], tri_update_body)

    def zup_body(ib, cb):
        (m, z) = ib
        lng, lnb, weg, beg = cb
        zn = z + _silu(_lnorm(m, lng, lnb))
        return (_linblk(zn, weg, beg),)

    (gp2,) = _tcmap(
        "t5z", TPad, 4096, [Mp1, z0],
        [_g2(al0e["ln_edges"]), _bb2(al0e["ln_edges"]),
         _w(al1e["edge_gate"]), _b2(al1e["edge_gate"])],
        [64], zup_body)

    M2, S2 = _sc_node_egg(TS2, TD2, G2, src_p, dst_p)
    x2, TS3, TD3, SU3 = _tcmap(
        "t4b", NPAD, 1264, [x1, SU2, S2],
        [_g2(al1n["ln_nodes"]), _bb2(al1n["ln_nodes"])] + eggw(gc0),
        [64, 128, 128, 64], node_update_body)
    ylg2, TSp2, TDp2, SUp2 = _tcmap(
        "t5b", EP, 4096, [M2, y1],
        [_g2(al1n["ln_edges"]), _bb2(al1n["ln_edges"])] + eggw(al1e),
        [64, 128, 64, 64], edge_update_body)
    if dense3:
        Sp2 = _sc_edge_egg_dense(TSp2, TDp2, gp2, lgs_p, False, TPad)
    else:
        TDp2g = jnp.pad(TDp2, ((0, 0), (0, 64)))
        Q2 = _sc_edge_egg(TSp2, TDp2g, gp2, lgs_p, lgd_p, TPad, False, qrows)
        Sp2 = _sc_gather3(Q2, *idx3)
    y2, G3 = _tcmap(
        "t6b", EP, 4096, [ylg2, SUp2, Sp2],
        [_g2(al1e["ln_nodes"]), _bb2(al1e["ln_nodes"]),
         _w(gc0["edge_gate"]), _b2(gc0["edge_gate"])],
        [64, 64], tri_update_body)

    M3, S3 = _sc_node_egg(TS3, TD3, G3, src_p, dst_p)

    def t4c_body(ib, cb):
        (x, su, S, esed) = ib
        lng, lnb, wsg, bsg, wdg, bdg = cb
        s = S[0, :, 0:64] + S[1, :, 0:64]
        sh = S[0, :, 64:128] + S[1, :, 64:128]
        h = sh / (s + 1e-6)
        xn = x + _silu(_lnorm(su + h, lng, lnb))
        a4 = _linblk(xn, wsg, bsg)
        d4 = _linblk(xn, wdg, bdg)
        zp = jnp.zeros((xn.shape[0], 60), _f32)
        ta = jnp.concatenate([a4, esed[:, 0:4], zp], axis=1)
        tdx = jnp.concatenate([d4, esed[:, 4:8], zp], axis=1)
        return ta, tdx

    TA, TDX = _tcmap(
        "t4c", NPAD, 1264, [x2, SU3, S3, ES],
        [_g2(gc0["ln_nodes"]), _bb2(gc0["ln_nodes"]),
         _w(gc1["src_gate"]), _b2(gc1["src_gate"]),
         _w(gc1["dst_gate"]), _b2(gc1["dst_gate"])],
        [128, 128], t4c_body)

    def t5c_body(ib, cb):
        (m, y) = ib
        lng, lnb, weg, beg = cb
        yn = y + _silu(_lnorm(m, lng, lnb))
        g4 = _linblk(yn, weg, beg)
        g4p = jnp.concatenate([g4, jnp.zeros_like(yn[:, 0:16])], axis=1)
        return yn, g4p

    y3, G4P = _tcmap(
        "t5c", EP, 4096, [M3, y2],
        [_g2(gc0["ln_edges"]), _bb2(gc0["ln_edges"]),
         _w(gc1["edge_gate"]), _b2(gc1["edge_gate"])],
        [64, 80], t5c_body)

    M4X = _sc_gather_add80(TA, TDX, G4P, src_p, dst_p)

    def t7(m4x_ref, y3_ref, ue_ref, lng_ref, lnb_ref, fw_ref, fb_ref, out_ref):
        i = pl.program_id(0)
        m4 = m4x_ref[:, 0:64]
        esd = m4x_ref[:, 64:68]
        y = y3_ref[...]
        yn = y + _silu(_lnorm(m4, lng_ref[...], lnb_ref[...]))
        bond = jax.nn.sigmoid(
            jnp.dot(yn, fw_ref[...], preferred_element_type=_f32)
            + fb_ref[...])
        pe = jnp.exp(esd)
        bl = ue_ref[:, 3:4]
        cutv = ue_ref[:, 4:5]
        f_rep = pe[:, 0:1] * jnp.exp(-pe[:, 1:2] * bl)
        f_att = pe[:, 2:3] * jnp.exp(-pe[:, 3:4] * bl)
        V = cutv * (f_rep - bond * f_att)
        rowid = i * 4096 + lax.broadcasted_iota(jnp.int32, (4096, 1), 0)
        V = jnp.where(rowid < E, V, 0.0)
        bs = jnp.sum(V)

        @pl.when(i == 0)
        def _():
            out_ref[...] = jnp.zeros((1, 1), _f32)
        out_ref[...] += jnp.reshape(bs, (1, 1))

    tot = pl.pallas_call(
        t7,
        grid=(EP // 4096,),
        in_specs=[pl.BlockSpec((4096, 80), lambda i: (i, 0)),
                  pl.BlockSpec((4096, 64), lambda i: (i, 0)),
                  pl.BlockSpec((4096, 16), lambda i: (i, 0)),
                  pl.BlockSpec((1, 64), lambda i: (0, 0)),
                  pl.BlockSpec((1, 64), lambda i: (0, 0)),
                  pl.BlockSpec((64, 1), lambda i: (0, 0)),
                  pl.BlockSpec((1, 1), lambda i: (0, 0))],
        out_specs=pl.BlockSpec((1, 1), lambda i: (0, 0)),
        out_shape=jax.ShapeDtypeStruct((1, 1), _f32),
    )(M4X, y3, ue16,
      _g2(gc1["ln_edges"]), _bb2(gc1["ln_edges"]),
      _w(p["fc"]), p["fc"]["b"].reshape(1, 1))

    return tot[0, 0] / np.float32(N)


# speedup vs baseline: 1.2867x; 1.0513x over previous
import functools

import jax
import jax.numpy as jnp
import numpy as np
from jax import lax
from jax.experimental import pallas as pl
from jax.experimental.pallas import tpu as pltpu
from jax.experimental.pallas import tpu_sc as plsc

N = 10000
E = 160000
H = 64
K = 3
EP = 163840
NPAD = 10112
NC = 2
NS = 16
NW = NC * NS
C = 128

_f32 = jnp.float32


def _tcmap(name, nrows, block, ins, consts, out_dims, body):
    grid = nrows // block
    in_specs = []
    for a in ins:
        if a.ndim == 3:
            in_specs.append(pl.BlockSpec((a.shape[0], block, a.shape[2]),
                                         lambda i: (0, i, 0)))
        else:
            rb = block * a.shape[0] // nrows
            in_specs.append(pl.BlockSpec((rb, a.shape[1]), lambda i: (i, 0)))
    for c in consts:
        in_specs.append(pl.BlockSpec(c.shape, lambda i: (0,) * c.ndim))
    out_specs = [pl.BlockSpec((block, d), lambda i: (i, 0)) for d in out_dims]
    nin, ncon = len(ins), len(consts)

    def kern(*refs):
        ib = [refs[i][...] for i in range(nin)]
        cb = [refs[nin + i][...] for i in range(ncon)]
        outs = body(ib, cb)
        for k, ob in enumerate(outs):
            refs[nin + ncon + k][...] = ob

    return pl.pallas_call(
        kern,
        grid=(grid,),
        in_specs=in_specs,
        out_specs=out_specs,
        out_shape=[jax.ShapeDtypeStruct((nrows, d), _f32) for d in out_dims],
    )(*ins, *consts)


def _silu(x):
    return x * jax.nn.sigmoid(x)


def _lnorm(x, g, b):
    mu = jnp.mean(x, axis=-1, keepdims=True)
    var = jnp.mean((x - mu) ** 2, axis=-1, keepdims=True)
    return g * (x - mu) / jnp.sqrt(var + 1e-5) + b


def _mlpblk(x, w, b, g, bb):
    return _silu(_lnorm(jnp.dot(x, w, preferred_element_type=_f32) + b, g, bb))


def _linblk(x, w, b):
    y = jnp.dot(x, w, preferred_element_type=_f32)
    return y if b is None else y + b


_MESH = plsc.VectorSubcoreMesh(core_axis_name="c", subcore_axis_name="s")


def _wid_base(rows_pw):
    c = lax.axis_index("c")
    s = lax.axis_index("s")
    return (s * NC + c) * rows_pw


def _sc_gather_pairs(table, idx_a, idx_b, nrows, ow):
    rows_pw = nrows // NW
    nchunk = rows_pw // C
    np16 = ow // 16

    @functools.partial(
        pl.kernel,
        out_type=[jax.ShapeDtypeStruct((nrows, ow), _f32),
                  jax.ShapeDtypeStruct((nrows, ow), _f32)],
        mesh=_MESH,
        scratch_types=[pltpu.VMEM((C,), jnp.int32),
                       pltpu.VMEM((C,), jnp.int32),
                       pltpu.VMEM((C, 128), _f32),
                       pltpu.VMEM((C, 128), _f32),
                       pltpu.VMEM((C, ow), _f32),
                       pltpu.VMEM((C, ow), _f32),
                       pltpu.SemaphoreType.DMA,
                       pltpu.SemaphoreType.DMA],
    )
    def k(tab, ia, ib, oa, ob, iva, ivb, ga, gb, pa, pb, sema, semb):
        base0 = _wid_base(rows_pw)

        def step(j, _):
            base = base0 + j * C
            pltpu.sync_copy(ia.at[pl.ds(base, C)], iva)
            pltpu.sync_copy(ib.at[pl.ds(base, C)], ivb)
            cpa = pltpu.async_copy(tab.at[iva], ga, sema)
            cpb = pltpu.async_copy(tab.at[ivb], gb, semb)
            cpa.wait()
            cpb.wait()

            @plsc.parallel_loop(0, C, 1, unroll=4)
            def row(i):
                for kk in range(np16):
                    sl = pl.ds(kk * 16, 16)
                    pa[i, sl] = ga[i, sl]
                    pb[i, sl] = gb[i, sl]
            pltpu.sync_copy(pa, oa.at[pl.ds(base, C)])
            pltpu.sync_copy(pb, ob.at[pl.ds(base, C)])
            return _

        lax.fori_loop(0, nchunk, step, None)

    return k(table, idx_a, idx_b)


def _sc_gather_one(table, idx, nrows, ow):
    rows_pw = nrows // NW
    nchunk = rows_pw // C
    np16 = ow // 16

    @functools.partial(
        pl.kernel,
        out_type=[jax.ShapeDtypeStruct((nrows, ow), _f32)],
        mesh=_MESH,
        scratch_types=[pltpu.VMEM((C,), jnp.int32),
                       pltpu.VMEM((C,), jnp.int32),
                       pltpu.VMEM((C, 128), _f32),
                       pltpu.VMEM((C, 128), _f32),
                       pltpu.VMEM((C, ow), _f32),
                       pltpu.SemaphoreType.DMA,
                       pltpu.SemaphoreType.DMA],
    )
    def k(tab, ia, oa, iva0, iva1, ga0, ga1, pa, sem0, sem1):
        base0 = _wid_base(rows_pw)
        iva = [iva0, iva1]
        ga = [ga0, ga1]
        sem = [sem0, sem1]

        def issue(j, b):
            base = base0 + j * C
            pltpu.sync_copy(ia.at[pl.ds(base, C)], iva[b])
            pltpu.async_copy(tab.at[iva[b]], ga[b], sem[b])

        def finish(j, b):
            pltpu.make_async_copy(tab.at[iva[b]], ga[b], sem[b]).wait()

            @plsc.parallel_loop(0, C, 1, unroll=4)
            def row(i):
                for kk in range(np16):
                    sl = pl.ds(kk * 16, 16)
                    pa[i, sl] = ga[b][i, sl]
            pltpu.sync_copy(pa, oa.at[pl.ds(base0 + j * C, C)])

        issue(0, 0)

        def step(p, _):
            j0 = 2 * p
            @pl.when(j0 + 1 < nchunk)
            def _():
                issue(j0 + 1, 1)
            finish(j0, 0)
            @pl.when(j0 + 2 < nchunk)
            def _():
                issue(j0 + 2, 0)
            @pl.when(j0 + 1 < nchunk)
            def _():
                finish(j0 + 1, 1)
            return _

        lax.fori_loop(0, (nchunk + 1) // 2, step, None)

    res = k(table, idx)
    return res[0] if isinstance(res, (list, tuple)) else res


def _sc_edge_egg_dense(ts, td64, g, lgs, write_m, tp3):
    CB = 64
    CT = 3 * CB
    rows_pw = tp3 // NW
    bonds_pw = rows_pw // 3
    nchunk = rows_pw // CT
    outs = [jax.ShapeDtypeStruct((EP, 128), _f32)]
    if write_m:
        outs.append(jax.ShapeDtypeStruct((tp3, 64), _f32))

    @functools.partial(
        pl.kernel,
        out_type=outs,
        mesh=_MESH,
        scratch_types=[pltpu.VMEM((CT,), jnp.int32),
                       pltpu.VMEM((CT,), jnp.int32),
                       pltpu.VMEM((CT, 128), _f32),
                       pltpu.VMEM((CT, 128), _f32),
                       pltpu.VMEM((CB, 64), _f32),
                       pltpu.VMEM((CB, 64), _f32),
                       pltpu.VMEM((CT, 64), _f32),
                       pltpu.VMEM((CT, 64), _f32),
                       pltpu.VMEM((CB, 128), _f32),
                       pltpu.SemaphoreType.DMA,
                       pltpu.SemaphoreType.DMA],
    )
    def k(tsr, tdr, gr, sr, so, *rest):
        if write_m:
            mo = rest[0]
            rest = rest[1:]
        else:
            mo = None
        (ivs0, ivs1, gs0, gs1, tdv0, tdv1, gv0, gv1, qs, sem0, sem1) = rest
        ivs = [ivs0, ivs1]
        gs = [gs0, gs1]
        tdv = [tdv0, tdv1]
        gv = [gv0, gv1]
        sem = [sem0, sem1]
        cid = lax.axis_index("c")
        sid = lax.axis_index("s")
        w = sid * NC + cid
        tbase0 = w * rows_pw
        bbase0 = w * bonds_pw

        def issue(j, b):
            tbase = tbase0 + j * CT
            bbase = bbase0 + j * CB
            pltpu.sync_copy(sr.at[pl.ds(tbase, CT)], ivs[b])
            pltpu.async_copy(tsr.at[ivs[b].at[pl.ds(0, C)]],
                             gs[b].at[pl.ds(0, C)], sem[b])
            pltpu.async_copy(tsr.at[ivs[b].at[pl.ds(C, CT - C)]],
                             gs[b].at[pl.ds(C, CT - C)], sem[b])
            pltpu.async_copy(tdr.at[pl.ds(bbase, CB)], tdv[b], sem[b])
            pltpu.async_copy(gr.at[pl.ds(tbase, CT)], gv[b], sem[b])

        def finish(j, b):
            tbase = tbase0 + j * CT
            bbase = bbase0 + j * CB
            pltpu.make_async_copy(tsr.at[ivs[b].at[pl.ds(0, C)]],
                                  gs[b].at[pl.ds(0, C)], sem[b]).wait()
            pltpu.make_async_copy(tsr.at[ivs[b].at[pl.ds(C, CT - C)]],
                                  gs[b].at[pl.ds(C, CT - C)], sem[b]).wait()
            pltpu.make_async_copy(tdr.at[pl.ds(bbase, CB)], tdv[b], sem[b]).wait()
            pltpu.make_async_copy(gr.at[pl.ds(tbase, CT)], gv[b], sem[b]).wait()
            gsb, tdb, gvb = gs[b], tdv[b], gv[b]

            @plsc.parallel_loop(0, CB, 1, unroll=2)
            def bond(bb):
                for kk in range(4):
                    sl = pl.ds(kk * 16, 16)
                    sl2 = pl.ds(64 + kk * 16, 16)
                    d = tdb[bb, sl]
                    ssum = jnp.zeros((16,), _f32)
                    shsum = jnp.zeros((16,), _f32)
                    for q in range(3):
                        i = bb * 3 + q
                        m = gsb[i, sl] + d + gvb[i, sl]
                        if write_m:
                            gvb[i, sl] = m
                        sig = 1.0 / (1.0 + jnp.exp(-m))
                        ssum = ssum + sig
                        shsum = shsum + sig * gsb[i, sl2]
                    qs[bb, sl] = ssum
                    qs[bb, sl2] = shsum
            if write_m:
                pltpu.sync_copy(gvb, mo.at[pl.ds(tbase, CT)])
            pltpu.sync_copy(qs, so.at[pl.ds(bbase, CB)])

        issue(0, 0)

        def step(p, _):
            j0 = 2 * p
            @pl.when(j0 + 1 < nchunk)
            def _():
                issue(j0 + 1, 1)
            finish(j0, 0)
            @pl.when(j0 + 2 < nchunk)
            def _():
                issue(j0 + 2, 0)
            @pl.when(j0 + 1 < nchunk)
            def _():
                finish(j0 + 1, 1)
            return _

        lax.fori_loop(0, (nchunk + 1) // 2, step, None)

    res = k(ts, td64, g, lgs)
    if write_m:
        return res[0], res[1]
    return res[0] if isinstance(res, (list, tuple)) else res


def _sc_node_egg(ts, td, g, src, dst):
    CN = 64
    rows_pw = EP // NW
    nchunk = rows_pw // CN
    zrows = NPAD // NS

    @functools.partial(
        pl.kernel,
        out_type=[jax.ShapeDtypeStruct((EP, 64), _f32),
                  jax.ShapeDtypeStruct((NC, NPAD, 128), _f32)],
        mesh=_MESH,
        scratch_types=[pltpu.VMEM((CN,), jnp.int32),
                       pltpu.VMEM((CN,), jnp.int32),
                       pltpu.VMEM((CN,), jnp.int32),
                       pltpu.VMEM((CN,), jnp.int32),
                       pltpu.VMEM((CN, 128), _f32),
                       pltpu.VMEM((CN, 128), _f32),
                       pltpu.VMEM((CN, 128), _f32),
                       pltpu.VMEM((CN, 128), _f32),
                       pltpu.VMEM((CN, 64), _f32),
                       pltpu.VMEM((CN, 64), _f32),
                       pltpu.VMEM_SHARED((NPAD, 128), _f32),
                       pltpu.SemaphoreType.DMA,
                       pltpu.SemaphoreType.DMA],
    )
    def k(tsr, tdr, gr, sr, dr, mo, so, ivs0, ivs1, ivd0, ivd1,
          gs0, gs1, gd0, gd1, gv0, gv1, acc, sem0, sem1):
        ivs = [ivs0, ivs1]
        ivd = [ivd0, ivd1]
        gs = [gs0, gs1]
        gd = [gd0, gd1]
        gv = [gv0, gv1]
        sem = [sem0, sem1]
        cid = lax.axis_index("c")
        sid = lax.axis_index("s")
        base0 = (sid * NC + cid) * rows_pw

        @plsc.parallel_loop(0, CN, 1, unroll=4)
        def zrow(i):
            for kk in range(8):
                gs0[i, pl.ds(kk * 16, 16)] = jnp.zeros((16,), _f32)
        for t in range(zrows // CN):
            pltpu.sync_copy(gs0, acc.at[pl.ds(sid * zrows + t * CN, CN)])
        rem = zrows - (zrows // CN) * CN
        pltpu.sync_copy(gs0.at[pl.ds(0, rem)],
                        acc.at[pl.ds(sid * zrows + zrows - rem, rem)])
        plsc.subcore_barrier()

        def issue(j, b):
            base = base0 + j * CN
            pltpu.sync_copy(sr.at[pl.ds(base, CN)], ivs[b])
            pltpu.sync_copy(dr.at[pl.ds(base, CN)], ivd[b])
            pltpu.async_copy(tsr.at[ivs[b]], gs[b], sem[b])
            pltpu.async_copy(tdr.at[ivd[b]], gd[b], sem[b])
            pltpu.async_copy(gr.at[pl.ds(base, CN)], gv[b], sem[b])

        def finish(j, b):
            base = base0 + j * CN
            pltpu.make_async_copy(tsr.at[ivs[b]], gs[b], sem[b]).wait()
            pltpu.make_async_copy(tdr.at[ivd[b]], gd[b], sem[b]).wait()
            pltpu.make_async_copy(gr.at[pl.ds(base, CN)], gv[b], sem[b]).wait()
            gsb, gdb, gvb = gs[b], gd[b], gv[b]

            @plsc.parallel_loop(0, CN, 1, unroll=2)
            def row(i):
                for kk in range(4):
                    a = gsb[i, pl.ds(kk * 16, 16)]
                    d = gdb[i, pl.ds(kk * 16, 16)]
                    gg = gvb[i, pl.ds(kk * 16, 16)]
                    m = a + d + gg
                    gvb[i, pl.ds(kk * 16, 16)] = m
                    sig = 1.0 / (1.0 + jnp.exp(-m))
                    gdb[i, pl.ds(kk * 16, 16)] = sig
                    bh = gsb[i, pl.ds(64 + kk * 16, 16)]
                    gdb[i, pl.ds(64 + kk * 16, 16)] = sig * bh
            pltpu.sync_copy(gvb, mo.at[pl.ds(base, CN)])
            pltpu.sync_copy(gdb, acc.at[ivd[b]], add=True)

        issue(0, 0)

        def step(p, _):
            j0 = 2 * p
            @pl.when(j0 + 1 < nchunk)
            def _():
                issue(j0 + 1, 1)
            finish(j0, 0)
            @pl.when(j0 + 2 < nchunk)
            def _():
                issue(j0 + 2, 0)
            @pl.when(j0 + 1 < nchunk)
            def _():
                finish(j0 + 1, 1)
            return _

        lax.fori_loop(0, (nchunk + 1) // 2, step, None)
        plsc.subcore_barrier()
        pltpu.sync_copy(acc.at[pl.ds(sid * zrows, zrows)],
                        so.at[cid, pl.ds(sid * zrows, zrows)])

    return k(ts, td, g, src, dst)


def _sc_edge_egg(ts, td, g, lgs, lgd, tp, write_m, qrows):
    rows_pw = tp // NW
    nchunk = rows_pw // C
    outs = [jax.ShapeDtypeStruct((qrows, 128), _f32)]
    if write_m:
        outs.append(jax.ShapeDtypeStruct((tp, 64), _f32))

    @functools.partial(
        pl.kernel,
        out_type=outs,
        mesh=_MESH,
        scratch_types=[pltpu.VMEM((C,), jnp.int32),
                       pltpu.VMEM((C,), jnp.int32),
                       pltpu.VMEM((C, 128), _f32),
                       pltpu.VMEM((C, 128), _f32),
                       pltpu.VMEM((C, 64), _f32),
                       pltpu.VMEM((C, 64), _f32),
                       pltpu.VMEM((C, 128), _f32),
                       pltpu.SemaphoreType.DMA,
                       pltpu.SemaphoreType.DMA],
    )
    def k(tsr, tdr, gr, sr, dr, qo, *rest):
        if write_m:
            mo = rest[0]
            ivs, ivd, gs, gd, gv, mv, qv, sema, semb = rest[1:]
        else:
            mo = None
            ivs, ivd, gs, gd, gv, mv, qv, sema, semb = rest
        cid = lax.axis_index("c")
        sid = lax.axis_index("s")
        base0 = (sid * NC + cid) * rows_pw

        @pl.when(jnp.logical_and(cid == 0, sid == 0))
        def _():
            def zrow(i, _):
                for kk in range(8):
                    qv[i, pl.ds(kk * 16, 16)] = jnp.zeros((16,), _f32)
                return _
            lax.fori_loop(0, C, zrow, None)
            pltpu.sync_copy(qv, qo.at[pl.ds(tp, C)])

        def step(j, _):
            base = base0 + j * C
            pltpu.sync_copy(sr.at[pl.ds(base, C)], ivs)
            pltpu.sync_copy(dr.at[pl.ds(base, C)], ivd)
            cpa = pltpu.async_copy(tsr.at[ivs], gs, sema)
            cpb = pltpu.async_copy(tdr.at[ivd], gd, semb)
            pltpu.sync_copy(gr.at[pl.ds(base, C)], gv)
            cpa.wait()
            cpb.wait()

            def row(i, _):
                for kk in range(4):
                    a = gs[i, pl.ds(kk * 16, 16)]
                    d = gd[i, pl.ds(kk * 16, 16)]
                    gg = gv[i, pl.ds(kk * 16, 16)]
                    m = a + d + gg
                    if write_m:
                        mv[i, pl.ds(kk * 16, 16)] = m
                    sig = 1.0 / (1.0 + jnp.exp(-m))
                    qv[i, pl.ds(kk * 16, 16)] = sig
                    bh = gs[i, pl.ds(64 + kk * 16, 16)]
                    qv[i, pl.ds(64 + kk * 16, 16)] = sig * bh
                return _
            lax.fori_loop(0, C, row, None)
            if write_m:
                pltpu.sync_copy(mv, mo.at[pl.ds(base, C)])
            pltpu.sync_copy(qv, qo.at[pl.ds(base, C)])
            return _

        lax.fori_loop(0, nchunk, step, None)

    res = k(ts, td, g, lgs, lgd)
    if write_m:
        return res[0], res[1]
    return res[0] if isinstance(res, (list, tuple)) else res


def _sc_gather3(q, i0, i1, i2):
    rows_pw = EP // NW
    nchunk = rows_pw // C

    @functools.partial(
        pl.kernel,
        out_type=[jax.ShapeDtypeStruct((EP, 128), _f32)],
        mesh=_MESH,
        scratch_types=[pltpu.VMEM((C,), jnp.int32),
                       pltpu.VMEM((C,), jnp.int32),
                       pltpu.VMEM((C,), jnp.int32),
                       pltpu.VMEM((C, 128), _f32),
                       pltpu.VMEM((C, 128), _f32),
                       pltpu.VMEM((C, 128), _f32),
                       pltpu.VMEM((C, 128), _f32),
                       pltpu.SemaphoreType.DMA,
                       pltpu.SemaphoreType.DMA,
                       pltpu.SemaphoreType.DMA],
    )
    def k(qr, r0, r1, r2, so, v0, v1, v2, g0, g1, g2, ov, s0, s1, s2):
        base0 = _wid_base(rows_pw)

        def step(j, _):
            base = base0 + j * C
            pltpu.sync_copy(r0.at[pl.ds(base, C)], v0)
            pltpu.sync_copy(r1.at[pl.ds(base, C)], v1)
            pltpu.sync_copy(r2.at[pl.ds(base, C)], v2)
            c0 = pltpu.async_copy(qr.at[v0], g0, s0)
            c1 = pltpu.async_copy(qr.at[v1], g1, s1)
            c2 = pltpu.async_copy(qr.at[v2], g2, s2)
            c0.wait()
            c1.wait()
            c2.wait()

            def row(i, _):
                for kk in range(8):
                    sl = pl.ds(kk * 16, 16)
                    ov[i, sl] = g0[i, sl] + g1[i, sl] + g2[i, sl]
                return _
            lax.fori_loop(0, C, row, None)
            pltpu.sync_copy(ov, so.at[pl.ds(base, C)])
            return _

        lax.fori_loop(0, nchunk, step, None)

    res = k(q, i0, i1, i2)
    return res[0] if isinstance(res, (list, tuple)) else res


def _sc_gather_add80(ta, tdx, g, src, dst):
    rows_pw = EP // NW
    nchunk = rows_pw // C

    @functools.partial(
        pl.kernel,
        out_type=[jax.ShapeDtypeStruct((EP, 80), _f32)],
        mesh=_MESH,
        scratch_types=[pltpu.VMEM((C,), jnp.int32),
                       pltpu.VMEM((C,), jnp.int32),
                       pltpu.VMEM((C, 128), _f32),
                       pltpu.VMEM((C, 128), _f32),
                       pltpu.VMEM((C, 80), _f32),
                       pltpu.VMEM((C, 80), _f32),
                       pltpu.SemaphoreType.DMA,
                       pltpu.SemaphoreType.DMA],
    )
    def k(tar, tdr, gr, sr, dr, mo, ivs, ivd, ga, gd, gv, mv, sema, semb):
        base0 = _wid_base(rows_pw)

        def step(j, _):
            base = base0 + j * C
            pltpu.sync_copy(sr.at[pl.ds(base, C)], ivs)
            pltpu.sync_copy(dr.at[pl.ds(base, C)], ivd)
            cpa = pltpu.async_copy(tar.at[ivs], ga, sema)
            cpb = pltpu.async_copy(tdr.at[ivd], gd, semb)
            pltpu.sync_copy(gr.at[pl.ds(base, C)], gv)
            cpa.wait()
            cpb.wait()

            @plsc.parallel_loop(0, C, 1, unroll=4)
            def row(i):
                for kk in range(5):
                    sl = pl.ds(kk * 16, 16)
                    mv[i, sl] = ga[i, sl] + gd[i, sl] + gv[i, sl]
            pltpu.sync_copy(mv, mo.at[pl.ds(base, C)])
            return _

        lax.fori_loop(0, nchunk, step, None)

    res = k(ta, tdx, g, src, dst)
    return res[0] if isinstance(res, (list, tuple)) else res


def _w(p):
    return p["w"]


def _b2(p):
    return p["b"].reshape(1, -1)


def _g2(p):
    return p["g"].reshape(1, -1)


def _bb2(p):
    return p["b"].reshape(1, -1)


def kernel(r, params, atom_numbers, edge_index, lg_src, lg_dst):
    T = lg_dst.shape[0]
    dense3 = (T == 3 * E)
    TPad = 3 * EP if dense3 else ((T + 4095) // 4096) * 4096

    src = edge_index[0].astype(jnp.int32)
    dst = edge_index[1].astype(jnp.int32)
    lgs = lg_src.astype(jnp.int32)
    lgd = lg_dst.astype(jnp.int32)

    src_p = jnp.concatenate([src, jnp.full((EP - E,), N, jnp.int32)])
    dst_p = jnp.concatenate([dst, jnp.full((EP - E,), N, jnp.int32)])
    lgs_p = jnp.concatenate([lgs, jnp.full((TPad - T,), E, jnp.int32)])
    lgd_p = jnp.concatenate([lgd, jnp.full((TPad - T,), E, jnp.int32)])
    if dense3:
        qrows = None
        idx3 = None
    else:
        qrows = TPad + C
        se = jnp.searchsorted(lgd, jnp.arange(E + 1, dtype=jnp.int32)).astype(jnp.int32)
        s_p = jnp.concatenate([se[:E], jnp.zeros((EP - E,), jnp.int32)])
        e_p = jnp.concatenate([se[1:], jnp.zeros((EP - E,), jnp.int32)])
        idx3 = [jnp.where(s_p + i < e_p, s_p + i, TPad).astype(jnp.int32)
                for i in range(K)]

    r16 = jnp.zeros((EP, 16), _f32).at[:E, 0:3].set(r.astype(_f32))
    an2 = jnp.concatenate([atom_numbers.astype(jnp.int32),
                           jnp.zeros((NPAD - N,), jnp.int32)]).reshape(NPAD, 1)

    p = params
    al0n, al0e = p["alignn"][0]["node"], p["alignn"][0]["edge"]
    al1n, al1e = p["alignn"][1]["node"], p["alignn"][1]["edge"]
    gc0, gc1 = p["gcn"][0], p["gcn"][1]

    cent_e = jnp.linspace(0.0, 8.0, 80).astype(_f32).reshape(1, 80)
    cent_a = jnp.linspace(-1.0, 1.0, 40).astype(_f32).reshape(1, 40)
    gam_e = float(79.0 / 8.0)
    gam_a = 19.5

    def t1_body(ib, cb):
        (X,) = ib
        (ce, w1, b1, g1_, bb1, w2, b2_, g2_, bb2, weg, beg) = cb
        bl2 = jnp.sum(X * X, axis=-1, keepdims=True)
        bl = jnp.sqrt(bl2)
        inv = 1.0 / jnp.maximum(bl, 1e-9)
        u = X * inv
        cutv = jnp.where(bl < 3.8, 1.0, 0.5 - 0.5 * jnp.sin(np.pi * (bl - 3.9) / 0.2))
        cutv = jnp.where(bl > 4.0, 0.0, cutv)
        rb = jnp.exp(-gam_e * (bl - ce) ** 2)
        y0 = _mlpblk(rb, w1, b1, g1_, bb1)
        y0 = _mlpblk(y0, w2, b2_, g2_, bb2)
        G1 = _linblk(y0, weg, beg)
        zpad = jnp.zeros((X.shape[0], 123), _f32)
        u_ext = jnp.concatenate([u[:, 0:3], bl, cutv, zpad], axis=1)
        ue16 = u_ext[:, 0:16]
        return y0, G1, u_ext, ue16

    y0, G1, u_ext, ue16 = _tcmap(
        "t1", EP, 4096, [r16],
        [cent_e,
         _w(p["edge_mlp1"]["lin"]), _b2(p["edge_mlp1"]["lin"]),
         _g2(p["edge_mlp1"]["ln"]), _bb2(p["edge_mlp1"]["ln"]),
         _w(p["edge_mlp2"]["lin"]), _b2(p["edge_mlp2"]["lin"]),
         _g2(p["edge_mlp2"]["ln"]), _bb2(p["edge_mlp2"]["ln"]),
         _w(al0n["edge_gate"]), _b2(al0n["edge_gate"])],
        [64, 64, 128, 16], t1_body)

    if dense3:
        uu1 = _sc_gather_one(u_ext, lgs_p, TPad, 16)
        uu2 = None
    else:
        uu1, uu2 = _sc_gather_pairs(u_ext, lgs_p, lgd_p, TPad, 16)

    def t2_body(ib, cb):
        (U1, U2) = ib
        (ca, w1, b1, g1_, bb1, w2, b2_, g2_, bb2, weg, beg) = cb
        cos = -jnp.sum(U1[:, 0:3] * U2[:, 0:3], axis=-1, keepdims=True)
        cos = jnp.clip(cos, -1.0, 1.0)
        rb = jnp.exp(-gam_a * (cos - ca) ** 2)
        z0 = _mlpblk(rb, w1, b1, g1_, bb1)
        z0 = _mlpblk(z0, w2, b2_, g2_, bb2)
        gp = _linblk(z0, weg, beg)
        return z0, gp

    def t2d_body(ib, cb):
        (U1, UE) = ib
        u2 = jnp.repeat(UE, 3, axis=0)
        return t2_body([U1, u2], cb)

    t2_ins = [uu1, ue16] if dense3 else [uu1, uu2]
    t2_block = 3072 if dense3 else 4096
    z0, gp1 = _tcmap(
        "t2", TPad, t2_block, t2_ins,
        [cent_a,
         _w(p["angle_mlp1"]["lin"]), _b2(p["angle_mlp1"]["lin"]),
         _g2(p["angle_mlp1"]["ln"]), _bb2(p["angle_mlp1"]["ln"]),
         _w(p["angle_mlp2"]["lin"]), _b2(p["angle_mlp2"]["lin"]),
         _g2(p["angle_mlp2"]["ln"]), _bb2(p["angle_mlp2"]["ln"]),
         _w(al0e["edge_gate"]), _b2(al0e["edge_gate"])],
        [64, 64], t2d_body if dense3 else t2_body)

    def t3_body(ib, cb):
        (an,) = ib
        (emb, wsg, bsg, wdu, bdu, wdg, bdg, wsu, bsu, wes, bes, wed) = cb
        onehot = (lax.broadcasted_iota(jnp.int32, (an.shape[0], 128), 1)
                  == an).astype(_f32)
        x0 = jnp.dot(onehot, emb, preferred_element_type=_f32)
        ts = jnp.concatenate([_linblk(x0, wsg, bsg), _linblk(x0, wdu, bdu)], axis=1)
        td = jnp.concatenate([_linblk(x0, wdg, bdg),
                              jnp.zeros((x0.shape[0], 64), _f32)], axis=1)
        su = _linblk(x0, wsu, bsu)
        es = _linblk(x0, wes, bes)
        ed = jnp.dot(x0, wed, preferred_element_type=_f32)
        zpad = jnp.zeros_like(es) * 0.0
        esed = jnp.concatenate([es, ed, zpad, zpad], axis=1)
        return x0, ts, td, su, esed

    x0, TS1, TD1, SU1, ES = _tcmap(
        "t3", NPAD, 1264, [an2],
        [p["atom_embedding"],
         _w(al0n["src_gate"]), _b2(al0n["src_gate"]),
         _w(al0n["dst_update"]), _b2(al0n["dst_update"]),
         _w(al0n["dst_gate"]), _b2(al0n["dst_gate"]),
         _w(al0n["src_update"]), _b2(al0n["src_update"]),
         _w(p["int_src"]), _b2(p["int_src"]),
         _w(p["int_dst"])],
        [64, 128, 128, 64, 16], t3_body)

    def node_update_body(ib, cb):
        (x, su, S) = ib
        lng, lnb = cb[0], cb[1]
        s = S[0, :, 0:64] + S[1, :, 0:64]
        sh = S[0, :, 64:128] + S[1, :, 64:128]
        h = sh / (s + 1e-6)
        xn = x + _silu(_lnorm(su + h, lng, lnb))
        outs = [xn]
        ws = cb[2:]
        res = []
        for t in range(0, len(ws), 2):
            res.append(_linblk(xn, ws[t], ws[t + 1]))
        if len(res) == 4:
            outs.append(jnp.concatenate([res[0], res[1]], axis=1))
            outs.append(jnp.concatenate([res[2], jnp.zeros_like(res[2])], axis=1))
            outs.append(res[3])
        else:
            outs.extend(res)
        return outs

    def edge_update_body(ib, cb):
        (m, y) = ib
        lng, lnb = cb[0], cb[1]
        yn = y + _silu(_lnorm(m, lng, lnb))
        outs = [yn]
        ws = cb[2:]
        res = []
        for t in range(0, len(ws), 2):
            res.append(_linblk(yn, ws[t], ws[t + 1]))
        if len(res) == 4:
            outs.append(jnp.concatenate([res[0], res[1]], axis=1))
            outs.append(res[2])
            outs.append(res[3])
        else:
            outs.extend(res)
        return outs

    def eggw(q):
        return [_w(q["src_gate"]), _b2(q["src_gate"]),
                _w(q["dst_update"]), _b2(q["dst_update"]),
                _w(q["dst_gate"]), _b2(q["dst_gate"]),
                _w(q["src_update"]), _b2(q["src_update"])]

    def tri_update_body(ib, cb):
        (ylg, su, Sp) = ib
        lng, lnb = cb[0], cb[1]
        s = Sp[:, 0:64]
        sh = Sp[:, 64:128]
        h = sh / (s + 1e-6)
        yn = ylg + _silu(_lnorm(su + h, lng, lnb))
        outs = [yn]
        ws = cb[2:]
        for t in range(0, len(ws), 2):
            outs.append(_linblk(yn, ws[t], ws[t + 1]))
        return outs

    M1, S1 = _sc_node_egg(TS1, TD1, G1, src_p, dst_p)
    x1, TS2, TD2, SU2 = _tcmap(
        "t4a", NPAD, 1264, [x0, SU1, S1],
        [_g2(al0n["ln_nodes"]), _bb2(al0n["ln_nodes"])] + eggw(al1n),
        [64, 128, 128, 64], node_update_body)
    ylg1, TSp1, TDp1, SUp1 = _tcmap(
        "t5a", EP, 4096, [M1, y0],
        [_g2(al0n["ln_edges"]), _bb2(al0n["ln_edges"])] + eggw(al0e),
        [64, 128, 64, 64], edge_update_body)
    if dense3:
        Sp1, Mp1 = _sc_edge_egg_dense(TSp1, TDp1, gp1, lgs_p, True, TPad)
    else:
        TDp1g = jnp.pad(TDp1, ((0, 0), (0, 64)))
        Q1, Mp1 = _sc_edge_egg(TSp1, TDp1g, gp1, lgs_p, lgd_p, TPad, True, qrows)
        Sp1 = _sc_gather3(Q1, *idx3)
    y1, G2 = _tcmap(
        "t6a", EP, 4096, [ylg1, SUp1, Sp1],
        [_g2(al0e["ln_nodes"]), _bb2(al0e["ln_nodes"]),
         _w(al1n["edge_gate"]), _b2(al1n["edge_gate"])],
        [64, 64], tri_update_body)

    def zup_body(ib, cb):
        (m, z) = ib
        lng, lnb, weg, beg = cb
        zn = z + _silu(_lnorm(m, lng, lnb))
        return (_linblk(zn, weg, beg),)

    (gp2,) = _tcmap(
        "t5z", TPad, 4096, [Mp1, z0],
        [_g2(al0e["ln_edges"]), _bb2(al0e["ln_edges"]),
         _w(al1e["edge_gate"]), _b2(al1e["edge_gate"])],
        [64], zup_body)

    M2, S2 = _sc_node_egg(TS2, TD2, G2, src_p, dst_p)
    x2, TS3, TD3, SU3 = _tcmap(
        "t4b", NPAD, 1264, [x1, SU2, S2],
        [_g2(al1n["ln_nodes"]), _bb2(al1n["ln_nodes"])] + eggw(gc0),
        [64, 128, 128, 64], node_update_body)
    ylg2, TSp2, TDp2, SUp2 = _tcmap(
        "t5b", EP, 4096, [M2, y1],
        [_g2(al1n["ln_edges"]), _bb2(al1n["ln_edges"])] + eggw(al1e),
        [64, 128, 64, 64], edge_update_body)
    if dense3:
        Sp2 = _sc_edge_egg_dense(TSp2, TDp2, gp2, lgs_p, False, TPad)
    else:
        TDp2g = jnp.pad(TDp2, ((0, 0), (0, 64)))
        Q2 = _sc_edge_egg(TSp2, TDp2g, gp2, lgs_p, lgd_p, TPad, False, qrows)
        Sp2 = _sc_gather3(Q2, *idx3)
    y2, G3 = _tcmap(
        "t6b", EP, 4096, [ylg2, SUp2, Sp2],
        [_g2(al1e["ln_nodes"]), _bb2(al1e["ln_nodes"]),
         _w(gc0["edge_gate"]), _b2(gc0["edge_gate"])],
        [64, 64], tri_update_body)

    M3, S3 = _sc_node_egg(TS3, TD3, G3, src_p, dst_p)

    def t4c_body(ib, cb):
        (x, su, S, esed) = ib
        lng, lnb, wsg, bsg, wdg, bdg = cb
        s = S[0, :, 0:64] + S[1, :, 0:64]
        sh = S[0, :, 64:128] + S[1, :, 64:128]
        h = sh / (s + 1e-6)
        xn = x + _silu(_lnorm(su + h, lng, lnb))
        a4 = _linblk(xn, wsg, bsg)
        d4 = _linblk(xn, wdg, bdg)
        zp = jnp.zeros((xn.shape[0], 60), _f32)
        ta = jnp.concatenate([a4, esed[:, 0:4], zp], axis=1)
        tdx = jnp.concatenate([d4, esed[:, 4:8], zp], axis=1)
        return ta, tdx

    TA, TDX = _tcmap(
        "t4c", NPAD, 1264, [x2, SU3, S3, ES],
        [_g2(gc0["ln_nodes"]), _bb2(gc0["ln_nodes"]),
         _w(gc1["src_gate"]), _b2(gc1["src_gate"]),
         _w(gc1["dst_gate"]), _b2(gc1["dst_gate"])],
        [128, 128], t4c_body)

    def t5c_body(ib, cb):
        (m, y) = ib
        lng, lnb, weg, beg = cb
        yn = y + _silu(_lnorm(m, lng, lnb))
        g4 = _linblk(yn, weg, beg)
        g4p = jnp.concatenate([g4, jnp.zeros_like(yn[:, 0:16])], axis=1)
        return yn, g4p

    y3, G4P = _tcmap(
        "t5c", EP, 4096, [M3, y2],
        [_g2(gc0["ln_edges"]), _bb2(gc0["ln_edges"]),
         _w(gc1["edge_gate"]), _b2(gc1["edge_gate"])],
        [64, 80], t5c_body)

    M4X = _sc_gather_add80(TA, TDX, G4P, src_p, dst_p)

    def t7(m4x_ref, y3_ref, ue_ref, lng_ref, lnb_ref, fw_ref, fb_ref, out_ref):
        i = pl.program_id(0)
        m4 = m4x_ref[:, 0:64]
        esd = m4x_ref[:, 64:68]
        y = y3_ref[...]
        yn = y + _silu(_lnorm(m4, lng_ref[...], lnb_ref[...]))
        bond = jax.nn.sigmoid(
            jnp.dot(yn, fw_ref[...], preferred_element_type=_f32)
            + fb_ref[...])
        pe = jnp.exp(esd)
        bl = ue_ref[:, 3:4]
        cutv = ue_ref[:, 4:5]
        f_rep = pe[:, 0:1] * jnp.exp(-pe[:, 1:2] * bl)
        f_att = pe[:, 2:3] * jnp.exp(-pe[:, 3:4] * bl)
        V = cutv * (f_rep - bond * f_att)
        rowid = i * 4096 + lax.broadcasted_iota(jnp.int32, (4096, 1), 0)
        V = jnp.where(rowid < E, V, 0.0)
        bs = jnp.sum(V)

        @pl.when(i == 0)
        def _():
            out_ref[...] = jnp.zeros((1, 1), _f32)
        out_ref[...] += jnp.reshape(bs, (1, 1))

    tot = pl.pallas_call(
        t7,
        grid=(EP // 4096,),
        in_specs=[pl.BlockSpec((4096, 80), lambda i: (i, 0)),
                  pl.BlockSpec((4096, 64), lambda i: (i, 0)),
                  pl.BlockSpec((4096, 16), lambda i: (i, 0)),
                  pl.BlockSpec((1, 64), lambda i: (0, 0)),
                  pl.BlockSpec((1, 64), lambda i: (0, 0)),
                  pl.BlockSpec((64, 1), lambda i: (0, 0)),
                  pl.BlockSpec((1, 1), lambda i: (0, 0))],
        out_specs=pl.BlockSpec((1, 1), lambda i: (0, 0)),
        out_shape=jax.ShapeDtypeStruct((1, 1), _f32),
    )(M4X, y3, ue16,
      _g2(gc1["ln_edges"]), _bb2(gc1["ln_edges"]),
      _w(p["fc"]), p["fc"]["b"].reshape(1, 1))

    return tot[0, 0] / np.float32(N)


# speedup vs baseline: 1.2873x; 1.0004x over previous
import functools

import jax
import jax.numpy as jnp
import numpy as np
from jax import lax
from jax.experimental import pallas as pl
from jax.experimental.pallas import tpu as pltpu
from jax.experimental.pallas import tpu_sc as plsc

N = 10000
E = 160000
H = 64
K = 3
EP = 163840
NPAD = 10112
NC = 2
NS = 16
NW = NC * NS
C = 128

_f32 = jnp.float32


def _tcmap(name, nrows, block, ins, consts, out_dims, body):
    grid = nrows // block
    in_specs = []
    for a in ins:
        if a.ndim == 3:
            in_specs.append(pl.BlockSpec((a.shape[0], block, a.shape[2]),
                                         lambda i: (0, i, 0)))
        else:
            rb = block * a.shape[0] // nrows
            in_specs.append(pl.BlockSpec((rb, a.shape[1]), lambda i: (i, 0)))
    for c in consts:
        in_specs.append(pl.BlockSpec(c.shape, lambda i: (0,) * c.ndim))
    out_specs = [pl.BlockSpec((block, d), lambda i: (i, 0)) for d in out_dims]
    nin, ncon = len(ins), len(consts)

    def kern(*refs):
        ib = [refs[i][...] for i in range(nin)]
        cb = [refs[nin + i][...] for i in range(ncon)]
        outs = body(ib, cb)
        for k, ob in enumerate(outs):
            refs[nin + ncon + k][...] = ob

    return pl.pallas_call(
        kern,
        grid=(grid,),
        in_specs=in_specs,
        out_specs=out_specs,
        out_shape=[jax.ShapeDtypeStruct((nrows, d), _f32) for d in out_dims],
    )(*ins, *consts)


def _silu(x):
    return x * jax.nn.sigmoid(x)


def _lnorm(x, g, b):
    mu = jnp.mean(x, axis=-1, keepdims=True)
    var = jnp.mean((x - mu) ** 2, axis=-1, keepdims=True)
    return g * (x - mu) / jnp.sqrt(var + 1e-5) + b


def _mlpblk(x, w, b, g, bb):
    return _silu(_lnorm(jnp.dot(x, w, preferred_element_type=_f32) + b, g, bb))


def _linblk(x, w, b):
    y = jnp.dot(x, w, preferred_element_type=_f32)
    return y if b is None else y + b


_MESH = plsc.VectorSubcoreMesh(core_axis_name="c", subcore_axis_name="s")


def _wid_base(rows_pw):
    c = lax.axis_index("c")
    s = lax.axis_index("s")
    return (s * NC + c) * rows_pw


def _sc_gather_pairs(table, idx_a, idx_b, nrows, ow):
    rows_pw = nrows // NW
    nchunk = rows_pw // C
    np16 = ow // 16

    @functools.partial(
        pl.kernel,
        out_type=[jax.ShapeDtypeStruct((nrows, ow), _f32),
                  jax.ShapeDtypeStruct((nrows, ow), _f32)],
        mesh=_MESH,
        scratch_types=[pltpu.VMEM((C,), jnp.int32),
                       pltpu.VMEM((C,), jnp.int32),
                       pltpu.VMEM((C, 128), _f32),
                       pltpu.VMEM((C, 128), _f32),
                       pltpu.VMEM((C, ow), _f32),
                       pltpu.VMEM((C, ow), _f32),
                       pltpu.SemaphoreType.DMA,
                       pltpu.SemaphoreType.DMA],
    )
    def k(tab, ia, ib, oa, ob, iva, ivb, ga, gb, pa, pb, sema, semb):
        base0 = _wid_base(rows_pw)

        def step(j, _):
            base = base0 + j * C
            pltpu.sync_copy(ia.at[pl.ds(base, C)], iva)
            pltpu.sync_copy(ib.at[pl.ds(base, C)], ivb)
            cpa = pltpu.async_copy(tab.at[iva], ga, sema)
            cpb = pltpu.async_copy(tab.at[ivb], gb, semb)
            cpa.wait()
            cpb.wait()

            @plsc.parallel_loop(0, C, 1, unroll=4)
            def row(i):
                for kk in range(np16):
                    sl = pl.ds(kk * 16, 16)
                    pa[i, sl] = ga[i, sl]
                    pb[i, sl] = gb[i, sl]
            pltpu.sync_copy(pa, oa.at[pl.ds(base, C)])
            pltpu.sync_copy(pb, ob.at[pl.ds(base, C)])
            return _

        lax.fori_loop(0, nchunk, step, None)

    return k(table, idx_a, idx_b)


def _sc_gather_one(table, idx, nrows, ow):
    rows_pw = nrows // NW
    nchunk = rows_pw // C
    np16 = ow // 16

    @functools.partial(
        pl.kernel,
        out_type=[jax.ShapeDtypeStruct((nrows, ow), _f32)],
        mesh=_MESH,
        scratch_types=[pltpu.VMEM((C,), jnp.int32),
                       pltpu.VMEM((C,), jnp.int32),
                       pltpu.VMEM((C, 128), _f32),
                       pltpu.VMEM((C, 128), _f32),
                       pltpu.VMEM((C, ow), _f32),
                       pltpu.SemaphoreType.DMA,
                       pltpu.SemaphoreType.DMA],
    )
    def k(tab, ia, oa, iva0, iva1, ga0, ga1, pa, sem0, sem1):
        base0 = _wid_base(rows_pw)
        iva = [iva0, iva1]
        ga = [ga0, ga1]
        sem = [sem0, sem1]

        def issue(j, b):
            base = base0 + j * C
            pltpu.sync_copy(ia.at[pl.ds(base, C)], iva[b])
            pltpu.async_copy(tab.at[iva[b]], ga[b], sem[b])

        def finish(j, b):
            pltpu.make_async_copy(tab.at[iva[b]], ga[b], sem[b]).wait()

            @plsc.parallel_loop(0, C, 1, unroll=4)
            def row(i):
                for kk in range(np16):
                    sl = pl.ds(kk * 16, 16)
                    pa[i, sl] = ga[b][i, sl]
            pltpu.sync_copy(pa, oa.at[pl.ds(base0 + j * C, C)])

        issue(0, 0)

        def step(p, _):
            j0 = 2 * p
            @pl.when(j0 + 1 < nchunk)
            def _():
                issue(j0 + 1, 1)
            finish(j0, 0)
            @pl.when(j0 + 2 < nchunk)
            def _():
                issue(j0 + 2, 0)
            @pl.when(j0 + 1 < nchunk)
            def _():
                finish(j0 + 1, 1)
            return _

        lax.fori_loop(0, (nchunk + 1) // 2, step, None)

    res = k(table, idx)
    return res[0] if isinstance(res, (list, tuple)) else res


def _sc_edge_egg_dense(ts, td64, g, lgs, write_m, tp3):
    CB = 64
    CT = 3 * CB
    rows_pw = tp3 // NW
    bonds_pw = rows_pw // 3
    nchunk = rows_pw // CT
    outs = [jax.ShapeDtypeStruct((EP, 128), _f32)]
    if write_m:
        outs.append(jax.ShapeDtypeStruct((tp3, 64), _f32))

    @functools.partial(
        pl.kernel,
        out_type=outs,
        mesh=_MESH,
        scratch_types=[pltpu.VMEM((CT,), jnp.int32),
                       pltpu.VMEM((CT,), jnp.int32),
                       pltpu.VMEM((CT, 128), _f32),
                       pltpu.VMEM((CT, 128), _f32),
                       pltpu.VMEM((CB, 64), _f32),
                       pltpu.VMEM((CB, 64), _f32),
                       pltpu.VMEM((CT, 64), _f32),
                       pltpu.VMEM((CT, 64), _f32),
                       pltpu.VMEM((CB, 128), _f32),
                       pltpu.SemaphoreType.DMA,
                       pltpu.SemaphoreType.DMA],
    )
    def k(tsr, tdr, gr, sr, so, *rest):
        if write_m:
            mo = rest[0]
            rest = rest[1:]
        else:
            mo = None
        (ivs0, ivs1, gs0, gs1, tdv0, tdv1, gv0, gv1, qs, sem0, sem1) = rest
        ivs = [ivs0, ivs1]
        gs = [gs0, gs1]
        tdv = [tdv0, tdv1]
        gv = [gv0, gv1]
        sem = [sem0, sem1]
        cid = lax.axis_index("c")
        sid = lax.axis_index("s")
        w = sid * NC + cid
        tbase0 = w * rows_pw
        bbase0 = w * bonds_pw

        def issue(j, b):
            tbase = tbase0 + j * CT
            bbase = bbase0 + j * CB
            pltpu.sync_copy(sr.at[pl.ds(tbase, CT)], ivs[b])
            pltpu.async_copy(tsr.at[ivs[b].at[pl.ds(0, C)]],
                             gs[b].at[pl.ds(0, C)], sem[b])
            pltpu.async_copy(tsr.at[ivs[b].at[pl.ds(C, CT - C)]],
                             gs[b].at[pl.ds(C, CT - C)], sem[b])
            pltpu.async_copy(tdr.at[pl.ds(bbase, CB)], tdv[b], sem[b])
            pltpu.async_copy(gr.at[pl.ds(tbase, CT)], gv[b], sem[b])

        def finish(j, b):
            tbase = tbase0 + j * CT
            bbase = bbase0 + j * CB
            pltpu.make_async_copy(tsr.at[ivs[b].at[pl.ds(0, C)]],
                                  gs[b].at[pl.ds(0, C)], sem[b]).wait()
            pltpu.make_async_copy(tsr.at[ivs[b].at[pl.ds(C, CT - C)]],
                                  gs[b].at[pl.ds(C, CT - C)], sem[b]).wait()
            pltpu.make_async_copy(tdr.at[pl.ds(bbase, CB)], tdv[b], sem[b]).wait()
            pltpu.make_async_copy(gr.at[pl.ds(tbase, CT)], gv[b], sem[b]).wait()
            gsb, tdb, gvb = gs[b], tdv[b], gv[b]

            @plsc.parallel_loop(0, CB, 1, unroll=2)
            def bond(bb):
                for kk in range(4):
                    sl = pl.ds(kk * 16, 16)
                    sl2 = pl.ds(64 + kk * 16, 16)
                    d = tdb[bb, sl]
                    ssum = jnp.zeros((16,), _f32)
                    shsum = jnp.zeros((16,), _f32)
                    for q in range(3):
                        i = bb * 3 + q
                        m = gsb[i, sl] + d + gvb[i, sl]
                        if write_m:
                            gvb[i, sl] = m
                        sig = 1.0 / (1.0 + jnp.exp(-m))
                        ssum = ssum + sig
                        shsum = shsum + sig * gsb[i, sl2]
                    qs[bb, sl] = ssum
                    qs[bb, sl2] = shsum
            if write_m:
                pltpu.sync_copy(gvb, mo.at[pl.ds(tbase, CT)])
            pltpu.sync_copy(qs, so.at[pl.ds(bbase, CB)])

        issue(0, 0)

        def step(p, _):
            j0 = 2 * p
            @pl.when(j0 + 1 < nchunk)
            def _():
                issue(j0 + 1, 1)
            finish(j0, 0)
            @pl.when(j0 + 2 < nchunk)
            def _():
                issue(j0 + 2, 0)
            @pl.when(j0 + 1 < nchunk)
            def _():
                finish(j0 + 1, 1)
            return _

        lax.fori_loop(0, (nchunk + 1) // 2, step, None)

    res = k(ts, td64, g, lgs)
    if write_m:
        return res[0], res[1]
    return res[0] if isinstance(res, (list, tuple)) else res


def _sc_node_egg(ts, td, g, src, dst):
    CN = 64
    rows_pw = EP // NW
    nchunk = rows_pw // CN
    zrows = NPAD // NS

    @functools.partial(
        pl.kernel,
        out_type=[jax.ShapeDtypeStruct((EP, 64), _f32),
                  jax.ShapeDtypeStruct((NC, NPAD, 128), _f32)],
        mesh=_MESH,
        scratch_types=[pltpu.VMEM((CN,), jnp.int32),
                       pltpu.VMEM((CN,), jnp.int32),
                       pltpu.VMEM((CN,), jnp.int32),
                       pltpu.VMEM((CN,), jnp.int32),
                       pltpu.VMEM((CN, 128), _f32),
                       pltpu.VMEM((CN, 128), _f32),
                       pltpu.VMEM((CN, 128), _f32),
                       pltpu.VMEM((CN, 128), _f32),
                       pltpu.VMEM((CN, 64), _f32),
                       pltpu.VMEM((CN, 64), _f32),
                       pltpu.VMEM_SHARED((NPAD, 128), _f32),
                       pltpu.SemaphoreType.DMA,
                       pltpu.SemaphoreType.DMA],
    )
    def k(tsr, tdr, gr, sr, dr, mo, so, ivs0, ivs1, ivd0, ivd1,
          gs0, gs1, gd0, gd1, gv0, gv1, acc, sem0, sem1):
        ivs = [ivs0, ivs1]
        ivd = [ivd0, ivd1]
        gs = [gs0, gs1]
        gd = [gd0, gd1]
        gv = [gv0, gv1]
        sem = [sem0, sem1]
        cid = lax.axis_index("c")
        sid = lax.axis_index("s")
        base0 = (sid * NC + cid) * rows_pw

        @plsc.parallel_loop(0, CN, 1, unroll=4)
        def zrow(i):
            for kk in range(8):
                gs0[i, pl.ds(kk * 16, 16)] = jnp.zeros((16,), _f32)
        for t in range(zrows // CN):
            pltpu.sync_copy(gs0, acc.at[pl.ds(sid * zrows + t * CN, CN)])
        rem = zrows - (zrows // CN) * CN
        pltpu.sync_copy(gs0.at[pl.ds(0, rem)],
                        acc.at[pl.ds(sid * zrows + zrows - rem, rem)])
        plsc.subcore_barrier()

        def issue(j, b):
            base = base0 + j * CN
            pltpu.sync_copy(sr.at[pl.ds(base, CN)], ivs[b])
            pltpu.sync_copy(dr.at[pl.ds(base, CN)], ivd[b])
            pltpu.async_copy(tsr.at[ivs[b]], gs[b], sem[b])
            pltpu.async_copy(tdr.at[ivd[b]], gd[b], sem[b])
            pltpu.async_copy(gr.at[pl.ds(base, CN)], gv[b], sem[b])

        def finish(j, b):
            base = base0 + j * CN
            pltpu.make_async_copy(tsr.at[ivs[b]], gs[b], sem[b]).wait()
            pltpu.make_async_copy(tdr.at[ivd[b]], gd[b], sem[b]).wait()
            pltpu.make_async_copy(gr.at[pl.ds(base, CN)], gv[b], sem[b]).wait()
            gsb, gdb, gvb = gs[b], gd[b], gv[b]

            @plsc.parallel_loop(0, CN, 1, unroll=2)
            def row(i):
                for kk in range(4):
                    a = gsb[i, pl.ds(kk * 16, 16)]
                    d = gdb[i, pl.ds(kk * 16, 16)]
                    gg = gvb[i, pl.ds(kk * 16, 16)]
                    m = a + d + gg
                    gvb[i, pl.ds(kk * 16, 16)] = m
                    sig = 1.0 / (1.0 + jnp.exp(-m))
                    gdb[i, pl.ds(kk * 16, 16)] = sig
                    bh = gsb[i, pl.ds(64 + kk * 16, 16)]
                    gdb[i, pl.ds(64 + kk * 16, 16)] = sig * bh
            pltpu.sync_copy(gvb, mo.at[pl.ds(base, CN)])
            pltpu.sync_copy(gdb, acc.at[ivd[b]], add=True)

        issue(0, 0)

        def step(p, _):
            j0 = 2 * p
            @pl.when(j0 + 1 < nchunk)
            def _():
                issue(j0 + 1, 1)
            finish(j0, 0)
            @pl.when(j0 + 2 < nchunk)
            def _():
                issue(j0 + 2, 0)
            @pl.when(j0 + 1 < nchunk)
            def _():
                finish(j0 + 1, 1)
            return _

        lax.fori_loop(0, (nchunk + 1) // 2, step, None)
        plsc.subcore_barrier()
        pltpu.sync_copy(acc.at[pl.ds(sid * zrows, zrows)],
                        so.at[cid, pl.ds(sid * zrows, zrows)])

    return k(ts, td, g, src, dst)


def _sc_edge_egg(ts, td, g, lgs, lgd, tp, write_m, qrows):
    rows_pw = tp // NW
    nchunk = rows_pw // C
    outs = [jax.ShapeDtypeStruct((qrows, 128), _f32)]
    if write_m:
        outs.append(jax.ShapeDtypeStruct((tp, 64), _f32))

    @functools.partial(
        pl.kernel,
        out_type=outs,
        mesh=_MESH,
        scratch_types=[pltpu.VMEM((C,), jnp.int32),
                       pltpu.VMEM((C,), jnp.int32),
                       pltpu.VMEM((C, 128), _f32),
                       pltpu.VMEM((C, 128), _f32),
                       pltpu.VMEM((C, 64), _f32),
                       pltpu.VMEM((C, 64), _f32),
                       pltpu.VMEM((C, 128), _f32),
                       pltpu.SemaphoreType.DMA,
                       pltpu.SemaphoreType.DMA],
    )
    def k(tsr, tdr, gr, sr, dr, qo, *rest):
        if write_m:
            mo = rest[0]
            ivs, ivd, gs, gd, gv, mv, qv, sema, semb = rest[1:]
        else:
            mo = None
            ivs, ivd, gs, gd, gv, mv, qv, sema, semb = rest
        cid = lax.axis_index("c")
        sid = lax.axis_index("s")
        base0 = (sid * NC + cid) * rows_pw

        @pl.when(jnp.logical_and(cid == 0, sid == 0))
        def _():
            def zrow(i, _):
                for kk in range(8):
                    qv[i, pl.ds(kk * 16, 16)] = jnp.zeros((16,), _f32)
                return _
            lax.fori_loop(0, C, zrow, None)
            pltpu.sync_copy(qv, qo.at[pl.ds(tp, C)])

        def step(j, _):
            base = base0 + j * C
            pltpu.sync_copy(sr.at[pl.ds(base, C)], ivs)
            pltpu.sync_copy(dr.at[pl.ds(base, C)], ivd)
            cpa = pltpu.async_copy(tsr.at[ivs], gs, sema)
            cpb = pltpu.async_copy(tdr.at[ivd], gd, semb)
            pltpu.sync_copy(gr.at[pl.ds(base, C)], gv)
            cpa.wait()
            cpb.wait()

            def row(i, _):
                for kk in range(4):
                    a = gs[i, pl.ds(kk * 16, 16)]
                    d = gd[i, pl.ds(kk * 16, 16)]
                    gg = gv[i, pl.ds(kk * 16, 16)]
                    m = a + d + gg
                    if write_m:
                        mv[i, pl.ds(kk * 16, 16)] = m
                    sig = 1.0 / (1.0 + jnp.exp(-m))
                    qv[i, pl.ds(kk * 16, 16)] = sig
                    bh = gs[i, pl.ds(64 + kk * 16, 16)]
                    qv[i, pl.ds(64 + kk * 16, 16)] = sig * bh
                return _
            lax.fori_loop(0, C, row, None)
            if write_m:
                pltpu.sync_copy(mv, mo.at[pl.ds(base, C)])
            pltpu.sync_copy(qv, qo.at[pl.ds(base, C)])
            return _

        lax.fori_loop(0, nchunk, step, None)

    res = k(ts, td, g, lgs, lgd)
    if write_m:
        return res[0], res[1]
    return res[0] if isinstance(res, (list, tuple)) else res


def _sc_gather3(q, i0, i1, i2):
    rows_pw = EP // NW
    nchunk = rows_pw // C

    @functools.partial(
        pl.kernel,
        out_type=[jax.ShapeDtypeStruct((EP, 128), _f32)],
        mesh=_MESH,
        scratch_types=[pltpu.VMEM((C,), jnp.int32),
                       pltpu.VMEM((C,), jnp.int32),
                       pltpu.VMEM((C,), jnp.int32),
                       pltpu.VMEM((C, 128), _f32),
                       pltpu.VMEM((C, 128), _f32),
                       pltpu.VMEM((C, 128), _f32),
                       pltpu.VMEM((C, 128), _f32),
                       pltpu.SemaphoreType.DMA,
                       pltpu.SemaphoreType.DMA,
                       pltpu.SemaphoreType.DMA],
    )
    def k(qr, r0, r1, r2, so, v0, v1, v2, g0, g1, g2, ov, s0, s1, s2):
        base0 = _wid_base(rows_pw)

        def step(j, _):
            base = base0 + j * C
            pltpu.sync_copy(r0.at[pl.ds(base, C)], v0)
            pltpu.sync_copy(r1.at[pl.ds(base, C)], v1)
            pltpu.sync_copy(r2.at[pl.ds(base, C)], v2)
            c0 = pltpu.async_copy(qr.at[v0], g0, s0)
            c1 = pltpu.async_copy(qr.at[v1], g1, s1)
            c2 = pltpu.async_copy(qr.at[v2], g2, s2)
            c0.wait()
            c1.wait()
            c2.wait()

            def row(i, _):
                for kk in range(8):
                    sl = pl.ds(kk * 16, 16)
                    ov[i, sl] = g0[i, sl] + g1[i, sl] + g2[i, sl]
                return _
            lax.fori_loop(0, C, row, None)
            pltpu.sync_copy(ov, so.at[pl.ds(base, C)])
            return _

        lax.fori_loop(0, nchunk, step, None)

    res = k(q, i0, i1, i2)
    return res[0] if isinstance(res, (list, tuple)) else res


def _sc_gather_add80(ta, tdx, g, src, dst):
    rows_pw = EP // NW
    nchunk = rows_pw // C

    @functools.partial(
        pl.kernel,
        out_type=[jax.ShapeDtypeStruct((EP, 80), _f32)],
        mesh=_MESH,
        scratch_types=[pltpu.VMEM((C,), jnp.int32),
                       pltpu.VMEM((C,), jnp.int32),
                       pltpu.VMEM((C, 128), _f32),
                       pltpu.VMEM((C, 128), _f32),
                       pltpu.VMEM((C, 80), _f32),
                       pltpu.VMEM((C, 80), _f32),
                       pltpu.SemaphoreType.DMA,
                       pltpu.SemaphoreType.DMA],
    )
    def k(tar, tdr, gr, sr, dr, mo, ivs, ivd, ga, gd, gv, mv, sema, semb):
        base0 = _wid_base(rows_pw)

        def step(j, _):
            base = base0 + j * C
            pltpu.sync_copy(sr.at[pl.ds(base, C)], ivs)
            pltpu.sync_copy(dr.at[pl.ds(base, C)], ivd)
            cpa = pltpu.async_copy(tar.at[ivs], ga, sema)
            cpb = pltpu.async_copy(tdr.at[ivd], gd, semb)
            pltpu.sync_copy(gr.at[pl.ds(base, C)], gv)
            cpa.wait()
            cpb.wait()

            @plsc.parallel_loop(0, C, 1, unroll=4)
            def row(i):
                for kk in range(5):
                    sl = pl.ds(kk * 16, 16)
                    mv[i, sl] = ga[i, sl] + gd[i, sl] + gv[i, sl]
            pltpu.sync_copy(mv, mo.at[pl.ds(base, C)])
            return _

        lax.fori_loop(0, nchunk, step, None)

    res = k(ta, tdx, g, src, dst)
    return res[0] if isinstance(res, (list, tuple)) else res


def _w(p):
    return p["w"]


def _b2(p):
    return p["b"].reshape(1, -1)


def _g2(p):
    return p["g"].reshape(1, -1)


def _bb2(p):
    return p["b"].reshape(1, -1)


def kernel(r, params, atom_numbers, edge_index, lg_src, lg_dst):
    T = lg_dst.shape[0]
    dense3 = (T == 3 * E)
    TPad = 3 * EP if dense3 else ((T + 4095) // 4096) * 4096

    src = edge_index[0].astype(jnp.int32)
    dst = edge_index[1].astype(jnp.int32)
    lgs = lg_src.astype(jnp.int32)
    lgd = lg_dst.astype(jnp.int32)

    src_p = jnp.concatenate([src, jnp.full((EP - E,), N, jnp.int32)])
    dst_p = jnp.concatenate([dst, jnp.full((EP - E,), N, jnp.int32)])
    lgs_p = jnp.concatenate([lgs, jnp.full((TPad - T,), E, jnp.int32)])
    lgd_p = jnp.concatenate([lgd, jnp.full((TPad - T,), E, jnp.int32)])
    if dense3:
        qrows = None
        idx3 = None
    else:
        qrows = TPad + C
        se = jnp.searchsorted(lgd, jnp.arange(E + 1, dtype=jnp.int32)).astype(jnp.int32)
        s_p = jnp.concatenate([se[:E], jnp.zeros((EP - E,), jnp.int32)])
        e_p = jnp.concatenate([se[1:], jnp.zeros((EP - E,), jnp.int32)])
        idx3 = [jnp.where(s_p + i < e_p, s_p + i, TPad).astype(jnp.int32)
                for i in range(K)]

    r16 = jnp.zeros((EP, 16), _f32).at[:E, 0:3].set(r.astype(_f32))
    an2 = jnp.concatenate([atom_numbers.astype(jnp.int32),
                           jnp.zeros((NPAD - N,), jnp.int32)]).reshape(NPAD, 1)

    p = params
    al0n, al0e = p["alignn"][0]["node"], p["alignn"][0]["edge"]
    al1n, al1e = p["alignn"][1]["node"], p["alignn"][1]["edge"]
    gc0, gc1 = p["gcn"][0], p["gcn"][1]

    cent_e = jnp.linspace(0.0, 8.0, 80).astype(_f32).reshape(1, 80)
    cent_a = jnp.linspace(-1.0, 1.0, 40).astype(_f32).reshape(1, 40)
    gam_e = float(79.0 / 8.0)
    gam_a = 19.5

    def t1_body(ib, cb):
        (X,) = ib
        (ce, w1, b1, g1_, bb1, w2, b2_, g2_, bb2, weg, beg) = cb
        bl2 = jnp.sum(X * X, axis=-1, keepdims=True)
        bl = jnp.sqrt(bl2)
        inv = 1.0 / jnp.maximum(bl, 1e-9)
        u = X * inv
        cutv = jnp.where(bl < 3.8, 1.0, 0.5 - 0.5 * jnp.sin(np.pi * (bl - 3.9) / 0.2))
        cutv = jnp.where(bl > 4.0, 0.0, cutv)
        rb = jnp.exp(-gam_e * (bl - ce) ** 2)
        y0 = _mlpblk(rb, w1, b1, g1_, bb1)
        y0 = _mlpblk(y0, w2, b2_, g2_, bb2)
        G1 = _linblk(y0, weg, beg)
        zpad = jnp.zeros((X.shape[0], 123), _f32)
        u_ext = jnp.concatenate([u[:, 0:3], bl, cutv, zpad], axis=1)
        ue16 = u_ext[:, 0:16]
        return y0, G1, u_ext, ue16

    y0, G1, u_ext, ue16 = _tcmap(
        "t1", EP, 4096, [r16],
        [cent_e,
         _w(p["edge_mlp1"]["lin"]), _b2(p["edge_mlp1"]["lin"]),
         _g2(p["edge_mlp1"]["ln"]), _bb2(p["edge_mlp1"]["ln"]),
         _w(p["edge_mlp2"]["lin"]), _b2(p["edge_mlp2"]["lin"]),
         _g2(p["edge_mlp2"]["ln"]), _bb2(p["edge_mlp2"]["ln"]),
         _w(al0n["edge_gate"]), _b2(al0n["edge_gate"])],
        [64, 64, 128, 16], t1_body)

    if dense3:
        uu1 = _sc_gather_one(u_ext, lgs_p, TPad, 16)
        uu2 = None
    else:
        uu1, uu2 = _sc_gather_pairs(u_ext, lgs_p, lgd_p, TPad, 16)

    def t2_body(ib, cb):
        (U1, U2) = ib
        (ca, w1, b1, g1_, bb1, w2, b2_, g2_, bb2, weg, beg) = cb
        cos = -jnp.sum(U1[:, 0:3] * U2[:, 0:3], axis=-1, keepdims=True)
        cos = jnp.clip(cos, -1.0, 1.0)
        rb = jnp.exp(-gam_a * (cos - ca) ** 2)
        z0 = _mlpblk(rb, w1, b1, g1_, bb1)
        z0 = _mlpblk(z0, w2, b2_, g2_, bb2)
        gp = _linblk(z0, weg, beg)
        return z0, gp

    def t2d_body(ib, cb):
        (U1, UE) = ib
        u2 = jnp.repeat(UE, 3, axis=0)
        return t2_body([U1, u2], cb)

    t2_ins = [uu1, ue16] if dense3 else [uu1, uu2]
    t2_block = 3072 if dense3 else 4096
    z0, gp1 = _tcmap(
        "t2", TPad, t2_block, t2_ins,
        [cent_a,
         _w(p["angle_mlp1"]["lin"]), _b2(p["angle_mlp1"]["lin"]),
         _g2(p["angle_mlp1"]["ln"]), _bb2(p["angle_mlp1"]["ln"]),
         _w(p["angle_mlp2"]["lin"]), _b2(p["angle_mlp2"]["lin"]),
         _g2(p["angle_mlp2"]["ln"]), _bb2(p["angle_mlp2"]["ln"]),
         _w(al0e["edge_gate"]), _b2(al0e["edge_gate"])],
        [64, 64], t2d_body if dense3 else t2_body)

    def t3_body(ib, cb):
        (an,) = ib
        (emb, wsg, bsg, wdu, bdu, wdg, bdg, wsu, bsu, wes, bes, wed) = cb
        onehot = (lax.broadcasted_iota(jnp.int32, (an.shape[0], 128), 1)
                  == an).astype(_f32)
        x0 = jnp.dot(onehot, emb, preferred_element_type=_f32)
        wcat = jnp.concatenate([wsg, wdu, wdg, wsu, wes, wed], axis=1)
        bcat = jnp.concatenate(
            [bsg, bdu, bdg, bsu, bes, jnp.zeros((1, 4), _f32)], axis=1)
        rr = jnp.dot(x0, wcat, preferred_element_type=_f32) + bcat
        ts = rr[:, 0:128]
        td = jnp.concatenate([rr[:, 128:192],
                              jnp.zeros((x0.shape[0], 64), _f32)], axis=1)
        su = rr[:, 192:256]
        esed = jnp.concatenate([rr[:, 256:264],
                                jnp.zeros((x0.shape[0], 8), _f32)], axis=1)
        return x0, ts, td, su, esed

    x0, TS1, TD1, SU1, ES = _tcmap(
        "t3", NPAD, 1264, [an2],
        [p["atom_embedding"],
         _w(al0n["src_gate"]), _b2(al0n["src_gate"]),
         _w(al0n["dst_update"]), _b2(al0n["dst_update"]),
         _w(al0n["dst_gate"]), _b2(al0n["dst_gate"]),
         _w(al0n["src_update"]), _b2(al0n["src_update"]),
         _w(p["int_src"]), _b2(p["int_src"]),
         _w(p["int_dst"])],
        [64, 128, 128, 64, 16], t3_body)

    def node_update_body(ib, cb):
        (x, su, S) = ib
        lng, lnb = cb[0], cb[1]
        s = S[0, :, 0:64] + S[1, :, 0:64]
        sh = S[0, :, 64:128] + S[1, :, 64:128]
        h = sh / (s + 1e-6)
        xn = x + _silu(_lnorm(su + h, lng, lnb))
        outs = [xn]
        ws = cb[2:]
        wcat = jnp.concatenate(ws[0::2], axis=1)
        bcat = jnp.concatenate(ws[1::2], axis=1)
        rr = jnp.dot(xn, wcat, preferred_element_type=_f32) + bcat
        nres = len(ws) // 2
        res = [rr[:, 64 * t:64 * (t + 1)] for t in range(nres)]
        if nres == 4:
            outs.append(jnp.concatenate([res[0], res[1]], axis=1))
            outs.append(jnp.concatenate([res[2], jnp.zeros_like(res[2])], axis=1))
            outs.append(res[3])
        else:
            outs.extend(res)
        return outs

    def edge_update_body(ib, cb):
        (m, y) = ib
        lng, lnb = cb[0], cb[1]
        yn = y + _silu(_lnorm(m, lng, lnb))
        outs = [yn]
        ws = cb[2:]
        wcat = jnp.concatenate(ws[0::2], axis=1)
        bcat = jnp.concatenate(ws[1::2], axis=1)
        rr = jnp.dot(yn, wcat, preferred_element_type=_f32) + bcat
        nres = len(ws) // 2
        res = [rr[:, 64 * t:64 * (t + 1)] for t in range(nres)]
        if nres == 4:
            outs.append(jnp.concatenate([res[0], res[1]], axis=1))
            outs.append(res[2])
            outs.append(res[3])
        else:
            outs.extend(res)
        return outs

    def eggw(q):
        return [_w(q["src_gate"]), _b2(q["src_gate"]),
                _w(q["dst_update"]), _b2(q["dst_update"]),
                _w(q["dst_gate"]), _b2(q["dst_gate"]),
                _w(q["src_update"]), _b2(q["src_update"])]

    def tri_update_body(ib, cb):
        (ylg, su, Sp) = ib
        lng, lnb = cb[0], cb[1]
        s = Sp[:, 0:64]
        sh = Sp[:, 64:128]
        h = sh / (s + 1e-6)
        yn = ylg + _silu(_lnorm(su + h, lng, lnb))
        outs = [yn]
        ws = cb[2:]
        for t in range(0, len(ws), 2):
            outs.append(_linblk(yn, ws[t], ws[t + 1]))
        return outs

    M1, S1 = _sc_node_egg(TS1, TD1, G1, src_p, dst_p)
    x1, TS2, TD2, SU2 = _tcmap(
        "t4a", NPAD, 1264, [x0, SU1, S1],
        [_g2(al0n["ln_nodes"]), _bb2(al0n["ln_nodes"])] + eggw(al1n),
        [64, 128, 128, 64], node_update_body)
    ylg1, TSp1, TDp1, SUp1 = _tcmap(
        "t5a", EP, 4096, [M1, y0],
        [_g2(al0n["ln_edges"]), _bb2(al0n["ln_edges"])] + eggw(al0e),
        [64, 128, 64, 64], edge_update_body)
    if dense3:
        Sp1, Mp1 = _sc_edge_egg_dense(TSp1, TDp1, gp1, lgs_p, True, TPad)
    else:
        TDp1g = jnp.pad(TDp1, ((0, 0), (0, 64)))
        Q1, Mp1 = _sc_edge_egg(TSp1, TDp1g, gp1, lgs_p, lgd_p, TPad, True, qrows)
        Sp1 = _sc_gather3(Q1, *idx3)
    y1, G2 = _tcmap(
        "t6a", EP, 4096, [ylg1, SUp1, Sp1],
        [_g2(al0e["ln_nodes"]), _bb2(al0e["ln_nodes"]),
         _w(al1n["edge_gate"]), _b2(al1n["edge_gate"])],
        [64, 64], tri_update_body)

    def zup_body(ib, cb):
        (m, z) = ib
        lng, lnb, weg, beg = cb
        zn = z + _silu(_lnorm(m, lng, lnb))
        return (_linblk(zn, weg, beg),)

    (gp2,) = _tcmap(
        "t5z", TPad, 4096, [Mp1, z0],
        [_g2(al0e["ln_edges"]), _bb2(al0e["ln_edges"]),
         _w(al1e["edge_gate"]), _b2(al1e["edge_gate"])],
        [64], zup_body)

    M2, S2 = _sc_node_egg(TS2, TD2, G2, src_p, dst_p)
    x2, TS3, TD3, SU3 = _tcmap(
        "t4b", NPAD, 1264, [x1, SU2, S2],
        [_g2(al1n["ln_nodes"]), _bb2(al1n["ln_nodes"])] + eggw(gc0),
        [64, 128, 128, 64], node_update_body)
    ylg2, TSp2, TDp2, SUp2 = _tcmap(
        "t5b", EP, 4096, [M2, y1],
        [_g2(al1n["ln_edges"]), _bb2(al1n["ln_edges"])] + eggw(al1e),
        [64, 128, 64, 64], edge_update_body)
    if dense3:
        Sp2 = _sc_edge_egg_dense(TSp2, TDp2, gp2, lgs_p, False, TPad)
    else:
        TDp2g = jnp.pad(TDp2, ((0, 0), (0, 64)))
        Q2 = _sc_edge_egg(TSp2, TDp2g, gp2, lgs_p, lgd_p, TPad, False, qrows)
        Sp2 = _sc_gather3(Q2, *idx3)
    y2, G3 = _tcmap(
        "t6b", EP, 4096, [ylg2, SUp2, Sp2],
        [_g2(al1e["ln_nodes"]), _bb2(al1e["ln_nodes"]),
         _w(gc0["edge_gate"]), _b2(gc0["edge_gate"])],
        [64, 64], tri_update_body)

    M3, S3 = _sc_node_egg(TS3, TD3, G3, src_p, dst_p)

    def t4c_body(ib, cb):
        (x, su, S, esed) = ib
        lng, lnb, wsg, bsg, wdg, bdg = cb
        s = S[0, :, 0:64] + S[1, :, 0:64]
        sh = S[0, :, 64:128] + S[1, :, 64:128]
        h = sh / (s + 1e-6)
        xn = x + _silu(_lnorm(su + h, lng, lnb))
        rr = jnp.dot(xn, jnp.concatenate([wsg, wdg], axis=1),
                     preferred_element_type=_f32) + jnp.concatenate(
                         [bsg, bdg], axis=1)
        a4 = rr[:, 0:64]
        d4 = rr[:, 64:128]
        zp = jnp.zeros((xn.shape[0], 60), _f32)
        ta = jnp.concatenate([a4, esed[:, 0:4], zp], axis=1)
        tdx = jnp.concatenate([d4, esed[:, 4:8], zp], axis=1)
        return ta, tdx

    TA, TDX = _tcmap(
        "t4c", NPAD, 1264, [x2, SU3, S3, ES],
        [_g2(gc0["ln_nodes"]), _bb2(gc0["ln_nodes"]),
         _w(gc1["src_gate"]), _b2(gc1["src_gate"]),
         _w(gc1["dst_gate"]), _b2(gc1["dst_gate"])],
        [128, 128], t4c_body)

    def t5c_body(ib, cb):
        (m, y) = ib
        lng, lnb, weg, beg = cb
        yn = y + _silu(_lnorm(m, lng, lnb))
        g4 = _linblk(yn, weg, beg)
        g4p = jnp.concatenate([g4, jnp.zeros_like(yn[:, 0:16])], axis=1)
        return yn, g4p

    y3, G4P = _tcmap(
        "t5c", EP, 4096, [M3, y2],
        [_g2(gc0["ln_edges"]), _bb2(gc0["ln_edges"]),
         _w(gc1["edge_gate"]), _b2(gc1["edge_gate"])],
        [64, 80], t5c_body)

    M4X = _sc_gather_add80(TA, TDX, G4P, src_p, dst_p)

    def t7(m4x_ref, y3_ref, ue_ref, lng_ref, lnb_ref, fw_ref, fb_ref, out_ref):
        i = pl.program_id(0)
        m4 = m4x_ref[:, 0:64]
        esd = m4x_ref[:, 64:68]
        y = y3_ref[...]
        yn = y + _silu(_lnorm(m4, lng_ref[...], lnb_ref[...]))
        bond = jax.nn.sigmoid(
            jnp.dot(yn, fw_ref[...], preferred_element_type=_f32)
            + fb_ref[...])
        pe = jnp.exp(esd)
        bl = ue_ref[:, 3:4]
        cutv = ue_ref[:, 4:5]
        f_rep = pe[:, 0:1] * jnp.exp(-pe[:, 1:2] * bl)
        f_att = pe[:, 2:3] * jnp.exp(-pe[:, 3:4] * bl)
        V = cutv * (f_rep - bond * f_att)
        rowid = i * 4096 + lax.broadcasted_iota(jnp.int32, (4096, 1), 0)
        V = jnp.where(rowid < E, V, 0.0)
        bs = jnp.sum(V)

        @pl.when(i == 0)
        def _():
            out_ref[...] = jnp.zeros((1, 1), _f32)
        out_ref[...] += jnp.reshape(bs, (1, 1))

    tot = pl.pallas_call(
        t7,
        grid=(EP // 4096,),
        in_specs=[pl.BlockSpec((4096, 80), lambda i: (i, 0)),
                  pl.BlockSpec((4096, 64), lambda i: (i, 0)),
                  pl.BlockSpec((4096, 16), lambda i: (i, 0)),
                  pl.BlockSpec((1, 64), lambda i: (0, 0)),
                  pl.BlockSpec((1, 64), lambda i: (0, 0)),
                  pl.BlockSpec((64, 1), lambda i: (0, 0)),
                  pl.BlockSpec((1, 1), lambda i: (0, 0))],
        out_specs=pl.BlockSpec((1, 1), lambda i: (0, 0)),
        out_shape=jax.ShapeDtypeStruct((1, 1), _f32),
    )(M4X, y3, ue16,
      _g2(gc1["ln_edges"]), _bb2(gc1["ln_edges"]),
      _w(p["fc"]), p["fc"]["b"].reshape(1, 1))

    return tot[0, 0] / np.float32(N)


# speedup vs baseline: 1.3060x; 1.0146x over previous
import functools

import jax
import jax.numpy as jnp
import numpy as np
from jax import lax
from jax.experimental import pallas as pl
from jax.experimental.pallas import tpu as pltpu
from jax.experimental.pallas import tpu_sc as plsc

N = 10000
E = 160000
H = 64
K = 3
EP = 163840
NPAD = 10112
NC = 2
NS = 16
NW = NC * NS
C = 128

_f32 = jnp.float32


def _tcmap(name, nrows, block, ins, consts, out_dims, body):
    grid = nrows // block
    in_specs = []
    for a in ins:
        if a.ndim == 3:
            in_specs.append(pl.BlockSpec((a.shape[0], block, a.shape[2]),
                                         lambda i: (0, i, 0)))
        else:
            rb = block * a.shape[0] // nrows
            in_specs.append(pl.BlockSpec((rb, a.shape[1]), lambda i: (i, 0)))
    for c in consts:
        in_specs.append(pl.BlockSpec(c.shape, lambda i: (0,) * c.ndim))
    out_specs = [pl.BlockSpec((block, d), lambda i: (i, 0)) for d in out_dims]
    nin, ncon = len(ins), len(consts)

    def kern(*refs):
        ib = [refs[i][...] for i in range(nin)]
        cb = [refs[nin + i][...] for i in range(ncon)]
        outs = body(ib, cb)
        for k, ob in enumerate(outs):
            refs[nin + ncon + k][...] = ob

    return pl.pallas_call(
        kern,
        grid=(grid,),
        in_specs=in_specs,
        out_specs=out_specs,
        out_shape=[jax.ShapeDtypeStruct((nrows, d), _f32) for d in out_dims],
    )(*ins, *consts)


def _silu(x):
    return x * jax.nn.sigmoid(x)


def _lnorm(x, g, b):
    mu = jnp.mean(x, axis=-1, keepdims=True)
    var = jnp.mean((x - mu) ** 2, axis=-1, keepdims=True)
    return g * (x - mu) / jnp.sqrt(var + 1e-5) + b


def _mlpblk(x, w, b, g, bb):
    return _silu(_lnorm(jnp.dot(x, w, preferred_element_type=_f32) + b, g, bb))


def _linblk(x, w, b):
    y = jnp.dot(x, w, preferred_element_type=_f32)
    return y if b is None else y + b


_MESH = plsc.VectorSubcoreMesh(core_axis_name="c", subcore_axis_name="s")


def _wid_base(rows_pw):
    c = lax.axis_index("c")
    s = lax.axis_index("s")
    return (s * NC + c) * rows_pw


def _sc_gather_pairs(table, idx_a, idx_b, nrows, ow):
    rows_pw = nrows // NW
    nchunk = rows_pw // C
    np16 = ow // 16

    @functools.partial(
        pl.kernel,
        out_type=[jax.ShapeDtypeStruct((nrows, ow), _f32),
                  jax.ShapeDtypeStruct((nrows, ow), _f32)],
        mesh=_MESH,
        scratch_types=[pltpu.VMEM((C,), jnp.int32),
                       pltpu.VMEM((C,), jnp.int32),
                       pltpu.VMEM((C, 128), _f32),
                       pltpu.VMEM((C, 128), _f32),
                       pltpu.VMEM((C, ow), _f32),
                       pltpu.VMEM((C, ow), _f32),
                       pltpu.SemaphoreType.DMA,
                       pltpu.SemaphoreType.DMA],
    )
    def k(tab, ia, ib, oa, ob, iva, ivb, ga, gb, pa, pb, sema, semb):
        base0 = _wid_base(rows_pw)

        def step(j, _):
            base = base0 + j * C
            pltpu.sync_copy(ia.at[pl.ds(base, C)], iva)
            pltpu.sync_copy(ib.at[pl.ds(base, C)], ivb)
            cpa = pltpu.async_copy(tab.at[iva], ga, sema)
            cpb = pltpu.async_copy(tab.at[ivb], gb, semb)
            cpa.wait()
            cpb.wait()

            @plsc.parallel_loop(0, C, 1, unroll=4)
            def row(i):
                for kk in range(np16):
                    sl = pl.ds(kk * 16, 16)
                    pa[i, sl] = ga[i, sl]
                    pb[i, sl] = gb[i, sl]
            pltpu.sync_copy(pa, oa.at[pl.ds(base, C)])
            pltpu.sync_copy(pb, ob.at[pl.ds(base, C)])
            return _

        lax.fori_loop(0, nchunk, step, None)

    return k(table, idx_a, idx_b)


def _sc_gather_one(table, idx, nrows, ow):
    rows_pw = nrows // NW
    nchunk = rows_pw // C
    np16 = ow // 16

    @functools.partial(
        pl.kernel,
        out_type=[jax.ShapeDtypeStruct((nrows, ow), _f32)],
        mesh=_MESH,
        scratch_types=[pltpu.VMEM((C,), jnp.int32),
                       pltpu.VMEM((C,), jnp.int32),
                       pltpu.VMEM((C, 128), _f32),
                       pltpu.VMEM((C, 128), _f32),
                       pltpu.VMEM((C, ow), _f32),
                       pltpu.SemaphoreType.DMA,
                       pltpu.SemaphoreType.DMA],
    )
    def k(tab, ia, oa, iva0, iva1, ga0, ga1, pa, sem0, sem1):
        base0 = _wid_base(rows_pw)
        iva = [iva0, iva1]
        ga = [ga0, ga1]
        sem = [sem0, sem1]

        def issue(j, b):
            base = base0 + j * C
            pltpu.sync_copy(ia.at[pl.ds(base, C)], iva[b])
            pltpu.async_copy(tab.at[iva[b]], ga[b], sem[b])

        def finish(j, b):
            pltpu.make_async_copy(tab.at[iva[b]], ga[b], sem[b]).wait()

            @plsc.parallel_loop(0, C, 1, unroll=4)
            def row(i):
                for kk in range(np16):
                    sl = pl.ds(kk * 16, 16)
                    pa[i, sl] = ga[b][i, sl]
            pltpu.sync_copy(pa, oa.at[pl.ds(base0 + j * C, C)])

        issue(0, 0)

        def step(p, _):
            j0 = 2 * p
            @pl.when(j0 + 1 < nchunk)
            def _():
                issue(j0 + 1, 1)
            finish(j0, 0)
            @pl.when(j0 + 2 < nchunk)
            def _():
                issue(j0 + 2, 0)
            @pl.when(j0 + 1 < nchunk)
            def _():
                finish(j0 + 1, 1)
            return _

        lax.fori_loop(0, (nchunk + 1) // 2, step, None)

    res = k(table, idx)
    return res[0] if isinstance(res, (list, tuple)) else res


def _sc_edge_egg_dense(ts, td64, g, lgs, write_m, tp3):
    CB = 64
    CT = 3 * CB
    rows_pw = tp3 // NW
    bonds_pw = rows_pw // 3
    nchunk = rows_pw // CT
    outs = [jax.ShapeDtypeStruct((EP, 128), _f32)]
    if write_m:
        outs.append(jax.ShapeDtypeStruct((tp3, 64), _f32))

    @functools.partial(
        pl.kernel,
        out_type=outs,
        mesh=_MESH,
        scratch_types=[pltpu.VMEM((CT,), jnp.int32),
                       pltpu.VMEM((CT,), jnp.int32),
                       pltpu.VMEM((CT, 128), _f32),
                       pltpu.VMEM((CT, 128), _f32),
                       pltpu.VMEM((CB, 64), _f32),
                       pltpu.VMEM((CB, 64), _f32),
                       pltpu.VMEM((CT, 64), _f32),
                       pltpu.VMEM((CT, 64), _f32),
                       pltpu.VMEM((CB, 128), _f32),
                       pltpu.SemaphoreType.DMA,
                       pltpu.SemaphoreType.DMA],
    )
    def k(tsr, tdr, gr, sr, so, *rest):
        if write_m:
            mo = rest[0]
            rest = rest[1:]
        else:
            mo = None
        (ivs0, ivs1, gs0, gs1, tdv0, tdv1, gv0, gv1, qs, sem0, sem1) = rest
        ivs = [ivs0, ivs1]
        gs = [gs0, gs1]
        tdv = [tdv0, tdv1]
        gv = [gv0, gv1]
        sem = [sem0, sem1]
        cid = lax.axis_index("c")
        sid = lax.axis_index("s")
        w = sid * NC + cid
        tbase0 = w * rows_pw
        bbase0 = w * bonds_pw

        def issue(j, b):
            tbase = tbase0 + j * CT
            bbase = bbase0 + j * CB
            pltpu.sync_copy(sr.at[pl.ds(tbase, CT)], ivs[b])
            pltpu.async_copy(tsr.at[ivs[b].at[pl.ds(0, C)]],
                             gs[b].at[pl.ds(0, C)], sem[b])
            pltpu.async_copy(tsr.at[ivs[b].at[pl.ds(C, CT - C)]],
                             gs[b].at[pl.ds(C, CT - C)], sem[b])
            pltpu.async_copy(tdr.at[pl.ds(bbase, CB)], tdv[b], sem[b])
            pltpu.async_copy(gr.at[pl.ds(tbase, CT)], gv[b], sem[b])

        def finish(j, b):
            tbase = tbase0 + j * CT
            bbase = bbase0 + j * CB
            pltpu.make_async_copy(tsr.at[ivs[b].at[pl.ds(0, C)]],
                                  gs[b].at[pl.ds(0, C)], sem[b]).wait()
            pltpu.make_async_copy(tsr.at[ivs[b].at[pl.ds(C, CT - C)]],
                                  gs[b].at[pl.ds(C, CT - C)], sem[b]).wait()
            pltpu.make_async_copy(tdr.at[pl.ds(bbase, CB)], tdv[b], sem[b]).wait()
            pltpu.make_async_copy(gr.at[pl.ds(tbase, CT)], gv[b], sem[b]).wait()
            gsb, tdb, gvb = gs[b], tdv[b], gv[b]

            @plsc.parallel_loop(0, CB, 1, unroll=2)
            def bond(bb):
                for kk in range(4):
                    sl = pl.ds(kk * 16, 16)
                    sl2 = pl.ds(64 + kk * 16, 16)
                    d = tdb[bb, sl]
                    ssum = jnp.zeros((16,), _f32)
                    shsum = jnp.zeros((16,), _f32)
                    for q in range(3):
                        i = bb * 3 + q
                        m = gsb[i, sl] + d + gvb[i, sl]
                        if write_m:
                            gvb[i, sl] = m
                        sig = 1.0 / (1.0 + jnp.exp(-m))
                        ssum = ssum + sig
                        shsum = shsum + sig * gsb[i, sl2]
                    qs[bb, sl] = ssum
                    qs[bb, sl2] = shsum
            if write_m:
                pltpu.sync_copy(gvb, mo.at[pl.ds(tbase, CT)])
            pltpu.sync_copy(qs, so.at[pl.ds(bbase, CB)])

        issue(0, 0)

        def step(p, _):
            j0 = 2 * p
            @pl.when(j0 + 1 < nchunk)
            def _():
                issue(j0 + 1, 1)
            finish(j0, 0)
            @pl.when(j0 + 2 < nchunk)
            def _():
                issue(j0 + 2, 0)
            @pl.when(j0 + 1 < nchunk)
            def _():
                finish(j0 + 1, 1)
            return _

        lax.fori_loop(0, (nchunk + 1) // 2, step, None)

    res = k(ts, td64, g, lgs)
    if write_m:
        return res[0], res[1]
    return res[0] if isinstance(res, (list, tuple)) else res


def _sc_node_egg(ts, td, g, src, dst):
    CN = 64
    rows_pw = EP // NW
    nchunk = rows_pw // CN
    zrows = NPAD // NS

    @functools.partial(
        pl.kernel,
        out_type=[jax.ShapeDtypeStruct((EP, 64), _f32),
                  jax.ShapeDtypeStruct((NC, NPAD, 128), _f32)],
        mesh=_MESH,
        scratch_types=[pltpu.VMEM((CN,), jnp.int32),
                       pltpu.VMEM((CN,), jnp.int32),
                       pltpu.VMEM((CN,), jnp.int32),
                       pltpu.VMEM((CN,), jnp.int32),
                       pltpu.VMEM((CN, 128), _f32),
                       pltpu.VMEM((CN, 128), _f32),
                       pltpu.VMEM((CN, 128), _f32),
                       pltpu.VMEM((CN, 128), _f32),
                       pltpu.VMEM((CN, 64), _f32),
                       pltpu.VMEM((CN, 64), _f32),
                       pltpu.VMEM_SHARED((NPAD, 128), _f32),
                       pltpu.SemaphoreType.DMA,
                       pltpu.SemaphoreType.DMA],
    )
    def k(tsr, tdr, gr, sr, dr, mo, so, ivs0, ivs1, ivd0, ivd1,
          gs0, gs1, gd0, gd1, gv0, gv1, acc, sem0, sem1):
        ivs = [ivs0, ivs1]
        ivd = [ivd0, ivd1]
        gs = [gs0, gs1]
        gd = [gd0, gd1]
        gv = [gv0, gv1]
        sem = [sem0, sem1]
        cid = lax.axis_index("c")
        sid = lax.axis_index("s")
        base0 = (sid * NC + cid) * rows_pw

        @plsc.parallel_loop(0, CN, 1, unroll=4)
        def zrow(i):
            for kk in range(8):
                gs0[i, pl.ds(kk * 16, 16)] = jnp.zeros((16,), _f32)
        for t in range(zrows // CN):
            pltpu.sync_copy(gs0, acc.at[pl.ds(sid * zrows + t * CN, CN)])
        rem = zrows - (zrows // CN) * CN
        pltpu.sync_copy(gs0.at[pl.ds(0, rem)],
                        acc.at[pl.ds(sid * zrows + zrows - rem, rem)])
        plsc.subcore_barrier()

        def issue(j, b):
            base = base0 + j * CN
            pltpu.sync_copy(sr.at[pl.ds(base, CN)], ivs[b])
            pltpu.sync_copy(dr.at[pl.ds(base, CN)], ivd[b])
            pltpu.async_copy(tsr.at[ivs[b]], gs[b], sem[b])
            pltpu.async_copy(tdr.at[ivd[b]], gd[b], sem[b])
            pltpu.async_copy(gr.at[pl.ds(base, CN)], gv[b], sem[b])

        def finish(j, b):
            base = base0 + j * CN
            pltpu.make_async_copy(tsr.at[ivs[b]], gs[b], sem[b]).wait()
            pltpu.make_async_copy(tdr.at[ivd[b]], gd[b], sem[b]).wait()
            pltpu.make_async_copy(gr.at[pl.ds(base, CN)], gv[b], sem[b]).wait()
            gsb, gdb, gvb = gs[b], gd[b], gv[b]

            @plsc.parallel_loop(0, CN, 1, unroll=2)
            def row(i):
                for kk in range(4):
                    a = gsb[i, pl.ds(kk * 16, 16)]
                    d = gdb[i, pl.ds(kk * 16, 16)]
                    gg = gvb[i, pl.ds(kk * 16, 16)]
                    m = a + d + gg
                    gvb[i, pl.ds(kk * 16, 16)] = m
                    sig = 1.0 / (1.0 + jnp.exp(-m))
                    gdb[i, pl.ds(kk * 16, 16)] = sig
                    bh = gsb[i, pl.ds(64 + kk * 16, 16)]
                    gdb[i, pl.ds(64 + kk * 16, 16)] = sig * bh
            pltpu.sync_copy(gvb, mo.at[pl.ds(base, CN)])
            pltpu.sync_copy(gdb, acc.at[ivd[b]], add=True)

        issue(0, 0)

        def step(p, _):
            j0 = 2 * p
            @pl.when(j0 + 1 < nchunk)
            def _():
                issue(j0 + 1, 1)
            finish(j0, 0)
            @pl.when(j0 + 2 < nchunk)
            def _():
                issue(j0 + 2, 0)
            @pl.when(j0 + 1 < nchunk)
            def _():
                finish(j0 + 1, 1)
            return _

        lax.fori_loop(0, (nchunk + 1) // 2, step, None)
        plsc.subcore_barrier()
        pltpu.sync_copy(acc.at[pl.ds(sid * zrows, zrows)],
                        so.at[cid, pl.ds(sid * zrows, zrows)])

    return k(ts, td, g, src, dst)


def _sc_edge_egg(ts, td, g, lgs, lgd, tp, write_m, qrows):
    rows_pw = tp // NW
    nchunk = rows_pw // C
    outs = [jax.ShapeDtypeStruct((qrows, 128), _f32)]
    if write_m:
        outs.append(jax.ShapeDtypeStruct((tp, 64), _f32))

    @functools.partial(
        pl.kernel,
        out_type=outs,
        mesh=_MESH,
        scratch_types=[pltpu.VMEM((C,), jnp.int32),
                       pltpu.VMEM((C,), jnp.int32),
                       pltpu.VMEM((C, 128), _f32),
                       pltpu.VMEM((C, 128), _f32),
                       pltpu.VMEM((C, 64), _f32),
                       pltpu.VMEM((C, 64), _f32),
                       pltpu.VMEM((C, 128), _f32),
                       pltpu.SemaphoreType.DMA,
                       pltpu.SemaphoreType.DMA],
    )
    def k(tsr, tdr, gr, sr, dr, qo, *rest):
        if write_m:
            mo = rest[0]
            ivs, ivd, gs, gd, gv, mv, qv, sema, semb = rest[1:]
        else:
            mo = None
            ivs, ivd, gs, gd, gv, mv, qv, sema, semb = rest
        cid = lax.axis_index("c")
        sid = lax.axis_index("s")
        base0 = (sid * NC + cid) * rows_pw

        @pl.when(jnp.logical_and(cid == 0, sid == 0))
        def _():
            def zrow(i, _):
                for kk in range(8):
                    qv[i, pl.ds(kk * 16, 16)] = jnp.zeros((16,), _f32)
                return _
            lax.fori_loop(0, C, zrow, None)
            pltpu.sync_copy(qv, qo.at[pl.ds(tp, C)])

        def step(j, _):
            base = base0 + j * C
            pltpu.sync_copy(sr.at[pl.ds(base, C)], ivs)
            pltpu.sync_copy(dr.at[pl.ds(base, C)], ivd)
            cpa = pltpu.async_copy(tsr.at[ivs], gs, sema)
            cpb = pltpu.async_copy(tdr.at[ivd], gd, semb)
            pltpu.sync_copy(gr.at[pl.ds(base, C)], gv)
            cpa.wait()
            cpb.wait()

            def row(i, _):
                for kk in range(4):
                    a = gs[i, pl.ds(kk * 16, 16)]
                    d = gd[i, pl.ds(kk * 16, 16)]
                    gg = gv[i, pl.ds(kk * 16, 16)]
                    m = a + d + gg
                    if write_m:
                        mv[i, pl.ds(kk * 16, 16)] = m
                    sig = 1.0 / (1.0 + jnp.exp(-m))
                    qv[i, pl.ds(kk * 16, 16)] = sig
                    bh = gs[i, pl.ds(64 + kk * 16, 16)]
                    qv[i, pl.ds(64 + kk * 16, 16)] = sig * bh
                return _
            lax.fori_loop(0, C, row, None)
            if write_m:
                pltpu.sync_copy(mv, mo.at[pl.ds(base, C)])
            pltpu.sync_copy(qv, qo.at[pl.ds(base, C)])
            return _

        lax.fori_loop(0, nchunk, step, None)

    res = k(ts, td, g, lgs, lgd)
    if write_m:
        return res[0], res[1]
    return res[0] if isinstance(res, (list, tuple)) else res


def _sc_gather3(q, i0, i1, i2):
    rows_pw = EP // NW
    nchunk = rows_pw // C

    @functools.partial(
        pl.kernel,
        out_type=[jax.ShapeDtypeStruct((EP, 128), _f32)],
        mesh=_MESH,
        scratch_types=[pltpu.VMEM((C,), jnp.int32),
                       pltpu.VMEM((C,), jnp.int32),
                       pltpu.VMEM((C,), jnp.int32),
                       pltpu.VMEM((C, 128), _f32),
                       pltpu.VMEM((C, 128), _f32),
                       pltpu.VMEM((C, 128), _f32),
                       pltpu.VMEM((C, 128), _f32),
                       pltpu.SemaphoreType.DMA,
                       pltpu.SemaphoreType.DMA,
                       pltpu.SemaphoreType.DMA],
    )
    def k(qr, r0, r1, r2, so, v0, v1, v2, g0, g1, g2, ov, s0, s1, s2):
        base0 = _wid_base(rows_pw)

        def step(j, _):
            base = base0 + j * C
            pltpu.sync_copy(r0.at[pl.ds(base, C)], v0)
            pltpu.sync_copy(r1.at[pl.ds(base, C)], v1)
            pltpu.sync_copy(r2.at[pl.ds(base, C)], v2)
            c0 = pltpu.async_copy(qr.at[v0], g0, s0)
            c1 = pltpu.async_copy(qr.at[v1], g1, s1)
            c2 = pltpu.async_copy(qr.at[v2], g2, s2)
            c0.wait()
            c1.wait()
            c2.wait()

            def row(i, _):
                for kk in range(8):
                    sl = pl.ds(kk * 16, 16)
                    ov[i, sl] = g0[i, sl] + g1[i, sl] + g2[i, sl]
                return _
            lax.fori_loop(0, C, row, None)
            pltpu.sync_copy(ov, so.at[pl.ds(base, C)])
            return _

        lax.fori_loop(0, nchunk, step, None)

    res = k(q, i0, i1, i2)
    return res[0] if isinstance(res, (list, tuple)) else res


def _sc_gather_add80(ta, tdx, g, src, dst):
    rows_pw = EP // NW
    nchunk = rows_pw // C

    @functools.partial(
        pl.kernel,
        out_type=[jax.ShapeDtypeStruct((EP, 80), _f32)],
        mesh=_MESH,
        scratch_types=[pltpu.VMEM((C,), jnp.int32),
                       pltpu.VMEM((C,), jnp.int32),
                       pltpu.VMEM((C,), jnp.int32),
                       pltpu.VMEM((C,), jnp.int32),
                       pltpu.VMEM((C, 128), _f32),
                       pltpu.VMEM((C, 128), _f32),
                       pltpu.VMEM((C, 128), _f32),
                       pltpu.VMEM((C, 128), _f32),
                       pltpu.VMEM((C, 80), _f32),
                       pltpu.VMEM((C, 80), _f32),
                       pltpu.VMEM((C, 80), _f32),
                       pltpu.SemaphoreType.DMA,
                       pltpu.SemaphoreType.DMA],
    )
    def k(tar, tdr, gr, sr, dr, mo, ivs0, ivs1, ivd0, ivd1,
          ga0, ga1, gd0, gd1, gv0, gv1, mv, sem0, sem1):
        ivs = [ivs0, ivs1]
        ivd = [ivd0, ivd1]
        ga = [ga0, ga1]
        gd = [gd0, gd1]
        gv = [gv0, gv1]
        sem = [sem0, sem1]
        base0 = _wid_base(rows_pw)

        def issue(j, b):
            base = base0 + j * C
            pltpu.sync_copy(sr.at[pl.ds(base, C)], ivs[b])
            pltpu.sync_copy(dr.at[pl.ds(base, C)], ivd[b])
            pltpu.async_copy(tar.at[ivs[b]], ga[b], sem[b])
            pltpu.async_copy(tdr.at[ivd[b]], gd[b], sem[b])
            pltpu.async_copy(gr.at[pl.ds(base, C)], gv[b], sem[b])

        def finish(j, b):
            base = base0 + j * C
            pltpu.make_async_copy(tar.at[ivs[b]], ga[b], sem[b]).wait()
            pltpu.make_async_copy(tdr.at[ivd[b]], gd[b], sem[b]).wait()
            pltpu.make_async_copy(gr.at[pl.ds(base, C)], gv[b], sem[b]).wait()
            gab, gdb, gvb = ga[b], gd[b], gv[b]

            @plsc.parallel_loop(0, C, 1, unroll=4)
            def row(i):
                for kk in range(5):
                    sl = pl.ds(kk * 16, 16)
                    mv[i, sl] = gab[i, sl] + gdb[i, sl] + gvb[i, sl]
            pltpu.sync_copy(mv, mo.at[pl.ds(base, C)])

        issue(0, 0)

        def step(p, _):
            j0 = 2 * p
            @pl.when(j0 + 1 < nchunk)
            def _():
                issue(j0 + 1, 1)
            finish(j0, 0)
            @pl.when(j0 + 2 < nchunk)
            def _():
                issue(j0 + 2, 0)
            @pl.when(j0 + 1 < nchunk)
            def _():
                finish(j0 + 1, 1)
            return _

        lax.fori_loop(0, (nchunk + 1) // 2, step, None)

    res = k(ta, tdx, g, src, dst)
    return res[0] if isinstance(res, (list, tuple)) else res


def _w(p):
    return p["w"]


def _b2(p):
    return p["b"].reshape(1, -1)


def _g2(p):
    return p["g"].reshape(1, -1)


def _bb2(p):
    return p["b"].reshape(1, -1)


def kernel(r, params, atom_numbers, edge_index, lg_src, lg_dst):
    T = lg_dst.shape[0]
    dense3 = (T == 3 * E)
    TPad = 3 * EP if dense3 else ((T + 4095) // 4096) * 4096

    src = edge_index[0].astype(jnp.int32)
    dst = edge_index[1].astype(jnp.int32)
    lgs = lg_src.astype(jnp.int32)
    lgd = lg_dst.astype(jnp.int32)

    src_p = jnp.concatenate([src, jnp.full((EP - E,), N, jnp.int32)])
    dst_p = jnp.concatenate([dst, jnp.full((EP - E,), N, jnp.int32)])
    lgs_p = jnp.concatenate([lgs, jnp.full((TPad - T,), E, jnp.int32)])
    lgd_p = jnp.concatenate([lgd, jnp.full((TPad - T,), E, jnp.int32)])
    if dense3:
        qrows = None
        idx3 = None
    else:
        qrows = TPad + C
        se = jnp.searchsorted(lgd, jnp.arange(E + 1, dtype=jnp.int32)).astype(jnp.int32)
        s_p = jnp.concatenate([se[:E], jnp.zeros((EP - E,), jnp.int32)])
        e_p = jnp.concatenate([se[1:], jnp.zeros((EP - E,), jnp.int32)])
        idx3 = [jnp.where(s_p + i < e_p, s_p + i, TPad).astype(jnp.int32)
                for i in range(K)]

    r16 = jnp.zeros((EP, 16), _f32).at[:E, 0:3].set(r.astype(_f32))
    an2 = jnp.concatenate([atom_numbers.astype(jnp.int32),
                           jnp.zeros((NPAD - N,), jnp.int32)]).reshape(NPAD, 1)

    p = params
    al0n, al0e = p["alignn"][0]["node"], p["alignn"][0]["edge"]
    al1n, al1e = p["alignn"][1]["node"], p["alignn"][1]["edge"]
    gc0, gc1 = p["gcn"][0], p["gcn"][1]

    cent_e = jnp.linspace(0.0, 8.0, 80).astype(_f32).reshape(1, 80)
    cent_a = jnp.linspace(-1.0, 1.0, 40).astype(_f32).reshape(1, 40)
    gam_e = float(79.0 / 8.0)
    gam_a = 19.5

    def t1_body(ib, cb):
        (X,) = ib
        (ce, w1, b1, g1_, bb1, w2, b2_, g2_, bb2, weg, beg) = cb
        bl2 = jnp.sum(X * X, axis=-1, keepdims=True)
        bl = jnp.sqrt(bl2)
        inv = 1.0 / jnp.maximum(bl, 1e-9)
        u = X * inv
        cutv = jnp.where(bl < 3.8, 1.0, 0.5 - 0.5 * jnp.sin(np.pi * (bl - 3.9) / 0.2))
        cutv = jnp.where(bl > 4.0, 0.0, cutv)
        rb = jnp.exp(-gam_e * (bl - ce) ** 2)
        y0 = _mlpblk(rb, w1, b1, g1_, bb1)
        y0 = _mlpblk(y0, w2, b2_, g2_, bb2)
        G1 = _linblk(y0, weg, beg)
        zpad = jnp.zeros((X.shape[0], 123), _f32)
        u_ext = jnp.concatenate([u[:, 0:3], bl, cutv, zpad], axis=1)
        ue16 = u_ext[:, 0:16]
        return y0, G1, u_ext, ue16

    y0, G1, u_ext, ue16 = _tcmap(
        "t1", EP, 4096, [r16],
        [cent_e,
         _w(p["edge_mlp1"]["lin"]), _b2(p["edge_mlp1"]["lin"]),
         _g2(p["edge_mlp1"]["ln"]), _bb2(p["edge_mlp1"]["ln"]),
         _w(p["edge_mlp2"]["lin"]), _b2(p["edge_mlp2"]["lin"]),
         _g2(p["edge_mlp2"]["ln"]), _bb2(p["edge_mlp2"]["ln"]),
         _w(al0n["edge_gate"]), _b2(al0n["edge_gate"])],
        [64, 64, 128, 16], t1_body)

    if dense3:
        uu1 = _sc_gather_one(u_ext, lgs_p, TPad, 16)
        uu2 = None
    else:
        uu1, uu2 = _sc_gather_pairs(u_ext, lgs_p, lgd_p, TPad, 16)

    def t2_body(ib, cb):
        (U1, U2) = ib
        (ca, w1, b1, g1_, bb1, w2, b2_, g2_, bb2, weg, beg) = cb
        cos = -jnp.sum(U1[:, 0:3] * U2[:, 0:3], axis=-1, keepdims=True)
        cos = jnp.clip(cos, -1.0, 1.0)
        rb = jnp.exp(-gam_a * (cos - ca) ** 2)
        z0 = _mlpblk(rb, w1, b1, g1_, bb1)
        z0 = _mlpblk(z0, w2, b2_, g2_, bb2)
        gp = _linblk(z0, weg, beg)
        return z0, gp

    def t2d_body(ib, cb):
        (U1, UE) = ib
        u2 = jnp.repeat(UE, 3, axis=0)
        return t2_body([U1, u2], cb)

    t2_ins = [uu1, ue16] if dense3 else [uu1, uu2]
    t2_block = 3072 if dense3 else 4096
    z0, gp1 = _tcmap(
        "t2", TPad, t2_block, t2_ins,
        [cent_a,
         _w(p["angle_mlp1"]["lin"]), _b2(p["angle_mlp1"]["lin"]),
         _g2(p["angle_mlp1"]["ln"]), _bb2(p["angle_mlp1"]["ln"]),
         _w(p["angle_mlp2"]["lin"]), _b2(p["angle_mlp2"]["lin"]),
         _g2(p["angle_mlp2"]["ln"]), _bb2(p["angle_mlp2"]["ln"]),
         _w(al0e["edge_gate"]), _b2(al0e["edge_gate"])],
        [64, 64], t2d_body if dense3 else t2_body)

    def t3_body(ib, cb):
        (an,) = ib
        (emb, wsg, bsg, wdu, bdu, wdg, bdg, wsu, bsu, wes, bes, wed) = cb
        onehot = (lax.broadcasted_iota(jnp.int32, (an.shape[0], 128), 1)
                  == an).astype(_f32)
        x0 = jnp.dot(onehot, emb, preferred_element_type=_f32)
        wcat = jnp.concatenate([wsg, wdu, wdg, wsu, wes, wed], axis=1)
        bcat = jnp.concatenate(
            [bsg, bdu, bdg, bsu, bes, jnp.zeros((1, 4), _f32)], axis=1)
        rr = jnp.dot(x0, wcat, preferred_element_type=_f32) + bcat
        ts = rr[:, 0:128]
        td = jnp.concatenate([rr[:, 128:192],
                              jnp.zeros((x0.shape[0], 64), _f32)], axis=1)
        su = rr[:, 192:256]
        esed = jnp.concatenate([rr[:, 256:264],
                                jnp.zeros((x0.shape[0], 8), _f32)], axis=1)
        return x0, ts, td, su, esed

    x0, TS1, TD1, SU1, ES = _tcmap(
        "t3", NPAD, 1264, [an2],
        [p["atom_embedding"],
         _w(al0n["src_gate"]), _b2(al0n["src_gate"]),
         _w(al0n["dst_update"]), _b2(al0n["dst_update"]),
         _w(al0n["dst_gate"]), _b2(al0n["dst_gate"]),
         _w(al0n["src_update"]), _b2(al0n["src_update"]),
         _w(p["int_src"]), _b2(p["int_src"]),
         _w(p["int_dst"])],
        [64, 128, 128, 64, 16], t3_body)

    def node_update_body(ib, cb):
        (x, su, S) = ib
        lng, lnb = cb[0], cb[1]
        s = S[0, :, 0:64] + S[1, :, 0:64]
        sh = S[0, :, 64:128] + S[1, :, 64:128]
        h = sh / (s + 1e-6)
        xn = x + _silu(_lnorm(su + h, lng, lnb))
        outs = [xn]
        ws = cb[2:]
        wcat = jnp.concatenate(ws[0::2], axis=1)
        bcat = jnp.concatenate(ws[1::2], axis=1)
        rr = jnp.dot(xn, wcat, preferred_element_type=_f32) + bcat
        nres = len(ws) // 2
        res = [rr[:, 64 * t:64 * (t + 1)] for t in range(nres)]
        if nres == 4:
            outs.append(jnp.concatenate([res[0], res[1]], axis=1))
            outs.append(jnp.concatenate([res[2], jnp.zeros_like(res[2])], axis=1))
            outs.append(res[3])
        else:
            outs.extend(res)
        return outs

    def edge_update_body(ib, cb):
        (m, y) = ib
        lng, lnb = cb[0], cb[1]
        yn = y + _silu(_lnorm(m, lng, lnb))
        outs = [yn]
        ws = cb[2:]
        wcat = jnp.concatenate(ws[0::2], axis=1)
        bcat = jnp.concatenate(ws[1::2], axis=1)
        rr = jnp.dot(yn, wcat, preferred_element_type=_f32) + bcat
        nres = len(ws) // 2
        res = [rr[:, 64 * t:64 * (t + 1)] for t in range(nres)]
        if nres == 4:
            outs.append(jnp.concatenate([res[0], res[1]], axis=1))
            outs.append(res[2])
            outs.append(res[3])
        else:
            outs.extend(res)
        return outs

    def eggw(q):
        return [_w(q["src_gate"]), _b2(q["src_gate"]),
                _w(q["dst_update"]), _b2(q["dst_update"]),
                _w(q["dst_gate"]), _b2(q["dst_gate"]),
                _w(q["src_update"]), _b2(q["src_update"])]

    def tri_update_body(ib, cb):
        (ylg, su, Sp) = ib
        lng, lnb = cb[0], cb[1]
        s = Sp[:, 0:64]
        sh = Sp[:, 64:128]
        h = sh / (s + 1e-6)
        yn = ylg + _silu(_lnorm(su + h, lng, lnb))
        outs = [yn]
        ws = cb[2:]
        for t in range(0, len(ws), 2):
            outs.append(_linblk(yn, ws[t], ws[t + 1]))
        return outs

    M1, S1 = _sc_node_egg(TS1, TD1, G1, src_p, dst_p)
    x1, TS2, TD2, SU2 = _tcmap(
        "t4a", NPAD, 1264, [x0, SU1, S1],
        [_g2(al0n["ln_nodes"]), _bb2(al0n["ln_nodes"])] + eggw(al1n),
        [64, 128, 128, 64], node_update_body)
    ylg1, TSp1, TDp1, SUp1 = _tcmap(
        "t5a", EP, 4096, [M1, y0],
        [_g2(al0n["ln_edges"]), _bb2(al0n["ln_edges"])] + eggw(al0e),
        [64, 128, 64, 64], edge_update_body)
    if dense3:
        Sp1, Mp1 = _sc_edge_egg_dense(TSp1, TDp1, gp1, lgs_p, True, TPad)
    else:
        TDp1g = jnp.pad(TDp1, ((0, 0), (0, 64)))
        Q1, Mp1 = _sc_edge_egg(TSp1, TDp1g, gp1, lgs_p, lgd_p, TPad, True, qrows)
        Sp1 = _sc_gather3(Q1, *idx3)
    y1, G2 = _tcmap(
        "t6a", EP, 4096, [ylg1, SUp1, Sp1],
        [_g2(al0e["ln_nodes"]), _bb2(al0e["ln_nodes"]),
         _w(al1n["edge_gate"]), _b2(al1n["edge_gate"])],
        [64, 64], tri_update_body)

    def zup_body(ib, cb):
        (m, z) = ib
        lng, lnb, weg, beg = cb
        zn = z + _silu(_lnorm(m, lng, lnb))
        return (_linblk(zn, weg, beg),)

    (gp2,) = _tcmap(
        "t5z", TPad, 4096, [Mp1, z0],
        [_g2(al0e["ln_edges"]), _bb2(al0e["ln_edges"]),
         _w(al1e["edge_gate"]), _b2(al1e["edge_gate"])],
        [64], zup_body)

    M2, S2 = _sc_node_egg(TS2, TD2, G2, src_p, dst_p)
    x2, TS3, TD3, SU3 = _tcmap(
        "t4b", NPAD, 1264, [x1, SU2, S2],
        [_g2(al1n["ln_nodes"]), _bb2(al1n["ln_nodes"])] + eggw(gc0),
        [64, 128, 128, 64], node_update_body)
    ylg2, TSp2, TDp2, SUp2 = _tcmap(
        "t5b", EP, 4096, [M2, y1],
        [_g2(al1n["ln_edges"]), _bb2(al1n["ln_edges"])] + eggw(al1e),
        [64, 128, 64, 64], edge_update_body)
    if dense3:
        Sp2 = _sc_edge_egg_dense(TSp2, TDp2, gp2, lgs_p, False, TPad)
    else:
        TDp2g = jnp.pad(TDp2, ((0, 0), (0, 64)))
        Q2 = _sc_edge_egg(TSp2, TDp2g, gp2, lgs_p, lgd_p, TPad, False, qrows)
        Sp2 = _sc_gather3(Q2, *idx3)
    y2, G3 = _tcmap(
        "t6b", EP, 4096, [ylg2, SUp2, Sp2],
        [_g2(al1e["ln_nodes"]), _bb2(al1e["ln_nodes"]),
         _w(gc0["edge_gate"]), _b2(gc0["edge_gate"])],
        [64, 64], tri_update_body)

    M3, S3 = _sc_node_egg(TS3, TD3, G3, src_p, dst_p)

    def t4c_body(ib, cb):
        (x, su, S, esed) = ib
        lng, lnb, wsg, bsg, wdg, bdg = cb
        s = S[0, :, 0:64] + S[1, :, 0:64]
        sh = S[0, :, 64:128] + S[1, :, 64:128]
        h = sh / (s + 1e-6)
        xn = x + _silu(_lnorm(su + h, lng, lnb))
        rr = jnp.dot(xn, jnp.concatenate([wsg, wdg], axis=1),
                     preferred_element_type=_f32) + jnp.concatenate(
                         [bsg, bdg], axis=1)
        a4 = rr[:, 0:64]
        d4 = rr[:, 64:128]
        zp = jnp.zeros((xn.shape[0], 60), _f32)
        ta = jnp.concatenate([a4, esed[:, 0:4], zp], axis=1)
        tdx = jnp.concatenate([d4, esed[:, 4:8], zp], axis=1)
        return ta, tdx

    TA, TDX = _tcmap(
        "t4c", NPAD, 1264, [x2, SU3, S3, ES],
        [_g2(gc0["ln_nodes"]), _bb2(gc0["ln_nodes"]),
         _w(gc1["src_gate"]), _b2(gc1["src_gate"]),
         _w(gc1["dst_gate"]), _b2(gc1["dst_gate"])],
        [128, 128], t4c_body)

    def t5c_body(ib, cb):
        (m, y) = ib
        lng, lnb, weg, beg = cb
        yn = y + _silu(_lnorm(m, lng, lnb))
        g4 = _linblk(yn, weg, beg)
        g4p = jnp.concatenate([g4, jnp.zeros_like(yn[:, 0:16])], axis=1)
        return yn, g4p

    y3, G4P = _tcmap(
        "t5c", EP, 4096, [M3, y2],
        [_g2(gc0["ln_edges"]), _bb2(gc0["ln_edges"]),
         _w(gc1["edge_gate"]), _b2(gc1["edge_gate"])],
        [64, 80], t5c_body)

    M4X = _sc_gather_add80(TA, TDX, G4P, src_p, dst_p)

    def t7(m4x_ref, y3_ref, ue_ref, lng_ref, lnb_ref, fw_ref, fb_ref, out_ref):
        i = pl.program_id(0)
        m4 = m4x_ref[:, 0:64]
        esd = m4x_ref[:, 64:68]
        y = y3_ref[...]
        yn = y + _silu(_lnorm(m4, lng_ref[...], lnb_ref[...]))
        bond = jax.nn.sigmoid(
            jnp.dot(yn, fw_ref[...], preferred_element_type=_f32)
            + fb_ref[...])
        pe = jnp.exp(esd)
        bl = ue_ref[:, 3:4]
        cutv = ue_ref[:, 4:5]
        f_rep = pe[:, 0:1] * jnp.exp(-pe[:, 1:2] * bl)
        f_att = pe[:, 2:3] * jnp.exp(-pe[:, 3:4] * bl)
        V = cutv * (f_rep - bond * f_att)
        rowid = i * 4096 + lax.broadcasted_iota(jnp.int32, (4096, 1), 0)
        V = jnp.where(rowid < E, V, 0.0)
        bs = jnp.sum(V)

        @pl.when(i == 0)
        def _():
            out_ref[...] = jnp.zeros((1, 1), _f32)
        out_ref[...] += jnp.reshape(bs, (1, 1))

    tot = pl.pallas_call(
        t7,
        grid=(EP // 4096,),
        in_specs=[pl.BlockSpec((4096, 80), lambda i: (i, 0)),
                  pl.BlockSpec((4096, 64), lambda i: (i, 0)),
                  pl.BlockSpec((4096, 16), lambda i: (i, 0)),
                  pl.BlockSpec((1, 64), lambda i: (0, 0)),
                  pl.BlockSpec((1, 64), lambda i: (0, 0)),
                  pl.BlockSpec((64, 1), lambda i: (0, 0)),
                  pl.BlockSpec((1, 1), lambda i: (0, 0))],
        out_specs=pl.BlockSpec((1, 1), lambda i: (0, 0)),
        out_shape=jax.ShapeDtypeStruct((1, 1), _f32),
    )(M4X, y3, ue16,
      _g2(gc1["ln_edges"]), _bb2(gc1["ln_edges"]),
      _w(p["fc"]), p["fc"]["b"].reshape(1, 1))

    return tot[0, 0] / np.float32(N)
